# baseline (speedup 1.0000x reference)
import functools

import jax
import jax.numpy as jnp
import numpy as np
from jax import lax
from jax.experimental import pallas as pl
from jax.experimental.pallas import tpu as pltpu
from jax.experimental.pallas import tpu_sc as plsc

_EPS_BN = 1e-5
_INV = np.float32(1.0) / np.sqrt(np.float32(1.0 + _EPS_BN))

_L = 16
_NB = 272


def _lane_gather(vec, idx):
    return lax.gather(
        vec, idx[:, None],
        dimension_numbers=lax.GatherDimensionNumbers(
            offset_dims=(), collapsed_slice_dims=(0,), start_index_map=(0,)),
        slice_sizes=(1,),
        mode=lax.GatherScatterMode.PROMISE_IN_BOUNDS)


def _ballq_tec(n, k, qpt, *refs):
    (px_h, py_h, pz_h, yx_h, yy_h, yz_h, out_h,
     px_v, py_v, pz_v, yx_v, yy_v, yz_v, bits_v, hist_v, row_v) = refs
    nvec = n // _L
    wid = lax.axis_index("s") * 2 + lax.axis_index("c")

    pltpu.sync_copy(px_h, px_v)
    pltpu.sync_copy(py_h, py_v)
    pltpu.sync_copy(pz_h, pz_v)
    pltpu.sync_copy(yx_h, yx_v)
    pltpu.sync_copy(yy_h, yy_v)
    pltpu.sync_copy(yz_h, yz_v)

    lane = lax.iota(jnp.int32, _L)
    ones = jnp.full((_L,), 1, jnp.int32)

    def clear_hist(j, c):
        hist_v[pl.ds(j * _L, _L)] = jnp.zeros((_L,), jnp.int32)
        return c

    def scan_hist(k_rem):
        def sj(j, st):
            found, bsel, below, run = st
            def sl(l, a):
                return a + hist_v[pl.ds(l * _NB + j * _L, _L)]
            acc = lax.fori_loop(0, _L, sl, jnp.zeros((_L,), jnp.int32))
            tot = jnp.sum(acc)
            cum = plsc.cumsum(acc) + run
            hit = cum > k_rem
            nhit = jnp.sum(hit.astype(jnp.int32))
            ffs = plsc.all_reduce_ffs(hit)
            excl = cum - acc
            below_here = jnp.sum(jnp.where(lane == ffs, excl, 0))
            bin_here = j * _L + jnp.max(ffs)
            take = (found == 0) & (nhit > 0)
            bsel = jnp.where(take, bin_here, bsel)
            below = jnp.where(take, below_here, below)
            found = jnp.where(nhit > 0, 1, found)
            return (found, bsel, below, run + tot)
        z = jnp.int32(0)
        _, bsel, below, _ = lax.fori_loop(0, _NB // _L, sj, (z, z, z, z))
        return bsel, below

    def hist_pass(shift, mask, pshift, prefix):
        lax.fori_loop(0, _NB, clear_hist, 0)
        def pi(i, c):
            b = bits_v[pl.ds(i * _L, _L)]
            binv = (b >> shift) & mask
            m = (b >> pshift) == prefix
            plsc.addupdate_scatter(hist_v, [lane * _NB + binv], ones, mask=m)
            return c
        lax.fori_loop(0, nvec, pi, 0)

    def per_query(lq, carry):
        q = wid * qpt + lq
        qbase = (q // _L) * _L
        qoff = jnp.full((_L,), q - qbase, jnp.int32)
        yx = _lane_gather(yx_v[pl.ds(qbase, _L)], qoff)
        yy = _lane_gather(yy_v[pl.ds(qbase, _L)], qoff)
        yz = _lane_gather(yz_v[pl.ds(qbase, _L)], qoff)

        lax.fori_loop(0, _NB, clear_hist, 0)
        def p1(i, c):
            dx = px_v[pl.ds(i * _L, _L)] - yx
            dy = py_v[pl.ds(i * _L, _L)] - yy
            dz = pz_v[pl.ds(i * _L, _L)] - yz
            d2 = dx * dx + dy * dy + dz * dz
            b = lax.bitcast_convert_type(d2, jnp.int32)
            bits_v[pl.ds(i * _L, _L)] = b
            plsc.addupdate_scatter(hist_v, [lane * _NB + (b >> 22)], ones)
            return c
        lax.fori_loop(0, nvec, p1, 0)

        k0 = jnp.int32(k - 1)
        b1, below1 = scan_hist(k0)
        k1 = k0 - below1

        hist_pass(14, 0xFF, 22, b1)
        b2, below2 = scan_hist(k1)
        k2 = k1 - below2
        pre2 = (b1 << 8) | b2

        hist_pass(6, 0xFF, 14, pre2)
        b3, below3 = scan_hist(k2)
        k3 = k2 - below3
        pre3 = (pre2 << 8) | b3

        hist_pass(0, 0x3F, 6, pre3)
        b4, below4 = scan_hist(k3)

        t = (pre3 << 6) | b4
        count_lt = below1 + below2 + below3 + below4

        def fp(i, st):
            lt_base, eq_base = st
            b = bits_v[pl.ds(i * _L, _L)]
            lt = b < t
            eq = b == t
            lt_i = lt.astype(jnp.int32)
            eq_i = eq.astype(jnp.int32)
            pos_lt = lt_base + plsc.cumsum(lt_i) - 1
            pos_eq = eq_base + plsc.cumsum(eq_i) - 1
            idx_v = i * _L + lane
            plsc.store_scatter(
                row_v, [jnp.minimum(pos_lt, k - 1)], idx_v, mask=lt)
            eqm = eq & (pos_eq < k)
            plsc.store_scatter(
                row_v, [jnp.minimum(pos_eq, k - 1)], idx_v, mask=eqm)
            return (lt_base + jnp.sum(lt_i), eq_base + jnp.sum(eq_i))
        lax.fori_loop(0, nvec, fp, (jnp.int32(0), count_lt))

        pltpu.sync_copy(row_v, out_h.at[q])
        return carry

    lax.fori_loop(0, qpt, per_query, 0)


def _ballq_sc(y_pos, pos, k):
    m = y_pos.shape[0]
    n = pos.shape[0]
    qpt = m // 32
    kout = max(k, _L)
    mesh = plsc.VectorSubcoreMesh(core_axis_name="c", subcore_axis_name="s")
    fn = functools.partial(
        pl.kernel,
        mesh=mesh,
        compiler_params=pltpu.CompilerParams(needs_layout_passes=False),
        out_type=jax.ShapeDtypeStruct((m, kout), jnp.int32),
        scratch_types=[
            pltpu.VMEM((n,), jnp.float32),
            pltpu.VMEM((n,), jnp.float32),
            pltpu.VMEM((n,), jnp.float32),
            pltpu.VMEM((m,), jnp.float32),
            pltpu.VMEM((m,), jnp.float32),
            pltpu.VMEM((m,), jnp.float32),
            pltpu.VMEM((n,), jnp.int32),
            pltpu.VMEM((_NB * _L,), jnp.int32),
            pltpu.VMEM((kout,), jnp.int32),
        ],
    )(functools.partial(_ballq_tec, n, k, qpt))
    out = fn(pos[:, 0], pos[:, 1], pos[:, 2],
             y_pos[:, 0], y_pos[:, 1], y_pos[:, 2])
    return out[:, :k] if kout != k else out


def _fps_body(m, px_ref, py_ref, pz_ref, out_ref):
    px = px_ref[...]
    py = py_ref[...]
    pz = pz_ref[...]
    r = px.shape[0]
    row = jax.lax.broadcasted_iota(jnp.int32, (r, 128), 0)
    colv = jax.lax.broadcasted_iota(jnp.int32, (r, 128), 1)
    flat = row * 128 + colv
    out_ref[...] = jnp.zeros(out_ref.shape, jnp.int32)
    dists0 = jnp.full((r, 128), jnp.inf, jnp.float32)

    def body(i, carry):
        dists, last = carry
        sel = flat == last
        lx = jnp.sum(jnp.where(sel, px, 0.0))
        ly = jnp.sum(jnp.where(sel, py, 0.0))
        lz = jnp.sum(jnp.where(sel, pz, 0.0))
        dxx = px - lx
        dyy = py - ly
        dzz = pz - lz
        d = dxx * dxx + dyy * dyy + dzz * dzz
        dists = jnp.minimum(dists, d)
        mx = jnp.max(dists)
        idx = jnp.min(jnp.where(dists == mx, flat, jnp.int32(2 ** 30)))
        out_ref[pl.ds(i, 1), :] = jnp.reshape(idx, (1, 1))
        return (dists, idx)

    jax.lax.fori_loop(1, m, body, (dists0, jnp.int32(0)))


def _fps_idx(pos, num_samples):
    n = pos.shape[0]
    r = n // 128
    px = pos[:, 0].reshape(r, 128)
    py = pos[:, 1].reshape(r, 128)
    pz = pos[:, 2].reshape(r, 128)
    out = pl.pallas_call(
        functools.partial(_fps_body, num_samples),
        out_shape=jax.ShapeDtypeStruct((num_samples, 1), jnp.int32),
    )(px, py, pz)
    return out.reshape(num_samples)


def _sa_body(nl, kk, r2s, fouts, *refs):
    h_ref, d2_ref = refs[0], refs[1]
    wrefs = refs[2:-1]
    out_ref = refs[-1]
    h0 = h_ref[...]
    d2col = d2_ref[...]
    bm = d2col.shape[0] // kk
    col = 0
    for bi, r2 in enumerate(r2s):
        h = h0
        base = bi * nl * 4
        for li in range(nl):
            w = wrefs[base + li * 4][...]
            b = wrefs[base + li * 4 + 1][...]
            g = wrefs[base + li * 4 + 2][...]
            be = wrefs[base + li * 4 + 3][...]
            h = jnp.maximum(
                jnp.dot(h, w, preferred_element_type=jnp.float32) + b, 0.0)
            h = g * (h * _INV) + be
        fo = fouts[bi]
        penalty = jnp.where(d2col <= r2, 0.0, -jnp.inf)
        h = h + penalty
        o = jnp.max(h.reshape(bm, kk, fo), axis=1)
        o = jnp.where(jnp.isfinite(o), o, 0.0)
        out_ref[:, col:col + fo] = o
        col += fo


def _sa_conv(h_in, d2k, r_list, conv_params, bm):
    m, kk, f = h_in.shape
    h_flat = h_in.reshape(m * kk, f)
    d2col = d2k.reshape(m * kk, 1)
    nl = len(conv_params[0])
    fouts = tuple(int(layers[-1]["W"].shape[1]) for layers in conv_params)
    r2s = tuple(np.float32(r * r) for r in r_list)
    wargs, wspecs = [], []
    for layers in conv_params:
        for lyr in layers:
            for nm in ("W", "b", "gamma", "beta"):
                a = lyr[nm]
                if a.ndim == 1:
                    a = a.reshape(1, -1)
                wargs.append(a)
                wspecs.append(pl.BlockSpec(a.shape, lambda i: (0, 0)))
    out_f = sum(fouts)
    grid = (m // bm,)
    fn = pl.pallas_call(
        functools.partial(_sa_body, nl, kk, r2s, fouts),
        grid=grid,
        in_specs=[
            pl.BlockSpec((bm * kk, f), lambda i: (i, 0)),
            pl.BlockSpec((bm * kk, 1), lambda i: (i, 0)),
        ] + wspecs,
        out_specs=pl.BlockSpec((bm, out_f), lambda i: (i, 0)),
        out_shape=jax.ShapeDtypeStruct((m, out_f), jnp.float32),
    )
    return fn(h_flat, d2col, *wargs)


def _sa_module(x, pos, ratio, r_list, conv_params, bm, max_nbrs=128):
    n = pos.shape[0]
    m = int(round(ratio * n))
    idx = _fps_idx(pos, m)
    y_pos = pos[idx]
    nbr = _ballq_sc(y_pos, pos, max_nbrs)
    x_j = x[nbr]
    rel = pos[nbr] - y_pos[:, None, :]
    d2k = jnp.sum(rel ** 2, axis=-1)
    h_in = jnp.concatenate([x_j, rel], axis=-1)
    return _sa_conv(h_in, d2k, r_list, conv_params, bm), y_pos


def _mlp_body(nl, with_head, *refs):
    h_ref = refs[0]
    wrefs = refs[1:-1]
    out_ref = refs[-1]
    h = h_ref[...]
    for li in range(nl):
        w = wrefs[li * 4][...]
        b = wrefs[li * 4 + 1][...]
        g = wrefs[li * 4 + 2][...]
        be = wrefs[li * 4 + 3][...]
        h = jnp.maximum(
            jnp.dot(h, w, preferred_element_type=jnp.float32) + b, 0.0)
        h = g * (h * _INV) + be
    if with_head:
        base = nl * 4
        w1, b1 = wrefs[base][...], wrefs[base + 1][...]
        w2, b2 = wrefs[base + 2][...], wrefs[base + 3][...]
        w3, b3 = wrefs[base + 4][...], wrefs[base + 5][...]
        h = jnp.maximum(jnp.dot(h, w1, preferred_element_type=jnp.float32) + b1, 0.0)
        h = jnp.maximum(jnp.dot(h, w2, preferred_element_type=jnp.float32) + b2, 0.0)
        h = jnp.dot(h, w3, preferred_element_type=jnp.float32) + b3
        mx = jnp.max(h, axis=-1, keepdims=True)
        sh = h - jax.lax.stop_gradient(mx)
        h = sh - jnp.log(jnp.sum(jnp.exp(sh), axis=-1, keepdims=True))
    out_ref[...] = h


def _mlp_rows(h, layers, br, head=None):
    rows, f = h.shape
    nl = len(layers)
    wargs, wspecs = [], []
    for lyr in layers:
        for nm in ("W", "b", "gamma", "beta"):
            a = lyr[nm]
            if a.ndim == 1:
                a = a.reshape(1, -1)
            wargs.append(a)
            wspecs.append(pl.BlockSpec(a.shape, lambda i: (0, 0)))
    if head is not None:
        for nm in ("W1", "b1", "W2", "b2", "W3", "b3"):
            a = head[nm]
            if a.ndim == 1:
                a = a.reshape(1, -1)
            wargs.append(a)
            wspecs.append(pl.BlockSpec(a.shape, lambda i: (0, 0)))
        out_f = head["W3"].shape[1]
    else:
        out_f = layers[-1]["W"].shape[1]
    fn = pl.pallas_call(
        functools.partial(_mlp_body, nl, head is not None),
        grid=(rows // br,),
        in_specs=[pl.BlockSpec((br, f), lambda i: (i, 0))] + wspecs,
        out_specs=pl.BlockSpec((br, out_f), lambda i: (i, 0)),
        out_shape=jax.ShapeDtypeStruct((rows, out_f), jnp.float32),
    )
    return fn(h, *wargs)


def _knn_interpolate(xf, posc, pos_skip, k=3):
    idx = _ballq_sc(pos_skip, posc, k)
    diff = pos_skip[:, None, :] - posc[idx]
    d2 = jnp.sum(diff ** 2, axis=-1)
    w = 1.0 / jnp.maximum(d2, 1e-16)
    w = w / jnp.sum(w, axis=1, keepdims=True)
    return jnp.sum(xf[idx] * w[:, :, None], axis=1)


def kernel(x, pos, batch, params):
    x1, pos1 = _sa_module(x, pos, 0.25, [0.05, 0.4], params["sa1"], bm=32)
    x2, pos2 = _sa_module(x1, pos1, 0.25, [0.2, 0.8], params["sa2"], bm=32)
    x3, pos3 = _sa_module(x2, pos2, 0.25, [0.4, 1.6], params["sa3"], bm=32)

    xi3 = _knn_interpolate(x3, pos3, pos2)
    f3 = _mlp_rows(jnp.concatenate([xi3, x2], axis=1), params["fp3"], br=256)
    xi2 = _knn_interpolate(f3, pos2, pos1)
    f2 = _mlp_rows(jnp.concatenate([xi2, x1], axis=1), params["fp2"], br=512)
    xi1 = _knn_interpolate(f2, pos1, pos)
    f1 = _mlp_rows(jnp.concatenate([xi1, x], axis=1), params["fp1"], br=1024)

    return _mlp_rows(f1, [], br=1024, head=params["cls"])


# speedup vs baseline: 1.5226x; 1.5226x over previous
import functools

import jax
import jax.numpy as jnp
import numpy as np
from jax import lax
from jax.experimental import pallas as pl
from jax.experimental.pallas import tpu as pltpu
from jax.experimental.pallas import tpu_sc as plsc

_EPS_BN = 1e-5
_INV = np.float32(1.0) / np.sqrt(np.float32(1.0 + _EPS_BN))

_L = 16
_NB = 272


def _lane_gather(vec, idx):
    return lax.gather(
        vec, idx[:, None],
        dimension_numbers=lax.GatherDimensionNumbers(
            offset_dims=(), collapsed_slice_dims=(0,), start_index_map=(0,)),
        slice_sizes=(1,),
        mode=lax.GatherScatterMode.PROMISE_IN_BOUNDS)


def _ballq_tec(n, k, qpt, *refs):
    (px_h, py_h, pz_h, yx_h, yy_h, yz_h, out_h,
     px_v, py_v, pz_v, yx_v, yy_v, yz_v, bits_v, hist_v, row_v) = refs
    nvec = n // _L
    wid = lax.axis_index("s") * 2 + lax.axis_index("c")

    pltpu.sync_copy(px_h, px_v)
    pltpu.sync_copy(py_h, py_v)
    pltpu.sync_copy(pz_h, pz_v)
    pltpu.sync_copy(yx_h, yx_v)
    pltpu.sync_copy(yy_h, yy_v)
    pltpu.sync_copy(yz_h, yz_v)

    lane = lax.iota(jnp.int32, _L)
    ones = jnp.full((_L,), 1, jnp.int32)
    _U = 4
    zeros = jnp.zeros((_L,), jnp.int32)

    def clear_hist(j, c):
        for u in range(_U):
            hist_v[pl.ds((j * _U + u) * _L, _L)] = zeros
        return c

    def scan_hist(k_rem, nbins):
        def sj(j, st):
            found, bsel, below, run = st
            acc = jnp.zeros((_L,), jnp.int32)
            for l in range(_L):
                acc = acc + hist_v[pl.ds(l * _NB + j * _L, _L)]
            tot = jnp.sum(acc)
            cum = plsc.cumsum(acc) + run
            hit = cum > k_rem
            nhit = jnp.sum(hit.astype(jnp.int32))
            ffs = plsc.all_reduce_ffs(hit)
            excl = cum - acc
            below_here = jnp.sum(jnp.where(lane == ffs, excl, 0))
            bin_here = j * _L + jnp.max(ffs)
            take = (found == 0) & (nhit > 0)
            bsel = jnp.where(take, bin_here, bsel)
            below = jnp.where(take, below_here, below)
            found = jnp.where(nhit > 0, 1, found)
            return (found, bsel, below, run + tot)
        z = jnp.int32(0)
        _, bsel, below, _ = lax.fori_loop(0, nbins // _L, sj, (z, z, z, z))
        return bsel, below

    def hist_pass(shift, mask, pshift, prefix):
        lax.fori_loop(0, _NB // _U, clear_hist, 0)
        def pi(i, c):
            for u in range(_U):
                b = bits_v[pl.ds((i * _U + u) * _L, _L)]
                binv = (b >> shift) & mask
                m = (b >> pshift) == prefix
                plsc.addupdate_scatter(
                    hist_v, [lane * _NB + binv], ones, mask=m)
            return c
        lax.fori_loop(0, nvec // _U, pi, 0)

    def per_query(lq, carry):
        q = wid * qpt + lq
        qbase = (q // _L) * _L
        qoff = jnp.full((_L,), q - qbase, jnp.int32)
        yx = _lane_gather(yx_v[pl.ds(qbase, _L)], qoff)
        yy = _lane_gather(yy_v[pl.ds(qbase, _L)], qoff)
        yz = _lane_gather(yz_v[pl.ds(qbase, _L)], qoff)

        lax.fori_loop(0, _NB // _U, clear_hist, 0)
        def p1(i, c):
            for u in range(_U):
                sl = pl.ds((i * _U + u) * _L, _L)
                dx = px_v[sl] - yx
                dy = py_v[sl] - yy
                dz = pz_v[sl] - yz
                d2 = dx * dx + dy * dy + dz * dz
                b = lax.bitcast_convert_type(d2, jnp.int32)
                bits_v[sl] = b
                plsc.addupdate_scatter(hist_v, [lane * _NB + (b >> 22)], ones)
            return c
        lax.fori_loop(0, nvec // _U, p1, 0)

        k0 = jnp.int32(k - 1)
        b1, below1 = scan_hist(k0, _NB)
        k1 = k0 - below1

        hist_pass(14, 0xFF, 22, b1)
        b2, below2 = scan_hist(k1, 256)
        k2 = k1 - below2
        pre2 = (b1 << 8) | b2

        hist_pass(6, 0xFF, 14, pre2)
        b3, below3 = scan_hist(k2, 256)
        k3 = k2 - below3
        pre3 = (pre2 << 8) | b3

        hist_pass(0, 0x3F, 6, pre3)
        b4, below4 = scan_hist(k3, 64)

        t = (pre3 << 6) | b4
        count_lt = below1 + below2 + below3 + below4

        def fp(i, st):
            bs = [bits_v[pl.ds((i * _U + u) * _L, _L)] for u in range(_U)]
            sel = jnp.int32(0)
            for u in range(_U):
                sel = sel + jnp.sum((bs[u] <= t).astype(jnp.int32))

            def emit(st):
                lt_base, eq_base = st
                for u in range(_U):
                    b = bs[u]
                    lt = b < t
                    eq = b == t
                    lt_i = lt.astype(jnp.int32)
                    eq_i = eq.astype(jnp.int32)
                    pos_lt = lt_base + plsc.cumsum(lt_i) - 1
                    pos_eq = eq_base + plsc.cumsum(eq_i) - 1
                    idx_v = (i * _U + u) * _L + lane
                    plsc.store_scatter(
                        row_v, [jnp.minimum(pos_lt, k - 1)], idx_v, mask=lt)
                    eqm = eq & (pos_eq < k)
                    plsc.store_scatter(
                        row_v, [jnp.minimum(pos_eq, k - 1)], idx_v, mask=eqm)
                    lt_base = lt_base + jnp.sum(lt_i)
                    eq_base = eq_base + jnp.sum(eq_i)
                return (lt_base, eq_base)

            return lax.cond(sel > 0, emit, lambda s: s, st)
        lax.fori_loop(0, nvec // _U, fp, (jnp.int32(0), count_lt))

        pltpu.sync_copy(row_v, out_h.at[q])
        return carry

    lax.fori_loop(0, qpt, per_query, 0)


def _ballq_sc(y_pos, pos, k):
    m = y_pos.shape[0]
    n = pos.shape[0]
    qpt = m // 32
    kout = max(k, _L)
    mesh = plsc.VectorSubcoreMesh(core_axis_name="c", subcore_axis_name="s")
    fn = functools.partial(
        pl.kernel,
        mesh=mesh,
        compiler_params=pltpu.CompilerParams(needs_layout_passes=False),
        out_type=jax.ShapeDtypeStruct((m, kout), jnp.int32),
        scratch_types=[
            pltpu.VMEM((n,), jnp.float32),
            pltpu.VMEM((n,), jnp.float32),
            pltpu.VMEM((n,), jnp.float32),
            pltpu.VMEM((m,), jnp.float32),
            pltpu.VMEM((m,), jnp.float32),
            pltpu.VMEM((m,), jnp.float32),
            pltpu.VMEM((n,), jnp.int32),
            pltpu.VMEM((_NB * _L,), jnp.int32),
            pltpu.VMEM((kout,), jnp.int32),
        ],
    )(functools.partial(_ballq_tec, n, k, qpt))
    out = fn(pos[:, 0], pos[:, 1], pos[:, 2],
             y_pos[:, 0], y_pos[:, 1], y_pos[:, 2])
    return out[:, :k] if kout != k else out


def _fps_body(m, px_ref, py_ref, pz_ref, out_ref):
    px = px_ref[...]
    py = py_ref[...]
    pz = pz_ref[...]
    r = px.shape[0]
    row = jax.lax.broadcasted_iota(jnp.int32, (r, 128), 0)
    colv = jax.lax.broadcasted_iota(jnp.int32, (r, 128), 1)
    flat = row * 128 + colv
    out_ref[...] = jnp.zeros(out_ref.shape, jnp.int32)
    dists0 = jnp.full((r, 128), jnp.inf, jnp.float32)

    def body(i, carry):
        dists, last = carry
        sel = flat == last
        lx = jnp.sum(jnp.where(sel, px, 0.0))
        ly = jnp.sum(jnp.where(sel, py, 0.0))
        lz = jnp.sum(jnp.where(sel, pz, 0.0))
        dxx = px - lx
        dyy = py - ly
        dzz = pz - lz
        d = dxx * dxx + dyy * dyy + dzz * dzz
        dists = jnp.minimum(dists, d)
        mx = jnp.max(dists)
        idx = jnp.min(jnp.where(dists == mx, flat, jnp.int32(2 ** 30)))
        out_ref[pl.ds(i, 1), :] = jnp.reshape(idx, (1, 1))
        return (dists, idx)

    jax.lax.fori_loop(1, m, body, (dists0, jnp.int32(0)))


def _fps_idx(pos, num_samples):
    n = pos.shape[0]
    r = n // 128
    px = pos[:, 0].reshape(r, 128)
    py = pos[:, 1].reshape(r, 128)
    pz = pos[:, 2].reshape(r, 128)
    out = pl.pallas_call(
        functools.partial(_fps_body, num_samples),
        out_shape=jax.ShapeDtypeStruct((num_samples, 1), jnp.int32),
    )(px, py, pz)
    return out.reshape(num_samples)


def _sa_body(nl, kk, r2s, fouts, *refs):
    h_ref, d2_ref = refs[0], refs[1]
    wrefs = refs[2:-1]
    out_ref = refs[-1]
    h0 = h_ref[...]
    d2col = d2_ref[...]
    bm = d2col.shape[0] // kk
    col = 0
    for bi, r2 in enumerate(r2s):
        h = h0
        base = bi * nl * 4
        for li in range(nl):
            w = wrefs[base + li * 4][...]
            b = wrefs[base + li * 4 + 1][...]
            g = wrefs[base + li * 4 + 2][...]
            be = wrefs[base + li * 4 + 3][...]
            h = jnp.maximum(
                jnp.dot(h, w, preferred_element_type=jnp.float32) + b, 0.0)
            h = g * (h * _INV) + be
        fo = fouts[bi]
        penalty = jnp.where(d2col <= r2, 0.0, -jnp.inf)
        h = h + penalty
        o = jnp.max(h.reshape(bm, kk, fo), axis=1)
        o = jnp.where(jnp.isfinite(o), o, 0.0)
        out_ref[:, col:col + fo] = o
        col += fo


def _sa_conv(h_in, d2k, r_list, conv_params, bm):
    m, kk, f = h_in.shape
    h_flat = h_in.reshape(m * kk, f)
    d2col = d2k.reshape(m * kk, 1)
    nl = len(conv_params[0])
    fouts = tuple(int(layers[-1]["W"].shape[1]) for layers in conv_params)
    r2s = tuple(np.float32(r * r) for r in r_list)
    wargs, wspecs = [], []
    for layers in conv_params:
        for lyr in layers:
            for nm in ("W", "b", "gamma", "beta"):
                a = lyr[nm]
                if a.ndim == 1:
                    a = a.reshape(1, -1)
                wargs.append(a)
                wspecs.append(pl.BlockSpec(a.shape, lambda i: (0, 0)))
    out_f = sum(fouts)
    grid = (m // bm,)
    fn = pl.pallas_call(
        functools.partial(_sa_body, nl, kk, r2s, fouts),
        grid=grid,
        in_specs=[
            pl.BlockSpec((bm * kk, f), lambda i: (i, 0)),
            pl.BlockSpec((bm * kk, 1), lambda i: (i, 0)),
        ] + wspecs,
        out_specs=pl.BlockSpec((bm, out_f), lambda i: (i, 0)),
        out_shape=jax.ShapeDtypeStruct((m, out_f), jnp.float32),
    )
    return fn(h_flat, d2col, *wargs)


def _sa_module(x, pos, ratio, r_list, conv_params, bm, max_nbrs=128):
    n = pos.shape[0]
    m = int(round(ratio * n))
    idx = _fps_idx(pos, m)
    y_pos = pos[idx]
    nbr = _ballq_sc(y_pos, pos, max_nbrs)
    x_j = x[nbr]
    rel = pos[nbr] - y_pos[:, None, :]
    d2k = jnp.sum(rel ** 2, axis=-1)
    h_in = jnp.concatenate([x_j, rel], axis=-1)
    return _sa_conv(h_in, d2k, r_list, conv_params, bm), y_pos


def _mlp_body(nl, with_head, *refs):
    h_ref = refs[0]
    wrefs = refs[1:-1]
    out_ref = refs[-1]
    h = h_ref[...]
    for li in range(nl):
        w = wrefs[li * 4][...]
        b = wrefs[li * 4 + 1][...]
        g = wrefs[li * 4 + 2][...]
        be = wrefs[li * 4 + 3][...]
        h = jnp.maximum(
            jnp.dot(h, w, preferred_element_type=jnp.float32) + b, 0.0)
        h = g * (h * _INV) + be
    if with_head:
        base = nl * 4
        w1, b1 = wrefs[base][...], wrefs[base + 1][...]
        w2, b2 = wrefs[base + 2][...], wrefs[base + 3][...]
        w3, b3 = wrefs[base + 4][...], wrefs[base + 5][...]
        h = jnp.maximum(jnp.dot(h, w1, preferred_element_type=jnp.float32) + b1, 0.0)
        h = jnp.maximum(jnp.dot(h, w2, preferred_element_type=jnp.float32) + b2, 0.0)
        h = jnp.dot(h, w3, preferred_element_type=jnp.float32) + b3
        mx = jnp.max(h, axis=-1, keepdims=True)
        sh = h - jax.lax.stop_gradient(mx)
        h = sh - jnp.log(jnp.sum(jnp.exp(sh), axis=-1, keepdims=True))
    out_ref[...] = h


def _mlp_rows(h, layers, br, head=None):
    rows, f = h.shape
    nl = len(layers)
    wargs, wspecs = [], []
    for lyr in layers:
        for nm in ("W", "b", "gamma", "beta"):
            a = lyr[nm]
            if a.ndim == 1:
                a = a.reshape(1, -1)
            wargs.append(a)
            wspecs.append(pl.BlockSpec(a.shape, lambda i: (0, 0)))
    if head is not None:
        for nm in ("W1", "b1", "W2", "b2", "W3", "b3"):
            a = head[nm]
            if a.ndim == 1:
                a = a.reshape(1, -1)
            wargs.append(a)
            wspecs.append(pl.BlockSpec(a.shape, lambda i: (0, 0)))
        out_f = head["W3"].shape[1]
    else:
        out_f = layers[-1]["W"].shape[1]
    fn = pl.pallas_call(
        functools.partial(_mlp_body, nl, head is not None),
        grid=(rows // br,),
        in_specs=[pl.BlockSpec((br, f), lambda i: (i, 0))] + wspecs,
        out_specs=pl.BlockSpec((br, out_f), lambda i: (i, 0)),
        out_shape=jax.ShapeDtypeStruct((rows, out_f), jnp.float32),
    )
    return fn(h, *wargs)


def _knn_interpolate(xf, posc, pos_skip, k=3):
    d2_sg = jax.lax.stop_gradient(
        jnp.sum((pos_skip[:, None, :] - posc[None, :, :]) ** 2, axis=-1))
    _, idx = jax.lax.approx_max_k(-d2_sg, k, recall_target=1.0)
    diff = pos_skip[:, None, :] - posc[idx]
    d2 = jnp.sum(diff ** 2, axis=-1)
    w = 1.0 / jnp.maximum(d2, 1e-16)
    w = w / jnp.sum(w, axis=1, keepdims=True)
    return jnp.sum(xf[idx] * w[:, :, None], axis=1)


def kernel(x, pos, batch, params):
    x1, pos1 = _sa_module(x, pos, 0.25, [0.05, 0.4], params["sa1"], bm=32)
    x2, pos2 = _sa_module(x1, pos1, 0.25, [0.2, 0.8], params["sa2"], bm=32)
    x3, pos3 = _sa_module(x2, pos2, 0.25, [0.4, 1.6], params["sa3"], bm=32)

    xi3 = _knn_interpolate(x3, pos3, pos2)
    f3 = _mlp_rows(jnp.concatenate([xi3, x2], axis=1), params["fp3"], br=256)
    xi2 = _knn_interpolate(f3, pos2, pos1)
    f2 = _mlp_rows(jnp.concatenate([xi2, x1], axis=1), params["fp2"], br=512)
    xi1 = _knn_interpolate(f2, pos1, pos)
    f1 = _mlp_rows(jnp.concatenate([xi1, x], axis=1), params["fp1"], br=1024)

    return _mlp_rows(f1, [], br=1024, head=params["cls"])


# speedup vs baseline: 1.5271x; 1.0029x over previous
import functools

import jax
import jax.numpy as jnp
import numpy as np
from jax import lax
from jax.experimental import pallas as pl
from jax.experimental.pallas import tpu as pltpu
from jax.experimental.pallas import tpu_sc as plsc

_EPS_BN = 1e-5
_INV = np.float32(1.0) / np.sqrt(np.float32(1.0 + _EPS_BN))

_L = 16
_NB = 272


def _lane_gather(vec, idx):
    return lax.gather(
        vec, idx[:, None],
        dimension_numbers=lax.GatherDimensionNumbers(
            offset_dims=(), collapsed_slice_dims=(0,), start_index_map=(0,)),
        slice_sizes=(1,),
        mode=lax.GatherScatterMode.PROMISE_IN_BOUNDS)


def _ballq_tec(n, k, qpt, *refs):
    (px_h, py_h, pz_h, yx_h, yy_h, yz_h, out_h,
     px_v, py_v, pz_v, yx_v, yy_v, yz_v, bits_v, cb_v, ci_v,
     hist_v, row_v) = refs
    nvec = n // _L
    wid = lax.axis_index("s") * 2 + lax.axis_index("c")

    pltpu.sync_copy(px_h, px_v)
    pltpu.sync_copy(py_h, py_v)
    pltpu.sync_copy(pz_h, pz_v)
    pltpu.sync_copy(yx_h, yx_v)
    pltpu.sync_copy(yy_h, yy_v)
    pltpu.sync_copy(yz_h, yz_v)

    lane = lax.iota(jnp.int32, _L)
    ones = jnp.full((_L,), 1, jnp.int32)
    _U = 4
    zeros = jnp.zeros((_L,), jnp.int32)

    def clear_hist(j, c):
        for u in range(_U):
            hist_v[pl.ds((j * _U + u) * _L, _L)] = zeros
        return c

    def scan_hist(k_rem, nbins):
        def sj(j, st):
            found, bsel, below, run = st
            acc = jnp.zeros((_L,), jnp.int32)
            for l in range(_L):
                acc = acc + hist_v[pl.ds(l * _NB + j * _L, _L)]
            tot = jnp.sum(acc)
            cum = plsc.cumsum(acc) + run
            hit = cum > k_rem
            nhit = jnp.sum(hit.astype(jnp.int32))
            ffs = plsc.all_reduce_ffs(hit)
            excl = cum - acc
            below_here = jnp.sum(jnp.where(lane == ffs, excl, 0))
            bin_here = j * _L + jnp.max(ffs)
            take = (found == 0) & (nhit > 0)
            bsel = jnp.where(take, bin_here, bsel)
            below = jnp.where(take, below_here, below)
            found = jnp.where(nhit > 0, 1, found)
            return (found, bsel, below, run + tot)
        z = jnp.int32(0)
        _, bsel, below, _ = lax.fori_loop(0, nbins // _L, sj, (z, z, z, z))
        return bsel, below

    def hist_pass_compact(shift, mask, pshift, prefix, cnt):
        lax.fori_loop(0, _NB // _U, clear_hist, 0)
        def pi(i, c):
            b = cb_v[pl.ds(i * _L, _L)]
            valid = (i * _L + lane) < cnt
            binv = (b >> shift) & mask
            m = valid & ((b >> pshift) == prefix)
            plsc.addupdate_scatter(hist_v, [lane * _NB + binv], ones, mask=m)
            return c
        lax.fori_loop(0, (cnt + _L - 1) // _L, pi, 0)

    def per_query(lq, carry):
        q = wid * qpt + lq
        qbase = (q // _L) * _L
        qoff = jnp.full((_L,), q - qbase, jnp.int32)
        yx = _lane_gather(yx_v[pl.ds(qbase, _L)], qoff)
        yy = _lane_gather(yy_v[pl.ds(qbase, _L)], qoff)
        yz = _lane_gather(yz_v[pl.ds(qbase, _L)], qoff)

        lax.fori_loop(0, _NB // _U, clear_hist, 0)
        def p1(i, c):
            for u in range(_U):
                sl = pl.ds((i * _U + u) * _L, _L)
                dx = px_v[sl] - yx
                dy = py_v[sl] - yy
                dz = pz_v[sl] - yz
                d2 = dx * dx + dy * dy + dz * dz
                b = lax.bitcast_convert_type(d2, jnp.int32)
                bits_v[sl] = b
                plsc.addupdate_scatter(hist_v, [lane * _NB + (b >> 22)], ones)
            return c
        lax.fori_loop(0, nvec // _U, p1, 0)

        k0 = jnp.int32(k - 1)
        b1, below1 = scan_hist(k0, _NB)
        k1 = k0 - below1

        def p2(i, st):
            bs = [bits_v[pl.ds((i * _U + u) * _L, _L)] for u in range(_U)]
            sel = jnp.int32(0)
            for u in range(_U):
                sel = sel + jnp.sum(((bs[u] >> 22) <= b1).astype(jnp.int32))

            def emit(st):
                a_base, c_base = st
                for u in range(_U):
                    b = bs[u]
                    binv = b >> 22
                    lt1 = binv < b1
                    e1 = binv == b1
                    lt_i = lt1.astype(jnp.int32)
                    e_i = e1.astype(jnp.int32)
                    pos_a = a_base + plsc.cumsum(lt_i) - 1
                    pos_c = c_base + plsc.cumsum(e_i) - 1
                    idx_v = (i * _U + u) * _L + lane
                    plsc.store_scatter(
                        row_v, [jnp.minimum(pos_a, k - 1)], idx_v, mask=lt1)
                    plsc.store_scatter(ci_v, [pos_c], idx_v, mask=e1)
                    plsc.store_scatter(cb_v, [pos_c], b, mask=e1)
                    a_base = a_base + jnp.sum(lt_i)
                    c_base = c_base + jnp.sum(e_i)
                return (a_base, c_base)

            return lax.cond(sel > 0, emit, lambda s: s, st)
        _, cnt = lax.fori_loop(0, nvec // _U, p2,
                               (jnp.int32(0), jnp.int32(0)))

        hist_pass_compact(14, 0xFF, 22, b1, cnt)
        b2, below2 = scan_hist(k1, 256)
        k2 = k1 - below2
        pre2 = (b1 << 8) | b2

        hist_pass_compact(6, 0xFF, 14, pre2, cnt)
        b3, below3 = scan_hist(k2, 256)
        k3 = k2 - below3
        pre3 = (pre2 << 8) | b3

        hist_pass_compact(0, 0x3F, 6, pre3, cnt)
        b4, below4 = scan_hist(k3, 64)

        t = (pre3 << 6) | b4
        count_lt = below1 + below2 + below3 + below4

        def fp(i, st):
            lt_base, eq_base = st
            b = cb_v[pl.ds(i * _L, _L)]
            valid = (i * _L + lane) < cnt
            lt = valid & (b < t)
            eq = valid & (b == t)
            lt_i = lt.astype(jnp.int32)
            eq_i = eq.astype(jnp.int32)
            pos_lt = lt_base + plsc.cumsum(lt_i) - 1
            pos_eq = eq_base + plsc.cumsum(eq_i) - 1
            idx_v = ci_v[pl.ds(i * _L, _L)]
            plsc.store_scatter(
                row_v, [jnp.minimum(pos_lt, k - 1)], idx_v, mask=lt)
            eqm = eq & (pos_eq < k)
            plsc.store_scatter(
                row_v, [jnp.minimum(pos_eq, k - 1)], idx_v, mask=eqm)
            return (lt_base + jnp.sum(lt_i), eq_base + jnp.sum(eq_i))
        lax.fori_loop(0, (cnt + _L - 1) // _L, fp, (below1, count_lt))

        pltpu.sync_copy(row_v, out_h.at[q])
        return carry

    lax.fori_loop(0, qpt, per_query, 0)


def _ballq_sc(y_pos, pos, k):
    m = y_pos.shape[0]
    n = pos.shape[0]
    qpt = m // 32
    kout = max(k, _L)
    mesh = plsc.VectorSubcoreMesh(core_axis_name="c", subcore_axis_name="s")
    fn = functools.partial(
        pl.kernel,
        mesh=mesh,
        compiler_params=pltpu.CompilerParams(needs_layout_passes=False),
        out_type=jax.ShapeDtypeStruct((m, kout), jnp.int32),
        scratch_types=[
            pltpu.VMEM((n,), jnp.float32),
            pltpu.VMEM((n,), jnp.float32),
            pltpu.VMEM((n,), jnp.float32),
            pltpu.VMEM((m,), jnp.float32),
            pltpu.VMEM((m,), jnp.float32),
            pltpu.VMEM((m,), jnp.float32),
            pltpu.VMEM((n,), jnp.int32),
            pltpu.VMEM((n,), jnp.int32),
            pltpu.VMEM((n,), jnp.int32),
            pltpu.VMEM((_NB * _L,), jnp.int32),
            pltpu.VMEM((kout,), jnp.int32),
        ],
    )(functools.partial(_ballq_tec, n, k, qpt))
    out = fn(pos[:, 0], pos[:, 1], pos[:, 2],
             y_pos[:, 0], y_pos[:, 1], y_pos[:, 2])
    return out[:, :k] if kout != k else out


def _fps_body(m, px_ref, py_ref, pz_ref, out_ref):
    px = px_ref[...]
    py = py_ref[...]
    pz = pz_ref[...]
    r = px.shape[0]
    row = jax.lax.broadcasted_iota(jnp.int32, (r, 128), 0)
    colv = jax.lax.broadcasted_iota(jnp.int32, (r, 128), 1)
    flat = row * 128 + colv
    out_ref[...] = jnp.zeros(out_ref.shape, jnp.int32)
    dists0 = jnp.full((r, 128), jnp.inf, jnp.float32)

    def body(i, carry):
        dists, last = carry
        sel = flat == last
        lx = jnp.sum(jnp.where(sel, px, 0.0))
        ly = jnp.sum(jnp.where(sel, py, 0.0))
        lz = jnp.sum(jnp.where(sel, pz, 0.0))
        dxx = px - lx
        dyy = py - ly
        dzz = pz - lz
        d = dxx * dxx + dyy * dyy + dzz * dzz
        dists = jnp.minimum(dists, d)
        mx = jnp.max(dists)
        idx = jnp.min(jnp.where(dists == mx, flat, jnp.int32(2 ** 30)))
        out_ref[pl.ds(i, 1), :] = jnp.reshape(idx, (1, 1))
        return (dists, idx)

    jax.lax.fori_loop(1, m, body, (dists0, jnp.int32(0)))


def _fps_idx(pos, num_samples):
    n = pos.shape[0]
    r = n // 128
    px = pos[:, 0].reshape(r, 128)
    py = pos[:, 1].reshape(r, 128)
    pz = pos[:, 2].reshape(r, 128)
    out = pl.pallas_call(
        functools.partial(_fps_body, num_samples),
        out_shape=jax.ShapeDtypeStruct((num_samples, 1), jnp.int32),
    )(px, py, pz)
    return out.reshape(num_samples)


def _sa_body(nl, kk, r2s, fouts, *refs):
    h_ref, d2_ref = refs[0], refs[1]
    wrefs = refs[2:-1]
    out_ref = refs[-1]
    h0 = h_ref[...]
    d2col = d2_ref[...]
    bm = d2col.shape[0] // kk
    col = 0
    for bi, r2 in enumerate(r2s):
        h = h0
        base = bi * nl * 4
        for li in range(nl):
            w = wrefs[base + li * 4][...]
            b = wrefs[base + li * 4 + 1][...]
            g = wrefs[base + li * 4 + 2][...]
            be = wrefs[base + li * 4 + 3][...]
            h = jnp.maximum(
                jnp.dot(h, w, preferred_element_type=jnp.float32) + b, 0.0)
            h = g * (h * _INV) + be
        fo = fouts[bi]
        penalty = jnp.where(d2col <= r2, 0.0, -jnp.inf)
        h = h + penalty
        o = jnp.max(h.reshape(bm, kk, fo), axis=1)
        o = jnp.where(jnp.isfinite(o), o, 0.0)
        out_ref[:, col:col + fo] = o
        col += fo


def _sa_conv(h_in, d2k, r_list, conv_params, bm):
    m, kk, f = h_in.shape
    h_flat = h_in.reshape(m * kk, f)
    d2col = d2k.reshape(m * kk, 1)
    nl = len(conv_params[0])
    fouts = tuple(int(layers[-1]["W"].shape[1]) for layers in conv_params)
    r2s = tuple(np.float32(r * r) for r in r_list)
    wargs, wspecs = [], []
    for layers in conv_params:
        for lyr in layers:
            for nm in ("W", "b", "gamma", "beta"):
                a = lyr[nm]
                if a.ndim == 1:
                    a = a.reshape(1, -1)
                wargs.append(a)
                wspecs.append(pl.BlockSpec(a.shape, lambda i: (0, 0)))
    out_f = sum(fouts)
    grid = (m // bm,)
    fn = pl.pallas_call(
        functools.partial(_sa_body, nl, kk, r2s, fouts),
        grid=grid,
        in_specs=[
            pl.BlockSpec((bm * kk, f), lambda i: (i, 0)),
            pl.BlockSpec((bm * kk, 1), lambda i: (i, 0)),
        ] + wspecs,
        out_specs=pl.BlockSpec((bm, out_f), lambda i: (i, 0)),
        out_shape=jax.ShapeDtypeStruct((m, out_f), jnp.float32),
    )
    return fn(h_flat, d2col, *wargs)


def _sa_module(x, pos, ratio, r_list, conv_params, bm, max_nbrs=128):
    n = pos.shape[0]
    m = int(round(ratio * n))
    idx = _fps_idx(pos, m)
    y_pos = pos[idx]
    nbr = _ballq_sc(y_pos, pos, max_nbrs)
    x_j = x[nbr]
    rel = pos[nbr] - y_pos[:, None, :]
    d2k = jnp.sum(rel ** 2, axis=-1)
    h_in = jnp.concatenate([x_j, rel], axis=-1)
    return _sa_conv(h_in, d2k, r_list, conv_params, bm), y_pos


def _mlp_body(nl, with_head, *refs):
    h_ref = refs[0]
    wrefs = refs[1:-1]
    out_ref = refs[-1]
    h = h_ref[...]
    for li in range(nl):
        w = wrefs[li * 4][...]
        b = wrefs[li * 4 + 1][...]
        g = wrefs[li * 4 + 2][...]
        be = wrefs[li * 4 + 3][...]
        h = jnp.maximum(
            jnp.dot(h, w, preferred_element_type=jnp.float32) + b, 0.0)
        h = g * (h * _INV) + be
    if with_head:
        base = nl * 4
        w1, b1 = wrefs[base][...], wrefs[base + 1][...]
        w2, b2 = wrefs[base + 2][...], wrefs[base + 3][...]
        w3, b3 = wrefs[base + 4][...], wrefs[base + 5][...]
        h = jnp.maximum(jnp.dot(h, w1, preferred_element_type=jnp.float32) + b1, 0.0)
        h = jnp.maximum(jnp.dot(h, w2, preferred_element_type=jnp.float32) + b2, 0.0)
        h = jnp.dot(h, w3, preferred_element_type=jnp.float32) + b3
        mx = jnp.max(h, axis=-1, keepdims=True)
        sh = h - jax.lax.stop_gradient(mx)
        h = sh - jnp.log(jnp.sum(jnp.exp(sh), axis=-1, keepdims=True))
    out_ref[...] = h


def _mlp_rows(h, layers, br, head=None):
    rows, f = h.shape
    nl = len(layers)
    wargs, wspecs = [], []
    for lyr in layers:
        for nm in ("W", "b", "gamma", "beta"):
            a = lyr[nm]
            if a.ndim == 1:
                a = a.reshape(1, -1)
            wargs.append(a)
            wspecs.append(pl.BlockSpec(a.shape, lambda i: (0, 0)))
    if head is not None:
        for nm in ("W1", "b1", "W2", "b2", "W3", "b3"):
            a = head[nm]
            if a.ndim == 1:
                a = a.reshape(1, -1)
            wargs.append(a)
            wspecs.append(pl.BlockSpec(a.shape, lambda i: (0, 0)))
        out_f = head["W3"].shape[1]
    else:
        out_f = layers[-1]["W"].shape[1]
    fn = pl.pallas_call(
        functools.partial(_mlp_body, nl, head is not None),
        grid=(rows // br,),
        in_specs=[pl.BlockSpec((br, f), lambda i: (i, 0))] + wspecs,
        out_specs=pl.BlockSpec((br, out_f), lambda i: (i, 0)),
        out_shape=jax.ShapeDtypeStruct((rows, out_f), jnp.float32),
    )
    return fn(h, *wargs)


def _knn_interpolate(xf, posc, pos_skip, k=3):
    d2_sg = jax.lax.stop_gradient(
        jnp.sum((pos_skip[:, None, :] - posc[None, :, :]) ** 2, axis=-1))
    _, idx = jax.lax.approx_max_k(-d2_sg, k, recall_target=1.0)
    diff = pos_skip[:, None, :] - posc[idx]
    d2 = jnp.sum(diff ** 2, axis=-1)
    w = 1.0 / jnp.maximum(d2, 1e-16)
    w = w / jnp.sum(w, axis=1, keepdims=True)
    return jnp.sum(xf[idx] * w[:, :, None], axis=1)


def kernel(x, pos, batch, params):
    x1, pos1 = _sa_module(x, pos, 0.25, [0.05, 0.4], params["sa1"], bm=32)
    x2, pos2 = _sa_module(x1, pos1, 0.25, [0.2, 0.8], params["sa2"], bm=32)
    x3, pos3 = _sa_module(x2, pos2, 0.25, [0.4, 1.6], params["sa3"], bm=32)

    xi3 = _knn_interpolate(x3, pos3, pos2)
    f3 = _mlp_rows(jnp.concatenate([xi3, x2], axis=1), params["fp3"], br=256)
    xi2 = _knn_interpolate(f3, pos2, pos1)
    f2 = _mlp_rows(jnp.concatenate([xi2, x1], axis=1), params["fp2"], br=512)
    xi1 = _knn_interpolate(f2, pos1, pos)
    f1 = _mlp_rows(jnp.concatenate([xi1, x], axis=1), params["fp1"], br=1024)

    return _mlp_rows(f1, [], br=1024, head=params["cls"])


# speedup vs baseline: 1.5309x; 1.0025x over previous
import functools

import jax
import jax.numpy as jnp
import numpy as np
from jax import lax
from jax.experimental import pallas as pl
from jax.experimental.pallas import tpu as pltpu
from jax.experimental.pallas import tpu_sc as plsc

_EPS_BN = 1e-5
_INV = np.float32(1.0) / np.sqrt(np.float32(1.0 + _EPS_BN))

_L = 16
_NB = 272
_ST = 273
_HW = 4416


def _lane_gather(vec, idx):
    return lax.gather(
        vec, idx[:, None],
        dimension_numbers=lax.GatherDimensionNumbers(
            offset_dims=(), collapsed_slice_dims=(0,), start_index_map=(0,)),
        slice_sizes=(1,),
        mode=lax.GatherScatterMode.PROMISE_IN_BOUNDS)


def _ballq_tec(n, k, qpt, *refs):
    (px_h, py_h, pz_h, yx_h, yy_h, yz_h, out_h,
     px_v, py_v, pz_v, yx_v, yy_v, yz_v, bits_v, cb_v, ci_v,
     hist_v, row_v) = refs
    nvec = n // _L
    wid = lax.axis_index("s") * 2 + lax.axis_index("c")

    pltpu.sync_copy(px_h, px_v)
    pltpu.sync_copy(py_h, py_v)
    pltpu.sync_copy(pz_h, pz_v)
    pltpu.sync_copy(yx_h, yx_v)
    pltpu.sync_copy(yy_h, yy_v)
    pltpu.sync_copy(yz_h, yz_v)

    lane = lax.iota(jnp.int32, _L)
    ones = jnp.full((_L,), 1, jnp.int32)
    _U = 4
    zeros = jnp.zeros((_L,), jnp.int32)

    def clear_hist(j, c):
        for u in range(_U):
            hist_v[pl.ds((j * _U + u) * _L, _L)] = zeros
        return c

    def scan_hist(k_rem, nbins):
        def sj(j, st):
            found, bsel, below, run = st
            acc = jnp.zeros((_L,), jnp.int32)
            for l in range(_L):
                acc = acc + hist_v[pl.ds(l * _ST + j * _L, _L)]
            tot = jnp.sum(acc)
            cum = plsc.cumsum(acc) + run
            hit = cum > k_rem
            nhit = jnp.sum(hit.astype(jnp.int32))
            ffs = plsc.all_reduce_ffs(hit)
            excl = cum - acc
            below_here = jnp.sum(jnp.where(lane == ffs, excl, 0))
            bin_here = j * _L + jnp.max(ffs)
            take = (found == 0) & (nhit > 0)
            bsel = jnp.where(take, bin_here, bsel)
            below = jnp.where(take, below_here, below)
            found = jnp.where(nhit > 0, 1, found)
            return (found, bsel, below, run + tot)
        z = jnp.int32(0)
        _, bsel, below, _ = lax.fori_loop(0, nbins // _L, sj, (z, z, z, z))
        return bsel, below

    def hist_pass_compact(shift, mask, pshift, prefix, cnt):
        lax.fori_loop(0, _HW // (_U * _L), clear_hist, 0)
        def pi(i, c):
            b = cb_v[pl.ds(i * _L, _L)]
            valid = (i * _L + lane) < cnt
            binv = (b >> shift) & mask
            m = valid & ((b >> pshift) == prefix)
            plsc.addupdate_scatter(hist_v, [lane * _ST + binv], ones, mask=m)
            return c
        lax.fori_loop(0, (cnt + _L - 1) // _L, pi, 0)

    def per_query(lq, carry):
        q = wid * qpt + lq
        qbase = (q // _L) * _L
        qoff = jnp.full((_L,), q - qbase, jnp.int32)
        yx = _lane_gather(yx_v[pl.ds(qbase, _L)], qoff)
        yy = _lane_gather(yy_v[pl.ds(qbase, _L)], qoff)
        yz = _lane_gather(yz_v[pl.ds(qbase, _L)], qoff)

        lax.fori_loop(0, _HW // (_U * _L), clear_hist, 0)
        def p1(i, c):
            for u in range(_U):
                sl = pl.ds((i * _U + u) * _L, _L)
                dx = px_v[sl] - yx
                dy = py_v[sl] - yy
                dz = pz_v[sl] - yz
                d2 = dx * dx + dy * dy + dz * dz
                b = lax.bitcast_convert_type(d2, jnp.int32)
                bits_v[sl] = b
                plsc.addupdate_scatter(hist_v, [lane * _ST + (b >> 22)], ones)
            return c
        lax.fori_loop(0, nvec // _U, p1, 0)

        k0 = jnp.int32(k - 1)
        b1, below1 = scan_hist(k0, _NB)
        k1 = k0 - below1

        def p2(i, st):
            bs = [bits_v[pl.ds((i * _U + u) * _L, _L)] for u in range(_U)]
            sel = jnp.int32(0)
            for u in range(_U):
                sel = sel + jnp.sum(((bs[u] >> 22) <= b1).astype(jnp.int32))

            def emit(st):
                a_base, c_base = st
                for u in range(_U):
                    b = bs[u]
                    binv = b >> 22
                    lt1 = binv < b1
                    e1 = binv == b1
                    lt_i = lt1.astype(jnp.int32)
                    e_i = e1.astype(jnp.int32)
                    pos_a = a_base + plsc.cumsum(lt_i) - 1
                    pos_c = c_base + plsc.cumsum(e_i) - 1
                    idx_v = (i * _U + u) * _L + lane
                    plsc.store_scatter(
                        row_v, [jnp.minimum(pos_a, k - 1)], idx_v, mask=lt1)
                    plsc.store_scatter(ci_v, [pos_c], idx_v, mask=e1)
                    plsc.store_scatter(cb_v, [pos_c], b, mask=e1)
                    a_base = a_base + jnp.sum(lt_i)
                    c_base = c_base + jnp.sum(e_i)
                return (a_base, c_base)

            return lax.cond(sel > 0, emit, lambda s: s, st)
        _, cnt = lax.fori_loop(0, nvec // _U, p2,
                               (jnp.int32(0), jnp.int32(0)))

        hist_pass_compact(14, 0xFF, 22, b1, cnt)
        b2, below2 = scan_hist(k1, 256)
        k2 = k1 - below2
        pre2 = (b1 << 8) | b2

        hist_pass_compact(6, 0xFF, 14, pre2, cnt)
        b3, below3 = scan_hist(k2, 256)
        k3 = k2 - below3
        pre3 = (pre2 << 8) | b3

        hist_pass_compact(0, 0x3F, 6, pre3, cnt)
        b4, below4 = scan_hist(k3, 64)

        t = (pre3 << 6) | b4
        count_lt = below1 + below2 + below3 + below4

        def fp(i, st):
            lt_base, eq_base = st
            b = cb_v[pl.ds(i * _L, _L)]
            valid = (i * _L + lane) < cnt
            lt = valid & (b < t)
            eq = valid & (b == t)
            lt_i = lt.astype(jnp.int32)
            eq_i = eq.astype(jnp.int32)
            pos_lt = lt_base + plsc.cumsum(lt_i) - 1
            pos_eq = eq_base + plsc.cumsum(eq_i) - 1
            idx_v = ci_v[pl.ds(i * _L, _L)]
            plsc.store_scatter(
                row_v, [jnp.minimum(pos_lt, k - 1)], idx_v, mask=lt)
            eqm = eq & (pos_eq < k)
            plsc.store_scatter(
                row_v, [jnp.minimum(pos_eq, k - 1)], idx_v, mask=eqm)
            return (lt_base + jnp.sum(lt_i), eq_base + jnp.sum(eq_i))
        lax.fori_loop(0, (cnt + _L - 1) // _L, fp, (below1, count_lt))

        pltpu.sync_copy(row_v, out_h.at[q])
        return carry

    lax.fori_loop(0, qpt, per_query, 0)


def _ballq_sc(y_pos, pos, k):
    m = y_pos.shape[0]
    n = pos.shape[0]
    qpt = m // 32
    kout = max(k, _L)
    mesh = plsc.VectorSubcoreMesh(core_axis_name="c", subcore_axis_name="s")
    fn = functools.partial(
        pl.kernel,
        mesh=mesh,
        compiler_params=pltpu.CompilerParams(needs_layout_passes=False),
        out_type=jax.ShapeDtypeStruct((m, kout), jnp.int32),
        scratch_types=[
            pltpu.VMEM((n,), jnp.float32),
            pltpu.VMEM((n,), jnp.float32),
            pltpu.VMEM((n,), jnp.float32),
            pltpu.VMEM((m,), jnp.float32),
            pltpu.VMEM((m,), jnp.float32),
            pltpu.VMEM((m,), jnp.float32),
            pltpu.VMEM((n,), jnp.int32),
            pltpu.VMEM((n,), jnp.int32),
            pltpu.VMEM((n,), jnp.int32),
            pltpu.VMEM((_HW,), jnp.int32),
            pltpu.VMEM((kout,), jnp.int32),
        ],
    )(functools.partial(_ballq_tec, n, k, qpt))
    out = fn(pos[:, 0], pos[:, 1], pos[:, 2],
             y_pos[:, 0], y_pos[:, 1], y_pos[:, 2])
    return out[:, :k] if kout != k else out


def _fps_body(m, px_ref, py_ref, pz_ref, out_ref):
    px = px_ref[...]
    py = py_ref[...]
    pz = pz_ref[...]
    r = px.shape[0]
    row = jax.lax.broadcasted_iota(jnp.int32, (r, 128), 0)
    colv = jax.lax.broadcasted_iota(jnp.int32, (r, 128), 1)
    flat = row * 128 + colv
    out_ref[...] = jnp.zeros(out_ref.shape, jnp.int32)
    dists0 = jnp.full((r, 128), jnp.inf, jnp.float32)

    def body(i, carry):
        dists, last = carry
        sel = flat == last
        lx = jnp.sum(jnp.where(sel, px, 0.0))
        ly = jnp.sum(jnp.where(sel, py, 0.0))
        lz = jnp.sum(jnp.where(sel, pz, 0.0))
        dxx = px - lx
        dyy = py - ly
        dzz = pz - lz
        d = dxx * dxx + dyy * dyy + dzz * dzz
        dists = jnp.minimum(dists, d)
        mx = jnp.max(dists)
        idx = jnp.min(jnp.where(dists == mx, flat, jnp.int32(2 ** 30)))
        out_ref[pl.ds(i, 1), :] = jnp.reshape(idx, (1, 1))
        return (dists, idx)

    jax.lax.fori_loop(1, m, body, (dists0, jnp.int32(0)))


def _fps_idx(pos, num_samples):
    n = pos.shape[0]
    r = n // 128
    px = pos[:, 0].reshape(r, 128)
    py = pos[:, 1].reshape(r, 128)
    pz = pos[:, 2].reshape(r, 128)
    out = pl.pallas_call(
        functools.partial(_fps_body, num_samples),
        out_shape=jax.ShapeDtypeStruct((num_samples, 1), jnp.int32),
    )(px, py, pz)
    return out.reshape(num_samples)


def _sa_body(nl, kk, r2s, fouts, *refs):
    h_ref, d2_ref = refs[0], refs[1]
    wrefs = refs[2:-1]
    out_ref = refs[-1]
    h0 = h_ref[...]
    d2col = d2_ref[...]
    bm = d2col.shape[0] // kk
    col = 0
    for bi, r2 in enumerate(r2s):
        h = h0
        base = bi * nl * 4
        for li in range(nl):
            w = wrefs[base + li * 4][...]
            b = wrefs[base + li * 4 + 1][...]
            g = wrefs[base + li * 4 + 2][...]
            be = wrefs[base + li * 4 + 3][...]
            h = jnp.maximum(
                jnp.dot(h, w, preferred_element_type=jnp.float32) + b, 0.0)
            h = g * (h * _INV) + be
        fo = fouts[bi]
        penalty = jnp.where(d2col <= r2, 0.0, -jnp.inf)
        h = h + penalty
        o = jnp.max(h.reshape(bm, kk, fo), axis=1)
        o = jnp.where(jnp.isfinite(o), o, 0.0)
        out_ref[:, col:col + fo] = o
        col += fo


def _sa_conv(h_in, d2k, r_list, conv_params, bm):
    m, kk, f = h_in.shape
    h_flat = h_in.reshape(m * kk, f)
    d2col = d2k.reshape(m * kk, 1)
    nl = len(conv_params[0])
    fouts = tuple(int(layers[-1]["W"].shape[1]) for layers in conv_params)
    r2s = tuple(np.float32(r * r) for r in r_list)
    wargs, wspecs = [], []
    for layers in conv_params:
        for lyr in layers:
            for nm in ("W", "b", "gamma", "beta"):
                a = lyr[nm]
                if a.ndim == 1:
                    a = a.reshape(1, -1)
                wargs.append(a)
                wspecs.append(pl.BlockSpec(a.shape, lambda i: (0, 0)))
    out_f = sum(fouts)
    grid = (m // bm,)
    fn = pl.pallas_call(
        functools.partial(_sa_body, nl, kk, r2s, fouts),
        grid=grid,
        in_specs=[
            pl.BlockSpec((bm * kk, f), lambda i: (i, 0)),
            pl.BlockSpec((bm * kk, 1), lambda i: (i, 0)),
        ] + wspecs,
        out_specs=pl.BlockSpec((bm, out_f), lambda i: (i, 0)),
        out_shape=jax.ShapeDtypeStruct((m, out_f), jnp.float32),
    )
    return fn(h_flat, d2col, *wargs)


def _sa_module(x, pos, ratio, r_list, conv_params, bm, max_nbrs=128):
    n = pos.shape[0]
    m = int(round(ratio * n))
    idx = _fps_idx(pos, m)
    y_pos = pos[idx]
    nbr = _ballq_sc(y_pos, pos, max_nbrs)
    x_j = x[nbr]
    rel = pos[nbr] - y_pos[:, None, :]
    d2k = jnp.sum(rel ** 2, axis=-1)
    h_in = jnp.concatenate([x_j, rel], axis=-1)
    return _sa_conv(h_in, d2k, r_list, conv_params, bm), y_pos


def _mlp_body(nl, with_head, *refs):
    h_ref = refs[0]
    wrefs = refs[1:-1]
    out_ref = refs[-1]
    h = h_ref[...]
    for li in range(nl):
        w = wrefs[li * 4][...]
        b = wrefs[li * 4 + 1][...]
        g = wrefs[li * 4 + 2][...]
        be = wrefs[li * 4 + 3][...]
        h = jnp.maximum(
            jnp.dot(h, w, preferred_element_type=jnp.float32) + b, 0.0)
        h = g * (h * _INV) + be
    if with_head:
        base = nl * 4
        w1, b1 = wrefs[base][...], wrefs[base + 1][...]
        w2, b2 = wrefs[base + 2][...], wrefs[base + 3][...]
        w3, b3 = wrefs[base + 4][...], wrefs[base + 5][...]
        h = jnp.maximum(jnp.dot(h, w1, preferred_element_type=jnp.float32) + b1, 0.0)
        h = jnp.maximum(jnp.dot(h, w2, preferred_element_type=jnp.float32) + b2, 0.0)
        h = jnp.dot(h, w3, preferred_element_type=jnp.float32) + b3
        mx = jnp.max(h, axis=-1, keepdims=True)
        sh = h - jax.lax.stop_gradient(mx)
        h = sh - jnp.log(jnp.sum(jnp.exp(sh), axis=-1, keepdims=True))
    out_ref[...] = h


def _mlp_rows(h, layers, br, head=None):
    rows, f = h.shape
    nl = len(layers)
    wargs, wspecs = [], []
    for lyr in layers:
        for nm in ("W", "b", "gamma", "beta"):
            a = lyr[nm]
            if a.ndim == 1:
                a = a.reshape(1, -1)
            wargs.append(a)
            wspecs.append(pl.BlockSpec(a.shape, lambda i: (0, 0)))
    if head is not None:
        for nm in ("W1", "b1", "W2", "b2", "W3", "b3"):
            a = head[nm]
            if a.ndim == 1:
                a = a.reshape(1, -1)
            wargs.append(a)
            wspecs.append(pl.BlockSpec(a.shape, lambda i: (0, 0)))
        out_f = head["W3"].shape[1]
    else:
        out_f = layers[-1]["W"].shape[1]
    fn = pl.pallas_call(
        functools.partial(_mlp_body, nl, head is not None),
        grid=(rows // br,),
        in_specs=[pl.BlockSpec((br, f), lambda i: (i, 0))] + wspecs,
        out_specs=pl.BlockSpec((br, out_f), lambda i: (i, 0)),
        out_shape=jax.ShapeDtypeStruct((rows, out_f), jnp.float32),
    )
    return fn(h, *wargs)


def _knn_interpolate(xf, posc, pos_skip, k=3):
    d2_sg = jax.lax.stop_gradient(
        jnp.sum((pos_skip[:, None, :] - posc[None, :, :]) ** 2, axis=-1))
    _, idx = jax.lax.approx_max_k(-d2_sg, k, recall_target=1.0)
    diff = pos_skip[:, None, :] - posc[idx]
    d2 = jnp.sum(diff ** 2, axis=-1)
    w = 1.0 / jnp.maximum(d2, 1e-16)
    w = w / jnp.sum(w, axis=1, keepdims=True)
    return jnp.sum(xf[idx] * w[:, :, None], axis=1)


def kernel(x, pos, batch, params):
    x1, pos1 = _sa_module(x, pos, 0.25, [0.05, 0.4], params["sa1"], bm=32)
    x2, pos2 = _sa_module(x1, pos1, 0.25, [0.2, 0.8], params["sa2"], bm=32)
    x3, pos3 = _sa_module(x2, pos2, 0.25, [0.4, 1.6], params["sa3"], bm=32)

    xi3 = _knn_interpolate(x3, pos3, pos2)
    f3 = _mlp_rows(jnp.concatenate([xi3, x2], axis=1), params["fp3"], br=256)
    xi2 = _knn_interpolate(f3, pos2, pos1)
    f2 = _mlp_rows(jnp.concatenate([xi2, x1], axis=1), params["fp2"], br=512)
    xi1 = _knn_interpolate(f2, pos1, pos)
    f1 = _mlp_rows(jnp.concatenate([xi1, x], axis=1), params["fp1"], br=1024)

    return _mlp_rows(f1, [], br=1024, head=params["cls"])


# speedup vs baseline: 1.5311x; 1.0001x over previous
import functools

import jax
import jax.numpy as jnp
import numpy as np
from jax import lax
from jax.experimental import pallas as pl
from jax.experimental.pallas import tpu as pltpu
from jax.experimental.pallas import tpu_sc as plsc

_EPS_BN = 1e-5
_INV = np.float32(1.0) / np.sqrt(np.float32(1.0 + _EPS_BN))

_L = 16
_NB = 272
_ST = 273
_HW = 4416


def _lane_gather(vec, idx):
    return lax.gather(
        vec, idx[:, None],
        dimension_numbers=lax.GatherDimensionNumbers(
            offset_dims=(), collapsed_slice_dims=(0,), start_index_map=(0,)),
        slice_sizes=(1,),
        mode=lax.GatherScatterMode.PROMISE_IN_BOUNDS)


def _ballq_tec(n, k, qpt, *refs):
    (px_h, py_h, pz_h, yx_h, yy_h, yz_h, out_h,
     px_v, py_v, pz_v, yx_v, yy_v, yz_v, bits_v, cb_v, ci_v,
     hist_v, row_v) = refs
    nvec = n // _L
    wid = lax.axis_index("s") * 2 + lax.axis_index("c")

    pltpu.sync_copy(px_h, px_v)
    pltpu.sync_copy(py_h, py_v)
    pltpu.sync_copy(pz_h, pz_v)
    pltpu.sync_copy(yx_h, yx_v)
    pltpu.sync_copy(yy_h, yy_v)
    pltpu.sync_copy(yz_h, yz_v)

    lane = lax.iota(jnp.int32, _L)
    ones = jnp.full((_L,), 1, jnp.int32)
    _U = 4
    zeros = jnp.zeros((_L,), jnp.int32)

    def clear_hist(j, c):
        for u in range(_U):
            hist_v[pl.ds((j * _U + u) * _L, _L)] = zeros
        return c

    last = jnp.full((_L,), _L - 1, jnp.int32)

    def scan_hist(k_rem, nbins):
        def sj(j, st):
            found, bsel, below, run = st
            acc = jnp.zeros((_L,), jnp.int32)
            for l in range(_L):
                acc = acc + hist_v[pl.ds(l * _ST + j * _L, _L)]
            cum = plsc.cumsum(acc) + run
            run2 = _lane_gather(cum, last)
            hit = cum > k_rem
            nhit = plsc.all_reduce_population_count(hit)
            ffs = jnp.minimum(plsc.all_reduce_ffs(hit), _L - 1)
            excl = cum - acc
            below_here = _lane_gather(excl, ffs)
            bin_here = j * _L + ffs
            take = (found == 0) & (nhit > 0)
            bsel = jnp.where(take, bin_here, bsel)
            below = jnp.where(take, below_here, below)
            found = jnp.where(nhit > 0, 1, found)
            return (found, bsel, below, run2)
        z = jnp.zeros((_L,), jnp.int32)
        _, bsel, below, _ = lax.fori_loop(0, nbins // _L, sj, (z, z, z, z))
        return bsel, below

    def hist_pass_compact(shift, mask, pshift, prefix, cnt, cnt_s):
        lax.fori_loop(0, _HW // (_U * _L), clear_hist, 0)
        def pi(i, c):
            b = cb_v[pl.ds(i * _L, _L)]
            valid = (i * _L + lane) < cnt
            binv = (b >> shift) & mask
            m = valid & ((b >> pshift) == prefix)
            plsc.addupdate_scatter(hist_v, [lane * _ST + binv], ones, mask=m)
            return c
        lax.fori_loop(0, (cnt_s + _L - 1) // _L, pi, 0)

    def per_query(lq, carry):
        q = wid * qpt + lq
        qbase = (q // _L) * _L
        qoff = jnp.full((_L,), q - qbase, jnp.int32)
        yx = _lane_gather(yx_v[pl.ds(qbase, _L)], qoff)
        yy = _lane_gather(yy_v[pl.ds(qbase, _L)], qoff)
        yz = _lane_gather(yz_v[pl.ds(qbase, _L)], qoff)

        lax.fori_loop(0, _HW // (_U * _L), clear_hist, 0)
        def p1(i, c):
            for u in range(_U):
                sl = pl.ds((i * _U + u) * _L, _L)
                dx = px_v[sl] - yx
                dy = py_v[sl] - yy
                dz = pz_v[sl] - yz
                d2 = dx * dx + dy * dy + dz * dz
                b = lax.bitcast_convert_type(d2, jnp.int32)
                bits_v[sl] = b
                plsc.addupdate_scatter(hist_v, [lane * _ST + (b >> 22)], ones)
            return c
        lax.fori_loop(0, nvec // _U, p1, 0)

        k0 = jnp.full((_L,), k - 1, jnp.int32)
        b1, below1 = scan_hist(k0, _NB)
        k1 = k0 - below1

        def p2(i, st):
            bs = [bits_v[pl.ds((i * _U + u) * _L, _L)] for u in range(_U)]
            rel = (bs[0] >> 22) <= b1
            for u in range(1, _U):
                rel = rel | ((bs[u] >> 22) <= b1)
            sel = jnp.max(rel.astype(jnp.int32))

            def emit(st):
                a_base, c_base = st
                for u in range(_U):
                    b = bs[u]
                    binv = b >> 22
                    lt1 = binv < b1
                    e1 = binv == b1
                    pos_a = a_base + plsc.cumsum(lt1.astype(jnp.int32)) - 1
                    pos_c = c_base + plsc.cumsum(e1.astype(jnp.int32)) - 1
                    idx_v = (i * _U + u) * _L + lane
                    plsc.store_scatter(
                        row_v, [jnp.minimum(pos_a, k - 1)], idx_v, mask=lt1)
                    plsc.store_scatter(ci_v, [pos_c], idx_v, mask=e1)
                    plsc.store_scatter(cb_v, [pos_c], b, mask=e1)
                    a_base = a_base + plsc.all_reduce_population_count(lt1)
                    c_base = c_base + plsc.all_reduce_population_count(e1)
                return (a_base, c_base)

            return lax.cond(sel > 0, emit, lambda s: s, st)
        zv = jnp.zeros((_L,), jnp.int32)
        _, cnt = lax.fori_loop(0, nvec // _U, p2, (zv, zv))
        cnt_s = jnp.max(cnt)

        hist_pass_compact(14, 0xFF, 22, b1, cnt, cnt_s)
        b2, below2 = scan_hist(k1, 256)
        k2 = k1 - below2
        pre2 = (b1 << 8) | b2

        hist_pass_compact(6, 0xFF, 14, pre2, cnt, cnt_s)
        b3, below3 = scan_hist(k2, 256)
        k3 = k2 - below3
        pre3 = (pre2 << 8) | b3

        hist_pass_compact(0, 0x3F, 6, pre3, cnt, cnt_s)
        b4, below4 = scan_hist(k3, 64)

        t = (pre3 << 6) | b4
        count_lt = below1 + below2 + below3 + below4

        def fp(i, st):
            lt_base, eq_base = st
            b = cb_v[pl.ds(i * _L, _L)]
            valid = (i * _L + lane) < cnt
            lt = valid & (b < t)
            eq = valid & (b == t)
            pos_lt = lt_base + plsc.cumsum(lt.astype(jnp.int32)) - 1
            pos_eq = eq_base + plsc.cumsum(eq.astype(jnp.int32)) - 1
            idx_v = ci_v[pl.ds(i * _L, _L)]
            plsc.store_scatter(
                row_v, [jnp.minimum(pos_lt, k - 1)], idx_v, mask=lt)
            eqm = eq & (pos_eq < k)
            plsc.store_scatter(
                row_v, [jnp.minimum(pos_eq, k - 1)], idx_v, mask=eqm)
            return (lt_base + plsc.all_reduce_population_count(lt),
                    eq_base + plsc.all_reduce_population_count(eq))
        lax.fori_loop(0, (cnt_s + _L - 1) // _L, fp, (below1, count_lt))

        pltpu.sync_copy(row_v, out_h.at[q])
        return carry

    lax.fori_loop(0, qpt, per_query, 0)


def _ballq_sc(y_pos, pos, k):
    m = y_pos.shape[0]
    n = pos.shape[0]
    qpt = m // 32
    kout = max(k, _L)
    mesh = plsc.VectorSubcoreMesh(core_axis_name="c", subcore_axis_name="s")
    fn = functools.partial(
        pl.kernel,
        mesh=mesh,
        compiler_params=pltpu.CompilerParams(needs_layout_passes=False),
        out_type=jax.ShapeDtypeStruct((m, kout), jnp.int32),
        scratch_types=[
            pltpu.VMEM((n,), jnp.float32),
            pltpu.VMEM((n,), jnp.float32),
            pltpu.VMEM((n,), jnp.float32),
            pltpu.VMEM((m,), jnp.float32),
            pltpu.VMEM((m,), jnp.float32),
            pltpu.VMEM((m,), jnp.float32),
            pltpu.VMEM((n,), jnp.int32),
            pltpu.VMEM((n,), jnp.int32),
            pltpu.VMEM((n,), jnp.int32),
            pltpu.VMEM((_HW,), jnp.int32),
            pltpu.VMEM((kout,), jnp.int32),
        ],
    )(functools.partial(_ballq_tec, n, k, qpt))
    out = fn(pos[:, 0], pos[:, 1], pos[:, 2],
             y_pos[:, 0], y_pos[:, 1], y_pos[:, 2])
    return out[:, :k] if kout != k else out


def _fps_body(m, px_ref, py_ref, pz_ref, out_ref):
    px = px_ref[...]
    py = py_ref[...]
    pz = pz_ref[...]
    r = px.shape[0]
    row = jax.lax.broadcasted_iota(jnp.int32, (r, 128), 0)
    colv = jax.lax.broadcasted_iota(jnp.int32, (r, 128), 1)
    flat = row * 128 + colv
    out_ref[...] = jnp.zeros(out_ref.shape, jnp.int32)
    dists0 = jnp.full((r, 128), jnp.inf, jnp.float32)

    def body(i, carry):
        dists, last = carry
        sel = flat == last
        lx = jnp.sum(jnp.where(sel, px, 0.0))
        ly = jnp.sum(jnp.where(sel, py, 0.0))
        lz = jnp.sum(jnp.where(sel, pz, 0.0))
        dxx = px - lx
        dyy = py - ly
        dzz = pz - lz
        d = dxx * dxx + dyy * dyy + dzz * dzz
        dists = jnp.minimum(dists, d)
        mx = jnp.max(dists)
        idx = jnp.min(jnp.where(dists == mx, flat, jnp.int32(2 ** 30)))
        out_ref[pl.ds(i, 1), :] = jnp.reshape(idx, (1, 1))
        return (dists, idx)

    jax.lax.fori_loop(1, m, body, (dists0, jnp.int32(0)))


def _fps_idx(pos, num_samples):
    n = pos.shape[0]
    r = n // 128
    px = pos[:, 0].reshape(r, 128)
    py = pos[:, 1].reshape(r, 128)
    pz = pos[:, 2].reshape(r, 128)
    out = pl.pallas_call(
        functools.partial(_fps_body, num_samples),
        out_shape=jax.ShapeDtypeStruct((num_samples, 1), jnp.int32),
    )(px, py, pz)
    return out.reshape(num_samples)


def _sa_body(nl, kk, r2s, fouts, *refs):
    h_ref, d2_ref = refs[0], refs[1]
    wrefs = refs[2:-1]
    out_ref = refs[-1]
    h0 = h_ref[...]
    d2col = d2_ref[...]
    bm = d2col.shape[0] // kk
    col = 0
    for bi, r2 in enumerate(r2s):
        h = h0
        base = bi * nl * 4
        for li in range(nl):
            w = wrefs[base + li * 4][...]
            b = wrefs[base + li * 4 + 1][...]
            g = wrefs[base + li * 4 + 2][...]
            be = wrefs[base + li * 4 + 3][...]
            h = jnp.maximum(
                jnp.dot(h, w, preferred_element_type=jnp.float32) + b, 0.0)
            h = g * (h * _INV) + be
        fo = fouts[bi]
        penalty = jnp.where(d2col <= r2, 0.0, -jnp.inf)
        h = h + penalty
        o = jnp.max(h.reshape(bm, kk, fo), axis=1)
        o = jnp.where(jnp.isfinite(o), o, 0.0)
        out_ref[:, col:col + fo] = o
        col += fo


def _sa_conv(h_in, d2k, r_list, conv_params, bm):
    m, kk, f = h_in.shape
    h_flat = h_in.reshape(m * kk, f)
    d2col = d2k.reshape(m * kk, 1)
    nl = len(conv_params[0])
    fouts = tuple(int(layers[-1]["W"].shape[1]) for layers in conv_params)
    r2s = tuple(np.float32(r * r) for r in r_list)
    wargs, wspecs = [], []
    for layers in conv_params:
        for lyr in layers:
            for nm in ("W", "b", "gamma", "beta"):
                a = lyr[nm]
                if a.ndim == 1:
                    a = a.reshape(1, -1)
                wargs.append(a)
                wspecs.append(pl.BlockSpec(a.shape, lambda i: (0, 0)))
    out_f = sum(fouts)
    grid = (m // bm,)
    fn = pl.pallas_call(
        functools.partial(_sa_body, nl, kk, r2s, fouts),
        grid=grid,
        in_specs=[
            pl.BlockSpec((bm * kk, f), lambda i: (i, 0)),
            pl.BlockSpec((bm * kk, 1), lambda i: (i, 0)),
        ] + wspecs,
        out_specs=pl.BlockSpec((bm, out_f), lambda i: (i, 0)),
        out_shape=jax.ShapeDtypeStruct((m, out_f), jnp.float32),
    )
    return fn(h_flat, d2col, *wargs)


def _sa_module(x, pos, ratio, r_list, conv_params, bm, max_nbrs=128):
    n = pos.shape[0]
    m = int(round(ratio * n))
    idx = _fps_idx(pos, m)
    y_pos = pos[idx]
    nbr = _ballq_sc(y_pos, pos, max_nbrs)
    x_j = x[nbr]
    rel = pos[nbr] - y_pos[:, None, :]
    d2k = jnp.sum(rel ** 2, axis=-1)
    h_in = jnp.concatenate([x_j, rel], axis=-1)
    return _sa_conv(h_in, d2k, r_list, conv_params, bm), y_pos


def _mlp_body(nl, with_head, *refs):
    h_ref = refs[0]
    wrefs = refs[1:-1]
    out_ref = refs[-1]
    h = h_ref[...]
    for li in range(nl):
        w = wrefs[li * 4][...]
        b = wrefs[li * 4 + 1][...]
        g = wrefs[li * 4 + 2][...]
        be = wrefs[li * 4 + 3][...]
        h = jnp.maximum(
            jnp.dot(h, w, preferred_element_type=jnp.float32) + b, 0.0)
        h = g * (h * _INV) + be
    if with_head:
        base = nl * 4
        w1, b1 = wrefs[base][...], wrefs[base + 1][...]
        w2, b2 = wrefs[base + 2][...], wrefs[base + 3][...]
        w3, b3 = wrefs[base + 4][...], wrefs[base + 5][...]
        h = jnp.maximum(jnp.dot(h, w1, preferred_element_type=jnp.float32) + b1, 0.0)
        h = jnp.maximum(jnp.dot(h, w2, preferred_element_type=jnp.float32) + b2, 0.0)
        h = jnp.dot(h, w3, preferred_element_type=jnp.float32) + b3
        mx = jnp.max(h, axis=-1, keepdims=True)
        sh = h - jax.lax.stop_gradient(mx)
        h = sh - jnp.log(jnp.sum(jnp.exp(sh), axis=-1, keepdims=True))
    out_ref[...] = h


def _mlp_rows(h, layers, br, head=None):
    rows, f = h.shape
    nl = len(layers)
    wargs, wspecs = [], []
    for lyr in layers:
        for nm in ("W", "b", "gamma", "beta"):
            a = lyr[nm]
            if a.ndim == 1:
                a = a.reshape(1, -1)
            wargs.append(a)
            wspecs.append(pl.BlockSpec(a.shape, lambda i: (0, 0)))
    if head is not None:
        for nm in ("W1", "b1", "W2", "b2", "W3", "b3"):
            a = head[nm]
            if a.ndim == 1:
                a = a.reshape(1, -1)
            wargs.append(a)
            wspecs.append(pl.BlockSpec(a.shape, lambda i: (0, 0)))
        out_f = head["W3"].shape[1]
    else:
        out_f = layers[-1]["W"].shape[1]
    fn = pl.pallas_call(
        functools.partial(_mlp_body, nl, head is not None),
        grid=(rows // br,),
        in_specs=[pl.BlockSpec((br, f), lambda i: (i, 0))] + wspecs,
        out_specs=pl.BlockSpec((br, out_f), lambda i: (i, 0)),
        out_shape=jax.ShapeDtypeStruct((rows, out_f), jnp.float32),
    )
    return fn(h, *wargs)


def _knn_interpolate(xf, posc, pos_skip, k=3):
    d2_sg = jax.lax.stop_gradient(
        jnp.sum((pos_skip[:, None, :] - posc[None, :, :]) ** 2, axis=-1))
    _, idx = jax.lax.approx_max_k(-d2_sg, k, recall_target=1.0)
    diff = pos_skip[:, None, :] - posc[idx]
    d2 = jnp.sum(diff ** 2, axis=-1)
    w = 1.0 / jnp.maximum(d2, 1e-16)
    w = w / jnp.sum(w, axis=1, keepdims=True)
    return jnp.sum(xf[idx] * w[:, :, None], axis=1)


def kernel(x, pos, batch, params):
    x1, pos1 = _sa_module(x, pos, 0.25, [0.05, 0.4], params["sa1"], bm=32)
    x2, pos2 = _sa_module(x1, pos1, 0.25, [0.2, 0.8], params["sa2"], bm=32)
    x3, pos3 = _sa_module(x2, pos2, 0.25, [0.4, 1.6], params["sa3"], bm=32)

    xi3 = _knn_interpolate(x3, pos3, pos2)
    f3 = _mlp_rows(jnp.concatenate([xi3, x2], axis=1), params["fp3"], br=256)
    xi2 = _knn_interpolate(f3, pos2, pos1)
    f2 = _mlp_rows(jnp.concatenate([xi2, x1], axis=1), params["fp2"], br=512)
    xi1 = _knn_interpolate(f2, pos1, pos)
    f1 = _mlp_rows(jnp.concatenate([xi1, x], axis=1), params["fp1"], br=1024)

    return _mlp_rows(f1, [], br=1024, head=params["cls"])


# speedup vs baseline: 1.5311x; 1.0000x over previous
import functools

import jax
import jax.numpy as jnp
import numpy as np
from jax import lax
from jax.experimental import pallas as pl
from jax.experimental.pallas import tpu as pltpu
from jax.experimental.pallas import tpu_sc as plsc

_EPS_BN = 1e-5
_INV = np.float32(1.0) / np.sqrt(np.float32(1.0 + _EPS_BN))

_L = 16
_NB = 272
_ST = 273
_HW = 4416


def _lane_gather(vec, idx):
    return lax.gather(
        vec, idx[:, None],
        dimension_numbers=lax.GatherDimensionNumbers(
            offset_dims=(), collapsed_slice_dims=(0,), start_index_map=(0,)),
        slice_sizes=(1,),
        mode=lax.GatherScatterMode.PROMISE_IN_BOUNDS)


def _ballq_tec(n, k, qpt, *refs):
    (px_h, py_h, pz_h, yx_h, yy_h, yz_h, out_h,
     px_v, py_v, pz_v, yx_v, yy_v, yz_v, bits_v, cb_v, ci_v,
     hist_v, row_v) = refs
    nvec = n // _L
    wid = lax.axis_index("s") * 2 + lax.axis_index("c")

    pltpu.sync_copy(px_h, px_v)
    pltpu.sync_copy(py_h, py_v)
    pltpu.sync_copy(pz_h, pz_v)
    pltpu.sync_copy(yx_h, yx_v)
    pltpu.sync_copy(yy_h, yy_v)
    pltpu.sync_copy(yz_h, yz_v)

    lane = lax.iota(jnp.int32, _L)
    ones = jnp.full((_L,), 1, jnp.int32)
    _U = 4
    zeros = jnp.zeros((_L,), jnp.int32)

    def clear_hist():
        @functools.partial(plsc.parallel_loop, 0, _HW // _L, unroll=_U)
        def _(j):
            hist_v[pl.ds(j * _L, _L)] = zeros

    last = jnp.full((_L,), _L - 1, jnp.int32)

    def scan_hist(k_rem, nbins):
        def sj(j, st):
            found, bsel, below, run = st
            acc = jnp.zeros((_L,), jnp.int32)
            for l in range(_L):
                acc = acc + hist_v[pl.ds(l * _ST + j * _L, _L)]
            cum = plsc.cumsum(acc) + run
            run2 = _lane_gather(cum, last)
            hit = cum > k_rem
            nhit = plsc.all_reduce_population_count(hit)
            ffs = jnp.minimum(plsc.all_reduce_ffs(hit), _L - 1)
            excl = cum - acc
            below_here = _lane_gather(excl, ffs)
            bin_here = j * _L + ffs
            take = (found == 0) & (nhit > 0)
            bsel = jnp.where(take, bin_here, bsel)
            below = jnp.where(take, below_here, below)
            found = jnp.where(nhit > 0, 1, found)
            return (found, bsel, below, run2)
        z = jnp.zeros((_L,), jnp.int32)
        _, bsel, below, _ = lax.fori_loop(0, nbins // _L, sj, (z, z, z, z))
        return bsel, below

    def hist_pass_compact(shift, mask, pshift, prefix, cnt, cnt_s):
        clear_hist()

        @functools.partial(
            plsc.parallel_loop, 0, (cnt_s + _L - 1) // _L, unroll=2)
        def _(i):
            b = cb_v[pl.ds(i * _L, _L)]
            valid = (i * _L + lane) < cnt
            binv = (b >> shift) & mask
            m = valid & ((b >> pshift) == prefix)
            plsc.addupdate_scatter(hist_v, [lane * _ST + binv], ones, mask=m)

    def per_query(lq, carry):
        q = wid * qpt + lq
        qbase = (q // _L) * _L
        qoff = jnp.full((_L,), q - qbase, jnp.int32)
        yx = _lane_gather(yx_v[pl.ds(qbase, _L)], qoff)
        yy = _lane_gather(yy_v[pl.ds(qbase, _L)], qoff)
        yz = _lane_gather(yz_v[pl.ds(qbase, _L)], qoff)

        clear_hist()

        @functools.partial(plsc.parallel_loop, 0, nvec, unroll=_U)
        def _(i):
            sl = pl.ds(i * _L, _L)
            dx = px_v[sl] - yx
            dy = py_v[sl] - yy
            dz = pz_v[sl] - yz
            d2 = dx * dx + dy * dy + dz * dz
            b = lax.bitcast_convert_type(d2, jnp.int32)
            bits_v[sl] = b
            plsc.addupdate_scatter(hist_v, [lane * _ST + (b >> 22)], ones)

        k0 = jnp.full((_L,), k - 1, jnp.int32)
        b1, below1 = scan_hist(k0, _NB)
        k1 = k0 - below1

        def p2(i, st):
            bs = [bits_v[pl.ds((i * _U + u) * _L, _L)] for u in range(_U)]
            rel = (bs[0] >> 22) <= b1
            for u in range(1, _U):
                rel = rel | ((bs[u] >> 22) <= b1)
            sel = jnp.max(rel.astype(jnp.int32))

            def emit(st):
                a_base, c_base = st
                for u in range(_U):
                    b = bs[u]
                    binv = b >> 22
                    lt1 = binv < b1
                    e1 = binv == b1
                    pos_a = a_base + plsc.cumsum(lt1.astype(jnp.int32)) - 1
                    pos_c = c_base + plsc.cumsum(e1.astype(jnp.int32)) - 1
                    idx_v = (i * _U + u) * _L + lane
                    plsc.store_scatter(
                        row_v, [jnp.minimum(pos_a, k - 1)], idx_v, mask=lt1)
                    plsc.store_scatter(ci_v, [pos_c], idx_v, mask=e1)
                    plsc.store_scatter(cb_v, [pos_c], b, mask=e1)
                    a_base = a_base + plsc.all_reduce_population_count(lt1)
                    c_base = c_base + plsc.all_reduce_population_count(e1)
                return (a_base, c_base)

            return lax.cond(sel > 0, emit, lambda s: s, st)
        zv = jnp.zeros((_L,), jnp.int32)
        _, cnt = lax.fori_loop(0, nvec // _U, p2, (zv, zv))
        cnt_s = jnp.max(cnt)

        hist_pass_compact(14, 0xFF, 22, b1, cnt, cnt_s)
        b2, below2 = scan_hist(k1, 256)
        k2 = k1 - below2
        pre2 = (b1 << 8) | b2

        hist_pass_compact(6, 0xFF, 14, pre2, cnt, cnt_s)
        b3, below3 = scan_hist(k2, 256)
        k3 = k2 - below3
        pre3 = (pre2 << 8) | b3

        hist_pass_compact(0, 0x3F, 6, pre3, cnt, cnt_s)
        b4, below4 = scan_hist(k3, 64)

        t = (pre3 << 6) | b4
        count_lt = below1 + below2 + below3 + below4

        def fp(i, st):
            lt_base, eq_base = st
            b = cb_v[pl.ds(i * _L, _L)]
            valid = (i * _L + lane) < cnt
            lt = valid & (b < t)
            eq = valid & (b == t)
            pos_lt = lt_base + plsc.cumsum(lt.astype(jnp.int32)) - 1
            pos_eq = eq_base + plsc.cumsum(eq.astype(jnp.int32)) - 1
            idx_v = ci_v[pl.ds(i * _L, _L)]
            plsc.store_scatter(
                row_v, [jnp.minimum(pos_lt, k - 1)], idx_v, mask=lt)
            eqm = eq & (pos_eq < k)
            plsc.store_scatter(
                row_v, [jnp.minimum(pos_eq, k - 1)], idx_v, mask=eqm)
            return (lt_base + plsc.all_reduce_population_count(lt),
                    eq_base + plsc.all_reduce_population_count(eq))
        lax.fori_loop(0, (cnt_s + _L - 1) // _L, fp, (below1, count_lt))

        pltpu.sync_copy(row_v, out_h.at[q])
        return carry

    lax.fori_loop(0, qpt, per_query, 0)


def _ballq_sc(y_pos, pos, k):
    m = y_pos.shape[0]
    n = pos.shape[0]
    qpt = m // 32
    kout = max(k, _L)
    mesh = plsc.VectorSubcoreMesh(core_axis_name="c", subcore_axis_name="s")
    fn = functools.partial(
        pl.kernel,
        mesh=mesh,
        compiler_params=pltpu.CompilerParams(needs_layout_passes=False),
        out_type=jax.ShapeDtypeStruct((m, kout), jnp.int32),
        scratch_types=[
            pltpu.VMEM((n,), jnp.float32),
            pltpu.VMEM((n,), jnp.float32),
            pltpu.VMEM((n,), jnp.float32),
            pltpu.VMEM((m,), jnp.float32),
            pltpu.VMEM((m,), jnp.float32),
            pltpu.VMEM((m,), jnp.float32),
            pltpu.VMEM((n,), jnp.int32),
            pltpu.VMEM((n,), jnp.int32),
            pltpu.VMEM((n,), jnp.int32),
            pltpu.VMEM((_HW,), jnp.int32),
            pltpu.VMEM((kout,), jnp.int32),
        ],
    )(functools.partial(_ballq_tec, n, k, qpt))
    out = fn(pos[:, 0], pos[:, 1], pos[:, 2],
             y_pos[:, 0], y_pos[:, 1], y_pos[:, 2])
    return out[:, :k] if kout != k else out


def _fps_body(m, px_ref, py_ref, pz_ref, out_ref):
    px = px_ref[...]
    py = py_ref[...]
    pz = pz_ref[...]
    r = px.shape[0]
    row = jax.lax.broadcasted_iota(jnp.int32, (r, 128), 0)
    colv = jax.lax.broadcasted_iota(jnp.int32, (r, 128), 1)
    flat = row * 128 + colv
    out_ref[...] = jnp.zeros(out_ref.shape, jnp.int32)
    dists0 = jnp.full((r, 128), jnp.inf, jnp.float32)

    def body(i, carry):
        dists, last = carry
        sel = flat == last
        lx = jnp.sum(jnp.where(sel, px, 0.0))
        ly = jnp.sum(jnp.where(sel, py, 0.0))
        lz = jnp.sum(jnp.where(sel, pz, 0.0))
        dxx = px - lx
        dyy = py - ly
        dzz = pz - lz
        d = dxx * dxx + dyy * dyy + dzz * dzz
        dists = jnp.minimum(dists, d)
        mx = jnp.max(dists)
        idx = jnp.min(jnp.where(dists == mx, flat, jnp.int32(2 ** 30)))
        out_ref[pl.ds(i, 1), :] = jnp.reshape(idx, (1, 1))
        return (dists, idx)

    jax.lax.fori_loop(1, m, body, (dists0, jnp.int32(0)))


def _fps_idx(pos, num_samples):
    n = pos.shape[0]
    r = n // 128
    px = pos[:, 0].reshape(r, 128)
    py = pos[:, 1].reshape(r, 128)
    pz = pos[:, 2].reshape(r, 128)
    out = pl.pallas_call(
        functools.partial(_fps_body, num_samples),
        out_shape=jax.ShapeDtypeStruct((num_samples, 1), jnp.int32),
    )(px, py, pz)
    return out.reshape(num_samples)


def _sa_body(nl, kk, r2s, fouts, *refs):
    h_ref, d2_ref = refs[0], refs[1]
    wrefs = refs[2:-1]
    out_ref = refs[-1]
    h0 = h_ref[...]
    d2col = d2_ref[...]
    bm = d2col.shape[0] // kk
    col = 0
    for bi, r2 in enumerate(r2s):
        h = h0
        base = bi * nl * 4
        for li in range(nl):
            w = wrefs[base + li * 4][...]
            b = wrefs[base + li * 4 + 1][...]
            g = wrefs[base + li * 4 + 2][...]
            be = wrefs[base + li * 4 + 3][...]
            h = jnp.maximum(
                jnp.dot(h, w, preferred_element_type=jnp.float32) + b, 0.0)
            h = g * (h * _INV) + be
        fo = fouts[bi]
        penalty = jnp.where(d2col <= r2, 0.0, -jnp.inf)
        h = h + penalty
        o = jnp.max(h.reshape(bm, kk, fo), axis=1)
        o = jnp.where(jnp.isfinite(o), o, 0.0)
        out_ref[:, col:col + fo] = o
        col += fo


def _sa_conv(h_in, d2k, r_list, conv_params, bm):
    m, kk, f = h_in.shape
    h_flat = h_in.reshape(m * kk, f)
    d2col = d2k.reshape(m * kk, 1)
    nl = len(conv_params[0])
    fouts = tuple(int(layers[-1]["W"].shape[1]) for layers in conv_params)
    r2s = tuple(np.float32(r * r) for r in r_list)
    wargs, wspecs = [], []
    for layers in conv_params:
        for lyr in layers:
            for nm in ("W", "b", "gamma", "beta"):
                a = lyr[nm]
                if a.ndim == 1:
                    a = a.reshape(1, -1)
                wargs.append(a)
                wspecs.append(pl.BlockSpec(a.shape, lambda i: (0, 0)))
    out_f = sum(fouts)
    grid = (m // bm,)
    fn = pl.pallas_call(
        functools.partial(_sa_body, nl, kk, r2s, fouts),
        grid=grid,
        in_specs=[
            pl.BlockSpec((bm * kk, f), lambda i: (i, 0)),
            pl.BlockSpec((bm * kk, 1), lambda i: (i, 0)),
        ] + wspecs,
        out_specs=pl.BlockSpec((bm, out_f), lambda i: (i, 0)),
        out_shape=jax.ShapeDtypeStruct((m, out_f), jnp.float32),
    )
    return fn(h_flat, d2col, *wargs)


def _sa_module(x, pos, ratio, r_list, conv_params, bm, max_nbrs=128):
    n = pos.shape[0]
    m = int(round(ratio * n))
    idx = _fps_idx(pos, m)
    y_pos = pos[idx]
    nbr = _ballq_sc(y_pos, pos, max_nbrs)
    x_j = x[nbr]
    rel = pos[nbr] - y_pos[:, None, :]
    d2k = jnp.sum(rel ** 2, axis=-1)
    h_in = jnp.concatenate([x_j, rel], axis=-1)
    return _sa_conv(h_in, d2k, r_list, conv_params, bm), y_pos


def _mlp_body(nl, with_head, *refs):
    h_ref = refs[0]
    wrefs = refs[1:-1]
    out_ref = refs[-1]
    h = h_ref[...]
    for li in range(nl):
        w = wrefs[li * 4][...]
        b = wrefs[li * 4 + 1][...]
        g = wrefs[li * 4 + 2][...]
        be = wrefs[li * 4 + 3][...]
        h = jnp.maximum(
            jnp.dot(h, w, preferred_element_type=jnp.float32) + b, 0.0)
        h = g * (h * _INV) + be
    if with_head:
        base = nl * 4
        w1, b1 = wrefs[base][...], wrefs[base + 1][...]
        w2, b2 = wrefs[base + 2][...], wrefs[base + 3][...]
        w3, b3 = wrefs[base + 4][...], wrefs[base + 5][...]
        h = jnp.maximum(jnp.dot(h, w1, preferred_element_type=jnp.float32) + b1, 0.0)
        h = jnp.maximum(jnp.dot(h, w2, preferred_element_type=jnp.float32) + b2, 0.0)
        h = jnp.dot(h, w3, preferred_element_type=jnp.float32) + b3
        mx = jnp.max(h, axis=-1, keepdims=True)
        sh = h - jax.lax.stop_gradient(mx)
        h = sh - jnp.log(jnp.sum(jnp.exp(sh), axis=-1, keepdims=True))
    out_ref[...] = h


def _mlp_rows(h, layers, br, head=None):
    rows, f = h.shape
    nl = len(layers)
    wargs, wspecs = [], []
    for lyr in layers:
        for nm in ("W", "b", "gamma", "beta"):
            a = lyr[nm]
            if a.ndim == 1:
                a = a.reshape(1, -1)
            wargs.append(a)
            wspecs.append(pl.BlockSpec(a.shape, lambda i: (0, 0)))
    if head is not None:
        for nm in ("W1", "b1", "W2", "b2", "W3", "b3"):
            a = head[nm]
            if a.ndim == 1:
                a = a.reshape(1, -1)
            wargs.append(a)
            wspecs.append(pl.BlockSpec(a.shape, lambda i: (0, 0)))
        out_f = head["W3"].shape[1]
    else:
        out_f = layers[-1]["W"].shape[1]
    fn = pl.pallas_call(
        functools.partial(_mlp_body, nl, head is not None),
        grid=(rows // br,),
        in_specs=[pl.BlockSpec((br, f), lambda i: (i, 0))] + wspecs,
        out_specs=pl.BlockSpec((br, out_f), lambda i: (i, 0)),
        out_shape=jax.ShapeDtypeStruct((rows, out_f), jnp.float32),
    )
    return fn(h, *wargs)


def _knn_interpolate(xf, posc, pos_skip, k=3):
    d2_sg = jax.lax.stop_gradient(
        jnp.sum((pos_skip[:, None, :] - posc[None, :, :]) ** 2, axis=-1))
    _, idx = jax.lax.approx_max_k(-d2_sg, k, recall_target=1.0)
    diff = pos_skip[:, None, :] - posc[idx]
    d2 = jnp.sum(diff ** 2, axis=-1)
    w = 1.0 / jnp.maximum(d2, 1e-16)
    w = w / jnp.sum(w, axis=1, keepdims=True)
    return jnp.sum(xf[idx] * w[:, :, None], axis=1)


def kernel(x, pos, batch, params):
    x1, pos1 = _sa_module(x, pos, 0.25, [0.05, 0.4], params["sa1"], bm=32)
    x2, pos2 = _sa_module(x1, pos1, 0.25, [0.2, 0.8], params["sa2"], bm=32)
    x3, pos3 = _sa_module(x2, pos2, 0.25, [0.4, 1.6], params["sa3"], bm=32)

    xi3 = _knn_interpolate(x3, pos3, pos2)
    f3 = _mlp_rows(jnp.concatenate([xi3, x2], axis=1), params["fp3"], br=256)
    xi2 = _knn_interpolate(f3, pos2, pos1)
    f2 = _mlp_rows(jnp.concatenate([xi2, x1], axis=1), params["fp2"], br=512)
    xi1 = _knn_interpolate(f2, pos1, pos)
    f1 = _mlp_rows(jnp.concatenate([xi1, x], axis=1), params["fp1"], br=1024)

    return _mlp_rows(f1, [], br=1024, head=params["cls"])


# speedup vs baseline: 1.9446x; 1.2700x over previous
import functools

import jax
import jax.numpy as jnp
import numpy as np
from jax import lax
from jax.experimental import pallas as pl
from jax.experimental.pallas import tpu as pltpu
from jax.experimental.pallas import tpu_sc as plsc

_EPS_BN = 1e-5
_INV = np.float32(1.0) / np.sqrt(np.float32(1.0 + _EPS_BN))

_L = 16
_NB = 272
_ST = 273
_HW = 4416


def _lane_gather(vec, idx):
    return lax.gather(
        vec, idx[:, None],
        dimension_numbers=lax.GatherDimensionNumbers(
            offset_dims=(), collapsed_slice_dims=(0,), start_index_map=(0,)),
        slice_sizes=(1,),
        mode=lax.GatherScatterMode.PROMISE_IN_BOUNDS)


def _ballq_tec(n, k, qpt, *refs):
    (px_h, py_h, pz_h, yx_h, yy_h, yz_h, out_h,
     px_v, py_v, pz_v, yx_v, yy_v, yz_v, bits_v, cb_v, ci_v,
     hist_v, row_v) = refs
    nvec = n // _L
    wid = lax.axis_index("s") * 2 + lax.axis_index("c")

    pltpu.sync_copy(px_h, px_v)
    pltpu.sync_copy(py_h, py_v)
    pltpu.sync_copy(pz_h, pz_v)
    pltpu.sync_copy(yx_h, yx_v)
    pltpu.sync_copy(yy_h, yy_v)
    pltpu.sync_copy(yz_h, yz_v)

    lane = lax.iota(jnp.int32, _L)
    ones = jnp.full((_L,), 1, jnp.int32)
    _U = 4
    zeros = jnp.zeros((_L,), jnp.int32)

    def clear_hist():
        def cj(j, c):
            for u in range(_U):
                hist_v[pl.ds((j * _U + u) * _L, _L)] = zeros
            return c
        lax.fori_loop(0, _HW // (_U * _L), cj, 0)

    last = jnp.full((_L,), _L - 1, jnp.int32)

    def scan_hist(k_rem, nbins):
        def sj(j, st):
            found, bsel, below, run = st
            acc = jnp.zeros((_L,), jnp.int32)
            for l in range(_L):
                acc = acc + hist_v[pl.ds(l * _ST + j * _L, _L)]
            cum = plsc.cumsum(acc) + run
            run2 = _lane_gather(cum, last)
            hit = cum > k_rem
            nhit = plsc.all_reduce_population_count(hit)
            ffs = jnp.minimum(plsc.all_reduce_ffs(hit), _L - 1)
            excl = cum - acc
            below_here = _lane_gather(excl, ffs)
            bin_here = j * _L + ffs
            take = (found == 0) & (nhit > 0)
            bsel = jnp.where(take, bin_here, bsel)
            below = jnp.where(take, below_here, below)
            found = jnp.where(nhit > 0, 1, found)
            return (found, bsel, below, run2)
        z = jnp.zeros((_L,), jnp.int32)
        _, bsel, below, _ = lax.fori_loop(0, nbins // _L, sj, (z, z, z, z))
        return bsel, below

    def hist_pass_compact(shift, mask, pshift, prefix, cnt, cnt_s):
        clear_hist()

        def pi(i, c):
            b = cb_v[pl.ds(i * _L, _L)]
            valid = (i * _L + lane) < cnt
            binv = (b >> shift) & mask
            m = valid & ((b >> pshift) == prefix)
            plsc.addupdate_scatter(hist_v, [lane * _ST + binv], ones, mask=m)
            return c
        lax.fori_loop(0, (cnt_s + _L - 1) // _L, pi, 0)

    def per_query(lq, carry):
        q = wid * qpt + lq
        qbase = (q // _L) * _L
        qoff = jnp.full((_L,), q - qbase, jnp.int32)
        yx = _lane_gather(yx_v[pl.ds(qbase, _L)], qoff)
        yy = _lane_gather(yy_v[pl.ds(qbase, _L)], qoff)
        yz = _lane_gather(yz_v[pl.ds(qbase, _L)], qoff)

        clear_hist()

        def p1(i, c):
            for u in range(_U):
                sl = pl.ds((i * _U + u) * _L, _L)
                dx = px_v[sl] - yx
                dy = py_v[sl] - yy
                dz = pz_v[sl] - yz
                d2 = dx * dx + dy * dy + dz * dz
                b = lax.bitcast_convert_type(d2, jnp.int32)
                bits_v[sl] = b
                plsc.addupdate_scatter(hist_v, [lane * _ST + (b >> 22)], ones)
            return c
        lax.fori_loop(0, nvec // _U, p1, 0)

        k0 = jnp.full((_L,), k - 1, jnp.int32)
        b1, below1 = scan_hist(k0, _NB)
        k1 = k0 - below1

        def p2(i, st):
            bs = [bits_v[pl.ds((i * _U + u) * _L, _L)] for u in range(_U)]
            rel = (bs[0] >> 22) <= b1
            for u in range(1, _U):
                rel = rel | ((bs[u] >> 22) <= b1)
            sel = jnp.max(rel.astype(jnp.int32))

            def emit(st):
                a_base, c_base = st
                for u in range(_U):
                    b = bs[u]
                    binv = b >> 22
                    lt1 = binv < b1
                    e1 = binv == b1
                    pos_a = a_base + plsc.cumsum(lt1.astype(jnp.int32)) - 1
                    pos_c = c_base + plsc.cumsum(e1.astype(jnp.int32)) - 1
                    idx_v = (i * _U + u) * _L + lane
                    plsc.store_scatter(
                        row_v, [jnp.minimum(pos_a, k - 1)], idx_v, mask=lt1)
                    plsc.store_scatter(ci_v, [pos_c], idx_v, mask=e1)
                    plsc.store_scatter(cb_v, [pos_c], b, mask=e1)
                    a_base = a_base + plsc.all_reduce_population_count(lt1)
                    c_base = c_base + plsc.all_reduce_population_count(e1)
                return (a_base, c_base)

            return lax.cond(sel > 0, emit, lambda s: s, st)
        zv = jnp.zeros((_L,), jnp.int32)
        _, cnt = lax.fori_loop(0, nvec // _U, p2, (zv, zv))
        cnt_s = jnp.max(cnt)

        hist_pass_compact(14, 0xFF, 22, b1, cnt, cnt_s)
        b2, below2 = scan_hist(k1, 256)
        k2 = k1 - below2
        pre2 = (b1 << 8) | b2

        hist_pass_compact(6, 0xFF, 14, pre2, cnt, cnt_s)
        b3, below3 = scan_hist(k2, 256)
        k3 = k2 - below3
        pre3 = (pre2 << 8) | b3

        hist_pass_compact(0, 0x3F, 6, pre3, cnt, cnt_s)
        b4, below4 = scan_hist(k3, 64)

        t = (pre3 << 6) | b4
        count_lt = below1 + below2 + below3 + below4

        def fp(i, st):
            lt_base, eq_base = st
            b = cb_v[pl.ds(i * _L, _L)]
            valid = (i * _L + lane) < cnt
            lt = valid & (b < t)
            eq = valid & (b == t)
            pos_lt = lt_base + plsc.cumsum(lt.astype(jnp.int32)) - 1
            pos_eq = eq_base + plsc.cumsum(eq.astype(jnp.int32)) - 1
            idx_v = ci_v[pl.ds(i * _L, _L)]
            plsc.store_scatter(
                row_v, [jnp.minimum(pos_lt, k - 1)], idx_v, mask=lt)
            eqm = eq & (pos_eq < k)
            plsc.store_scatter(
                row_v, [jnp.minimum(pos_eq, k - 1)], idx_v, mask=eqm)
            return (lt_base + plsc.all_reduce_population_count(lt),
                    eq_base + plsc.all_reduce_population_count(eq))
        lax.fori_loop(0, (cnt_s + _L - 1) // _L, fp, (below1, count_lt))

        pltpu.sync_copy(row_v, out_h.at[q])
        return carry

    lax.fori_loop(0, qpt, per_query, 0)


def _ballq_sc(y_pos, pos, k):
    m = y_pos.shape[0]
    n = pos.shape[0]
    qpt = m // 32
    kout = max(k, _L)
    mesh = plsc.VectorSubcoreMesh(core_axis_name="c", subcore_axis_name="s")
    fn = functools.partial(
        pl.kernel,
        mesh=mesh,
        compiler_params=pltpu.CompilerParams(needs_layout_passes=False),
        out_type=jax.ShapeDtypeStruct((m, kout), jnp.int32),
        scratch_types=[
            pltpu.VMEM((n,), jnp.float32),
            pltpu.VMEM((n,), jnp.float32),
            pltpu.VMEM((n,), jnp.float32),
            pltpu.VMEM((m,), jnp.float32),
            pltpu.VMEM((m,), jnp.float32),
            pltpu.VMEM((m,), jnp.float32),
            pltpu.VMEM((n,), jnp.int32),
            pltpu.VMEM((n,), jnp.int32),
            pltpu.VMEM((n,), jnp.int32),
            pltpu.VMEM((_HW,), jnp.int32),
            pltpu.VMEM((kout,), jnp.int32),
        ],
    )(functools.partial(_ballq_tec, n, k, qpt))
    out = fn(pos[:, 0], pos[:, 1], pos[:, 2],
             y_pos[:, 0], y_pos[:, 1], y_pos[:, 2])
    return out[:, :k] if kout != k else out


def _fps_body(m, px_ref, py_ref, pz_ref, out_ref):
    px = px_ref[...]
    py = py_ref[...]
    pz = pz_ref[...]
    r = px.shape[0]
    row = jax.lax.broadcasted_iota(jnp.int32, (r, 128), 0)
    colv = jax.lax.broadcasted_iota(jnp.int32, (r, 128), 1)
    flat = row * 128 + colv
    out_ref[...] = jnp.zeros(out_ref.shape, jnp.int32)
    dists0 = jnp.full((r, 128), jnp.inf, jnp.float32)

    def body(i, carry):
        dists, last = carry
        sel = flat == last
        lx = jnp.sum(jnp.where(sel, px, 0.0))
        ly = jnp.sum(jnp.where(sel, py, 0.0))
        lz = jnp.sum(jnp.where(sel, pz, 0.0))
        dxx = px - lx
        dyy = py - ly
        dzz = pz - lz
        d = dxx * dxx + dyy * dyy + dzz * dzz
        dists = jnp.minimum(dists, d)
        mx = jnp.max(dists)
        idx = jnp.min(jnp.where(dists == mx, flat, jnp.int32(2 ** 30)))
        out_ref[pl.ds(i, 1), :] = jnp.reshape(idx, (1, 1))
        return (dists, idx)

    jax.lax.fori_loop(1, m, body, (dists0, jnp.int32(0)))


def _fps_idx(pos, num_samples):
    n = pos.shape[0]
    r = n // 128
    px = pos[:, 0].reshape(r, 128)
    py = pos[:, 1].reshape(r, 128)
    pz = pos[:, 2].reshape(r, 128)
    out = pl.pallas_call(
        functools.partial(_fps_body, num_samples),
        out_shape=jax.ShapeDtypeStruct((num_samples, 1), jnp.int32),
    )(px, py, pz)
    return out.reshape(num_samples)


def _sa_body(nl, kk, r2s, fouts, *refs):
    h_ref, d2_ref = refs[0], refs[1]
    wrefs = refs[2:-1]
    out_ref = refs[-1]
    h0 = h_ref[...]
    d2col = d2_ref[...]
    bm = d2col.shape[0] // kk
    col = 0
    for bi, r2 in enumerate(r2s):
        h = h0
        base = bi * nl * 4
        for li in range(nl):
            w = wrefs[base + li * 4][...]
            b = wrefs[base + li * 4 + 1][...]
            g = wrefs[base + li * 4 + 2][...]
            be = wrefs[base + li * 4 + 3][...]
            h = jnp.maximum(
                jnp.dot(h, w, preferred_element_type=jnp.float32) + b, 0.0)
            h = g * (h * _INV) + be
        fo = fouts[bi]
        penalty = jnp.where(d2col <= r2, 0.0, -jnp.inf)
        h = h + penalty
        o = jnp.max(h.reshape(bm, kk, fo), axis=1)
        o = jnp.where(jnp.isfinite(o), o, 0.0)
        out_ref[:, col:col + fo] = o
        col += fo


def _sa_conv(h_in, d2k, r_list, conv_params, bm):
    m, kk, f = h_in.shape
    h_flat = h_in.reshape(m * kk, f)
    d2col = d2k.reshape(m * kk, 1)
    nl = len(conv_params[0])
    fouts = tuple(int(layers[-1]["W"].shape[1]) for layers in conv_params)
    r2s = tuple(np.float32(r * r) for r in r_list)
    wargs, wspecs = [], []
    for layers in conv_params:
        for lyr in layers:
            for nm in ("W", "b", "gamma", "beta"):
                a = lyr[nm]
                if a.ndim == 1:
                    a = a.reshape(1, -1)
                wargs.append(a)
                wspecs.append(pl.BlockSpec(a.shape, lambda i: (0, 0)))
    out_f = sum(fouts)
    grid = (m // bm,)
    fn = pl.pallas_call(
        functools.partial(_sa_body, nl, kk, r2s, fouts),
        grid=grid,
        in_specs=[
            pl.BlockSpec((bm * kk, f), lambda i: (i, 0)),
            pl.BlockSpec((bm * kk, 1), lambda i: (i, 0)),
        ] + wspecs,
        out_specs=pl.BlockSpec((bm, out_f), lambda i: (i, 0)),
        out_shape=jax.ShapeDtypeStruct((m, out_f), jnp.float32),
    )
    return fn(h_flat, d2col, *wargs)


def _sa_module(x, pos, ratio, r_list, conv_params, bm, max_nbrs=128):
    n = pos.shape[0]
    m = int(round(ratio * n))
    idx = _fps_idx(pos, m)
    y_pos = pos[idx]
    nbr = _ballq_sc(y_pos, pos, max_nbrs)
    x_j = x[nbr]
    rel = pos[nbr] - y_pos[:, None, :]
    d2k = jnp.sum(rel ** 2, axis=-1)
    h_in = jnp.concatenate([x_j, rel], axis=-1)
    return _sa_conv(h_in, d2k, r_list, conv_params, bm), y_pos


def _mlp_body(nl, with_head, *refs):
    h_ref = refs[0]
    wrefs = refs[1:-1]
    out_ref = refs[-1]
    h = h_ref[...]
    for li in range(nl):
        w = wrefs[li * 4][...]
        b = wrefs[li * 4 + 1][...]
        g = wrefs[li * 4 + 2][...]
        be = wrefs[li * 4 + 3][...]
        h = jnp.maximum(
            jnp.dot(h, w, preferred_element_type=jnp.float32) + b, 0.0)
        h = g * (h * _INV) + be
    if with_head:
        base = nl * 4
        w1, b1 = wrefs[base][...], wrefs[base + 1][...]
        w2, b2 = wrefs[base + 2][...], wrefs[base + 3][...]
        w3, b3 = wrefs[base + 4][...], wrefs[base + 5][...]
        h = jnp.maximum(jnp.dot(h, w1, preferred_element_type=jnp.float32) + b1, 0.0)
        h = jnp.maximum(jnp.dot(h, w2, preferred_element_type=jnp.float32) + b2, 0.0)
        h = jnp.dot(h, w3, preferred_element_type=jnp.float32) + b3
        mx = jnp.max(h, axis=-1, keepdims=True)
        sh = h - jax.lax.stop_gradient(mx)
        h = sh - jnp.log(jnp.sum(jnp.exp(sh), axis=-1, keepdims=True))
    out_ref[...] = h


def _mlp_rows(h, layers, br, head=None):
    rows, f = h.shape
    nl = len(layers)
    wargs, wspecs = [], []
    for lyr in layers:
        for nm in ("W", "b", "gamma", "beta"):
            a = lyr[nm]
            if a.ndim == 1:
                a = a.reshape(1, -1)
            wargs.append(a)
            wspecs.append(pl.BlockSpec(a.shape, lambda i: (0, 0)))
    if head is not None:
        for nm in ("W1", "b1", "W2", "b2", "W3", "b3"):
            a = head[nm]
            if a.ndim == 1:
                a = a.reshape(1, -1)
            wargs.append(a)
            wspecs.append(pl.BlockSpec(a.shape, lambda i: (0, 0)))
        out_f = head["W3"].shape[1]
    else:
        out_f = layers[-1]["W"].shape[1]
    fn = pl.pallas_call(
        functools.partial(_mlp_body, nl, head is not None),
        grid=(rows // br,),
        in_specs=[pl.BlockSpec((br, f), lambda i: (i, 0))] + wspecs,
        out_specs=pl.BlockSpec((br, out_f), lambda i: (i, 0)),
        out_shape=jax.ShapeDtypeStruct((rows, out_f), jnp.float32),
    )
    return fn(h, *wargs)


def _knn_interpolate(xf, posc, pos_skip, k=3):
    d2_sg = jax.lax.stop_gradient(
        jnp.sum((pos_skip[:, None, :] - posc[None, :, :]) ** 2, axis=-1))
    colid = jnp.arange(posc.shape[0], dtype=jnp.int32)[None, :]
    picks = []
    dcur = d2_sg
    for _ in range(k):
        i = jnp.argmin(dcur, axis=1).astype(jnp.int32)
        picks.append(i)
        dcur = jnp.where(colid == i[:, None], jnp.inf, dcur)
    idx = jnp.stack(picks, axis=1)
    diff = pos_skip[:, None, :] - posc[idx]
    d2 = jnp.sum(diff ** 2, axis=-1)
    w = 1.0 / jnp.maximum(d2, 1e-16)
    w = w / jnp.sum(w, axis=1, keepdims=True)
    return jnp.sum(xf[idx] * w[:, :, None], axis=1)


def kernel(x, pos, batch, params):
    x1, pos1 = _sa_module(x, pos, 0.25, [0.05, 0.4], params["sa1"], bm=32)
    x2, pos2 = _sa_module(x1, pos1, 0.25, [0.2, 0.8], params["sa2"], bm=32)
    x3, pos3 = _sa_module(x2, pos2, 0.25, [0.4, 1.6], params["sa3"], bm=32)

    xi3 = _knn_interpolate(x3, pos3, pos2)
    f3 = _mlp_rows(jnp.concatenate([xi3, x2], axis=1), params["fp3"], br=256)
    xi2 = _knn_interpolate(f3, pos2, pos1)
    f2 = _mlp_rows(jnp.concatenate([xi2, x1], axis=1), params["fp2"], br=512)
    xi1 = _knn_interpolate(f2, pos1, pos)
    f1 = _mlp_rows(jnp.concatenate([xi1, x], axis=1), params["fp1"], br=1024)

    return _mlp_rows(f1, [], br=1024, head=params["cls"])


# speedup vs baseline: 2.0977x; 1.0788x over previous
import functools

import jax
import jax.numpy as jnp
import numpy as np
from jax import lax
from jax.experimental import pallas as pl
from jax.experimental.pallas import tpu as pltpu
from jax.experimental.pallas import tpu_sc as plsc

_EPS_BN = 1e-5
_INV = np.float32(1.0) / np.sqrt(np.float32(1.0 + _EPS_BN))

_L = 16
_NB = 272
_ST = 273
_HW = 4416


def _lane_gather(vec, idx):
    return lax.gather(
        vec, idx[:, None],
        dimension_numbers=lax.GatherDimensionNumbers(
            offset_dims=(), collapsed_slice_dims=(0,), start_index_map=(0,)),
        slice_sizes=(1,),
        mode=lax.GatherScatterMode.PROMISE_IN_BOUNDS)


def _ballq_tec(n, k, qpt, ysliced, *refs):
    (px_h, py_h, pz_h, yx_h, yy_h, yz_h, out_h,
     px_v, py_v, pz_v, yx_v, yy_v, yz_v, bits_v, bits2_v, cb_v, ci_v,
     hist_v, hist2_v, row_v) = refs
    nvec = n // _L
    wid = lax.axis_index("s") * 2 + lax.axis_index("c")

    pltpu.sync_copy(px_h, px_v)
    pltpu.sync_copy(py_h, py_v)
    pltpu.sync_copy(pz_h, pz_v)
    if ysliced:
        pltpu.sync_copy(yx_h.at[pl.ds(wid * qpt, qpt)], yx_v)
        pltpu.sync_copy(yy_h.at[pl.ds(wid * qpt, qpt)], yy_v)
        pltpu.sync_copy(yz_h.at[pl.ds(wid * qpt, qpt)], yz_v)
    else:
        pltpu.sync_copy(yx_h, yx_v)
        pltpu.sync_copy(yy_h, yy_v)
        pltpu.sync_copy(yz_h, yz_v)

    lane = lax.iota(jnp.int32, _L)
    ones = jnp.full((_L,), 1, jnp.int32)
    _U = 4
    zeros = jnp.zeros((_L,), jnp.int32)
    last = jnp.full((_L,), _L - 1, jnp.int32)

    def clear_hist(href):
        def cj(j, c):
            for u in range(_U):
                href[pl.ds((j * _U + u) * _L, _L)] = zeros
            return c
        lax.fori_loop(0, _HW // (_U * _L), cj, 0)

    def scan_hist(href, k_rem, nbins):
        def sj(j, st):
            found, bsel, below, run = st
            acc = jnp.zeros((_L,), jnp.int32)
            for l in range(_L):
                acc = acc + href[pl.ds(l * _ST + j * _L, _L)]
            cum = plsc.cumsum(acc) + run
            run2 = _lane_gather(cum, last)
            hit = cum > k_rem
            nhit = plsc.all_reduce_population_count(hit)
            ffs = jnp.minimum(plsc.all_reduce_ffs(hit), _L - 1)
            excl = cum - acc
            below_here = _lane_gather(excl, ffs)
            bin_here = j * _L + ffs
            take = (found == 0) & (nhit > 0)
            bsel = jnp.where(take, bin_here, bsel)
            below = jnp.where(take, below_here, below)
            found = jnp.where(nhit > 0, 1, found)
            return (found, bsel, below, run2)
        z = jnp.zeros((_L,), jnp.int32)
        _, bsel, below, _ = lax.fori_loop(0, nbins // _L, sj, (z, z, z, z))
        return bsel, below

    def hist_pass_compact(shift, mask, pshift, prefix, cnt, cnt_s):
        clear_hist(hist_v)

        def pi(i, c):
            b = cb_v[pl.ds(i * _L, _L)]
            valid = (i * _L + lane) < cnt
            binv = (b >> shift) & mask
            m = valid & ((b >> pshift) == prefix)
            plsc.addupdate_scatter(hist_v, [lane * _ST + binv], ones, mask=m)
            return c
        lax.fori_loop(0, (cnt_s + _L - 1) // _L, pi, 0)

    def tail(q, bref, href):
        k0 = jnp.full((_L,), k - 1, jnp.int32)
        b1, below1 = scan_hist(href, k0, _NB)
        k1 = k0 - below1

        def p2(i, st):
            bs = [bref[pl.ds((i * _U + u) * _L, _L)] for u in range(_U)]
            rel = (bs[0] >> 22) <= b1
            for u in range(1, _U):
                rel = rel | ((bs[u] >> 22) <= b1)
            sel = jnp.max(rel.astype(jnp.int32))

            def emit(st):
                a_base, c_base = st
                for u in range(_U):
                    b = bs[u]
                    binv = b >> 22
                    lt1 = binv < b1
                    e1 = binv == b1
                    pos_a = a_base + plsc.cumsum(lt1.astype(jnp.int32)) - 1
                    pos_c = c_base + plsc.cumsum(e1.astype(jnp.int32)) - 1
                    idx_v = (i * _U + u) * _L + lane
                    plsc.store_scatter(
                        row_v, [jnp.minimum(pos_a, k - 1)], idx_v, mask=lt1)
                    plsc.store_scatter(ci_v, [pos_c], idx_v, mask=e1)
                    plsc.store_scatter(cb_v, [pos_c], b, mask=e1)
                    a_base = a_base + plsc.all_reduce_population_count(lt1)
                    c_base = c_base + plsc.all_reduce_population_count(e1)
                return (a_base, c_base)

            return lax.cond(sel > 0, emit, lambda s: s, st)
        zv = jnp.zeros((_L,), jnp.int32)
        _, cnt = lax.fori_loop(0, nvec // _U, p2, (zv, zv))
        cnt_s = jnp.max(cnt)

        hist_pass_compact(14, 0xFF, 22, b1, cnt, cnt_s)
        b2, below2 = scan_hist(hist_v, k1, 256)
        k2 = k1 - below2
        pre2 = (b1 << 8) | b2

        hist_pass_compact(6, 0xFF, 14, pre2, cnt, cnt_s)
        b3, below3 = scan_hist(hist_v, k2, 256)
        k3 = k2 - below3
        pre3 = (pre2 << 8) | b3

        hist_pass_compact(0, 0x3F, 6, pre3, cnt, cnt_s)
        b4, below4 = scan_hist(hist_v, k3, 64)

        t = (pre3 << 6) | b4
        count_lt = below1 + below2 + below3 + below4

        def fp(i, st):
            lt_base, eq_base = st
            b = cb_v[pl.ds(i * _L, _L)]
            valid = (i * _L + lane) < cnt
            lt = valid & (b < t)
            eq = valid & (b == t)
            pos_lt = lt_base + plsc.cumsum(lt.astype(jnp.int32)) - 1
            pos_eq = eq_base + plsc.cumsum(eq.astype(jnp.int32)) - 1
            idx_v = ci_v[pl.ds(i * _L, _L)]
            plsc.store_scatter(
                row_v, [jnp.minimum(pos_lt, k - 1)], idx_v, mask=lt)
            eqm = eq & (pos_eq < k)
            plsc.store_scatter(
                row_v, [jnp.minimum(pos_eq, k - 1)], idx_v, mask=eqm)
            return (lt_base + plsc.all_reduce_population_count(lt),
                    eq_base + plsc.all_reduce_population_count(eq))
        lax.fori_loop(0, (cnt_s + _L - 1) // _L, fp, (below1, count_lt))

        pltpu.sync_copy(row_v, out_h.at[q])

    def per_pair(lp, carry):
        lq0 = 2 * lp
        q0 = wid * qpt + lq0
        loc = lq0 if ysliced else q0
        base = (loc // _L) * _L
        off0 = jnp.full((_L,), loc - base, jnp.int32)
        off1 = off0 + 1
        yxc = yx_v[pl.ds(base, _L)]
        yyc = yy_v[pl.ds(base, _L)]
        yzc = yz_v[pl.ds(base, _L)]
        yx0 = _lane_gather(yxc, off0)
        yy0 = _lane_gather(yyc, off0)
        yz0 = _lane_gather(yzc, off0)
        yx1 = _lane_gather(yxc, off1)
        yy1 = _lane_gather(yyc, off1)
        yz1 = _lane_gather(yzc, off1)

        clear_hist(hist_v)
        clear_hist(hist2_v)

        def p1(i, c):
            for u in range(_U):
                sl = pl.ds((i * _U + u) * _L, _L)
                px = px_v[sl]
                py = py_v[sl]
                pz = pz_v[sl]
                dx0 = px - yx0
                dy0 = py - yy0
                dz0 = pz - yz0
                dx1 = px - yx1
                dy1 = py - yy1
                dz1 = pz - yz1
                d20 = dx0 * dx0 + dy0 * dy0 + dz0 * dz0
                d21 = dx1 * dx1 + dy1 * dy1 + dz1 * dz1
                b0 = lax.bitcast_convert_type(d20, jnp.int32)
                b1 = lax.bitcast_convert_type(d21, jnp.int32)
                bits_v[sl] = b0
                bits2_v[sl] = b1
                plsc.addupdate_scatter(hist_v, [lane * _ST + (b0 >> 22)], ones)
                plsc.addupdate_scatter(hist2_v, [lane * _ST + (b1 >> 22)], ones)
            return c
        lax.fori_loop(0, nvec // _U, p1, 0)

        tail(q0, bits_v, hist_v)
        tail(q0 + 1, bits2_v, hist2_v)
        return carry

    lax.fori_loop(0, qpt // 2, per_pair, 0)


def _ballq_sc(y_pos, pos, k):
    m = y_pos.shape[0]
    n = pos.shape[0]
    qpt = m // 32
    kout = max(k, _L)
    ysliced = qpt >= 32
    ylen = qpt if ysliced else m
    mesh = plsc.VectorSubcoreMesh(core_axis_name="c", subcore_axis_name="s")
    fn = functools.partial(
        pl.kernel,
        mesh=mesh,
        compiler_params=pltpu.CompilerParams(needs_layout_passes=False),
        out_type=jax.ShapeDtypeStruct((m, kout), jnp.int32),
        scratch_types=[
            pltpu.VMEM((n,), jnp.float32),
            pltpu.VMEM((n,), jnp.float32),
            pltpu.VMEM((n,), jnp.float32),
            pltpu.VMEM((ylen,), jnp.float32),
            pltpu.VMEM((ylen,), jnp.float32),
            pltpu.VMEM((ylen,), jnp.float32),
            pltpu.VMEM((n,), jnp.int32),
            pltpu.VMEM((n,), jnp.int32),
            pltpu.VMEM((n,), jnp.int32),
            pltpu.VMEM((n,), jnp.int32),
            pltpu.VMEM((_HW,), jnp.int32),
            pltpu.VMEM((_HW,), jnp.int32),
            pltpu.VMEM((kout,), jnp.int32),
        ],
    )(functools.partial(_ballq_tec, n, k, qpt, ysliced))
    out = fn(pos[:, 0], pos[:, 1], pos[:, 2],
             y_pos[:, 0], y_pos[:, 1], y_pos[:, 2])
    return out[:, :k] if kout != k else out


def _fps_body(m, px_ref, py_ref, pz_ref, out_ref):
    px = px_ref[...]
    py = py_ref[...]
    pz = pz_ref[...]
    r = px.shape[0]
    row = jax.lax.broadcasted_iota(jnp.int32, (r, 128), 0)
    colv = jax.lax.broadcasted_iota(jnp.int32, (r, 128), 1)
    flat = row * 128 + colv
    out_ref[...] = jnp.zeros(out_ref.shape, jnp.int32)
    dists0 = jnp.full((r, 128), jnp.inf, jnp.float32)

    def body(i, carry):
        dists, last = carry
        sel = flat == last
        lx = jnp.sum(jnp.where(sel, px, 0.0))
        ly = jnp.sum(jnp.where(sel, py, 0.0))
        lz = jnp.sum(jnp.where(sel, pz, 0.0))
        dxx = px - lx
        dyy = py - ly
        dzz = pz - lz
        d = dxx * dxx + dyy * dyy + dzz * dzz
        dists = jnp.minimum(dists, d)
        mx = jnp.max(dists)
        idx = jnp.min(jnp.where(dists == mx, flat, jnp.int32(2 ** 30)))
        out_ref[pl.ds(i, 1), :] = jnp.reshape(idx, (1, 1))
        return (dists, idx)

    jax.lax.fori_loop(1, m, body, (dists0, jnp.int32(0)))


def _fps_idx(pos, num_samples):
    n = pos.shape[0]
    r = n // 128
    px = pos[:, 0].reshape(r, 128)
    py = pos[:, 1].reshape(r, 128)
    pz = pos[:, 2].reshape(r, 128)
    out = pl.pallas_call(
        functools.partial(_fps_body, num_samples),
        out_shape=jax.ShapeDtypeStruct((num_samples, 1), jnp.int32),
    )(px, py, pz)
    return out.reshape(num_samples)


def _sa_body(nl, kk, r2s, fouts, *refs):
    h_ref, d2_ref = refs[0], refs[1]
    wrefs = refs[2:-1]
    out_ref = refs[-1]
    h0 = h_ref[...]
    d2col = d2_ref[...]
    bm = d2col.shape[0] // kk
    col = 0
    for bi, r2 in enumerate(r2s):
        h = h0
        base = bi * nl * 4
        for li in range(nl):
            w = wrefs[base + li * 4][...]
            b = wrefs[base + li * 4 + 1][...]
            g = wrefs[base + li * 4 + 2][...]
            be = wrefs[base + li * 4 + 3][...]
            h = jnp.maximum(
                jnp.dot(h, w, preferred_element_type=jnp.float32) + b, 0.0)
            h = g * (h * _INV) + be
        fo = fouts[bi]
        penalty = jnp.where(d2col <= r2, 0.0, -jnp.inf)
        h = h + penalty
        o = jnp.max(h.reshape(bm, kk, fo), axis=1)
        o = jnp.where(jnp.isfinite(o), o, 0.0)
        out_ref[:, col:col + fo] = o
        col += fo


def _sa_conv(h_in, d2k, r_list, conv_params, bm):
    m, kk, f = h_in.shape
    h_flat = h_in.reshape(m * kk, f)
    d2col = d2k.reshape(m * kk, 1)
    nl = len(conv_params[0])
    fouts = tuple(int(layers[-1]["W"].shape[1]) for layers in conv_params)
    r2s = tuple(np.float32(r * r) for r in r_list)
    wargs, wspecs = [], []
    for layers in conv_params:
        for lyr in layers:
            for nm in ("W", "b", "gamma", "beta"):
                a = lyr[nm]
                if a.ndim == 1:
                    a = a.reshape(1, -1)
                wargs.append(a)
                wspecs.append(pl.BlockSpec(a.shape, lambda i: (0, 0)))
    out_f = sum(fouts)
    grid = (m // bm,)
    fn = pl.pallas_call(
        functools.partial(_sa_body, nl, kk, r2s, fouts),
        grid=grid,
        in_specs=[
            pl.BlockSpec((bm * kk, f), lambda i: (i, 0)),
            pl.BlockSpec((bm * kk, 1), lambda i: (i, 0)),
        ] + wspecs,
        out_specs=pl.BlockSpec((bm, out_f), lambda i: (i, 0)),
        out_shape=jax.ShapeDtypeStruct((m, out_f), jnp.float32),
    )
    return fn(h_flat, d2col, *wargs)


def _sa_module(x, pos, ratio, r_list, conv_params, bm, max_nbrs=128):
    n = pos.shape[0]
    m = int(round(ratio * n))
    idx = _fps_idx(pos, m)
    y_pos = pos[idx]
    nbr = _ballq_sc(y_pos, pos, max_nbrs)
    x_j = x[nbr]
    rel = pos[nbr] - y_pos[:, None, :]
    d2k = jnp.sum(rel ** 2, axis=-1)
    h_in = jnp.concatenate([x_j, rel], axis=-1)
    return _sa_conv(h_in, d2k, r_list, conv_params, bm), y_pos


def _mlp_body(nl, with_head, *refs):
    h_ref = refs[0]
    wrefs = refs[1:-1]
    out_ref = refs[-1]
    h = h_ref[...]
    for li in range(nl):
        w = wrefs[li * 4][...]
        b = wrefs[li * 4 + 1][...]
        g = wrefs[li * 4 + 2][...]
        be = wrefs[li * 4 + 3][...]
        h = jnp.maximum(
            jnp.dot(h, w, preferred_element_type=jnp.float32) + b, 0.0)
        h = g * (h * _INV) + be
    if with_head:
        base = nl * 4
        w1, b1 = wrefs[base][...], wrefs[base + 1][...]
        w2, b2 = wrefs[base + 2][...], wrefs[base + 3][...]
        w3, b3 = wrefs[base + 4][...], wrefs[base + 5][...]
        h = jnp.maximum(jnp.dot(h, w1, preferred_element_type=jnp.float32) + b1, 0.0)
        h = jnp.maximum(jnp.dot(h, w2, preferred_element_type=jnp.float32) + b2, 0.0)
        h = jnp.dot(h, w3, preferred_element_type=jnp.float32) + b3
        mx = jnp.max(h, axis=-1, keepdims=True)
        sh = h - jax.lax.stop_gradient(mx)
        h = sh - jnp.log(jnp.sum(jnp.exp(sh), axis=-1, keepdims=True))
    out_ref[...] = h


def _mlp_rows(h, layers, br, head=None):
    rows, f = h.shape
    nl = len(layers)
    wargs, wspecs = [], []
    for lyr in layers:
        for nm in ("W", "b", "gamma", "beta"):
            a = lyr[nm]
            if a.ndim == 1:
                a = a.reshape(1, -1)
            wargs.append(a)
            wspecs.append(pl.BlockSpec(a.shape, lambda i: (0, 0)))
    if head is not None:
        for nm in ("W1", "b1", "W2", "b2", "W3", "b3"):
            a = head[nm]
            if a.ndim == 1:
                a = a.reshape(1, -1)
            wargs.append(a)
            wspecs.append(pl.BlockSpec(a.shape, lambda i: (0, 0)))
        out_f = head["W3"].shape[1]
    else:
        out_f = layers[-1]["W"].shape[1]
    fn = pl.pallas_call(
        functools.partial(_mlp_body, nl, head is not None),
        grid=(rows // br,),
        in_specs=[pl.BlockSpec((br, f), lambda i: (i, 0))] + wspecs,
        out_specs=pl.BlockSpec((br, out_f), lambda i: (i, 0)),
        out_shape=jax.ShapeDtypeStruct((rows, out_f), jnp.float32),
    )
    return fn(h, *wargs)


def _knn_interpolate(xf, posc, pos_skip, k=3):
    d2_sg = jax.lax.stop_gradient(
        jnp.sum((pos_skip[:, None, :] - posc[None, :, :]) ** 2, axis=-1))
    colid = jnp.arange(posc.shape[0], dtype=jnp.int32)[None, :]
    picks = []
    dcur = d2_sg
    for _ in range(k):
        i = jnp.argmin(dcur, axis=1).astype(jnp.int32)
        picks.append(i)
        dcur = jnp.where(colid == i[:, None], jnp.inf, dcur)
    idx = jnp.stack(picks, axis=1)
    diff = pos_skip[:, None, :] - posc[idx]
    d2 = jnp.sum(diff ** 2, axis=-1)
    w = 1.0 / jnp.maximum(d2, 1e-16)
    w = w / jnp.sum(w, axis=1, keepdims=True)
    return jnp.sum(xf[idx] * w[:, :, None], axis=1)


def kernel(x, pos, batch, params):
    x1, pos1 = _sa_module(x, pos, 0.25, [0.05, 0.4], params["sa1"], bm=32)
    x2, pos2 = _sa_module(x1, pos1, 0.25, [0.2, 0.8], params["sa2"], bm=32)
    x3, pos3 = _sa_module(x2, pos2, 0.25, [0.4, 1.6], params["sa3"], bm=32)

    xi3 = _knn_interpolate(x3, pos3, pos2)
    f3 = _mlp_rows(jnp.concatenate([xi3, x2], axis=1), params["fp3"], br=256)
    xi2 = _knn_interpolate(f3, pos2, pos1)
    f2 = _mlp_rows(jnp.concatenate([xi2, x1], axis=1), params["fp2"], br=512)
    xi1 = _knn_interpolate(f2, pos1, pos)
    f1 = _mlp_rows(jnp.concatenate([xi1, x], axis=1), params["fp1"], br=1024)

    return _mlp_rows(f1, [], br=1024, head=params["cls"])


# speedup vs baseline: 2.8220x; 1.3453x over previous
import functools

import jax
import jax.numpy as jnp
import numpy as np
from jax import lax
from jax.experimental import pallas as pl
from jax.experimental.pallas import tpu as pltpu
from jax.experimental.pallas import tpu_sc as plsc

_EPS_BN = 1e-5
_INV = np.float32(1.0) / np.sqrt(np.float32(1.0 + _EPS_BN))

_L = 16
_NB = 272
_ST = 273
_HW = 4416


def _lane_gather(vec, idx):
    return lax.gather(
        vec, idx[:, None],
        dimension_numbers=lax.GatherDimensionNumbers(
            offset_dims=(), collapsed_slice_dims=(0,), start_index_map=(0,)),
        slice_sizes=(1,),
        mode=lax.GatherScatterMode.PROMISE_IN_BOUNDS)


def _ballq_tec(n, k, qpt, ysliced, *refs):
    (px_h, py_h, pz_h, yx_h, yy_h, yz_h, out_h,
     px_v, py_v, pz_v, yx_v, yy_v, yz_v, bits_v, bits2_v, cb_v, ci_v,
     hist_v, hist2_v, row_v) = refs
    nvec = n // _L
    wid = lax.axis_index("s") * 2 + lax.axis_index("c")

    pltpu.sync_copy(px_h, px_v)
    pltpu.sync_copy(py_h, py_v)
    pltpu.sync_copy(pz_h, pz_v)
    if ysliced:
        pltpu.sync_copy(yx_h.at[pl.ds(wid * qpt, qpt)], yx_v)
        pltpu.sync_copy(yy_h.at[pl.ds(wid * qpt, qpt)], yy_v)
        pltpu.sync_copy(yz_h.at[pl.ds(wid * qpt, qpt)], yz_v)
    else:
        pltpu.sync_copy(yx_h, yx_v)
        pltpu.sync_copy(yy_h, yy_v)
        pltpu.sync_copy(yz_h, yz_v)

    lane = lax.iota(jnp.int32, _L)
    ones = jnp.full((_L,), 1, jnp.int32)
    _U = 4
    zeros = jnp.zeros((_L,), jnp.int32)
    last = jnp.full((_L,), _L - 1, jnp.int32)

    def clear_hist(href):
        def cj(j, c):
            for u in range(_U):
                href[pl.ds((j * _U + u) * _L, _L)] = zeros
            return c
        lax.fori_loop(0, _HW // (_U * _L), cj, 0)

    def scan_hist(href, k_rem, nbins):
        def sj(j, st):
            found, bsel, below, run = st
            acc = jnp.zeros((_L,), jnp.int32)
            for l in range(_L):
                acc = acc + href[pl.ds(l * _ST + j * _L, _L)]
            cum = plsc.cumsum(acc) + run
            run2 = _lane_gather(cum, last)
            hit = cum > k_rem
            nhit = plsc.all_reduce_population_count(hit)
            ffs = jnp.minimum(plsc.all_reduce_ffs(hit), _L - 1)
            excl = cum - acc
            below_here = _lane_gather(excl, ffs)
            bin_here = j * _L + ffs
            take = (found == 0) & (nhit > 0)
            bsel = jnp.where(take, bin_here, bsel)
            below = jnp.where(take, below_here, below)
            found = jnp.where(nhit > 0, 1, found)
            return (found, bsel, below, run2)
        z = jnp.zeros((_L,), jnp.int32)
        _, bsel, below, _ = lax.fori_loop(0, nbins // _L, sj, (z, z, z, z))
        return bsel, below

    def hist_pass_compact(shift, mask, pshift, prefix, cnt, cnt_s):
        clear_hist(hist_v)

        def pi(i, c):
            b = cb_v[pl.ds(i * _L, _L)]
            valid = (i * _L + lane) < cnt
            binv = (b >> shift) & mask
            m = valid & ((b >> pshift) == prefix)
            plsc.addupdate_scatter(hist_v, [lane * _ST + binv], ones, mask=m)
            return c
        lax.fori_loop(0, (cnt_s + _L - 1) // _L, pi, 0)

    def tail(q, bref, href):
        k0 = jnp.full((_L,), k - 1, jnp.int32)
        b1, below1 = scan_hist(href, k0, _NB)
        k1 = k0 - below1

        def p2(i, st):
            bs = [bref[pl.ds((i * _U + u) * _L, _L)] for u in range(_U)]
            rel = (bs[0] >> 22) <= b1
            for u in range(1, _U):
                rel = rel | ((bs[u] >> 22) <= b1)
            sel = jnp.max(rel.astype(jnp.int32))

            def emit(st):
                a_base, c_base = st
                for u in range(_U):
                    b = bs[u]
                    binv = b >> 22
                    lt1 = binv < b1
                    e1 = binv == b1
                    pos_a = a_base + plsc.cumsum(lt1.astype(jnp.int32)) - 1
                    pos_c = c_base + plsc.cumsum(e1.astype(jnp.int32)) - 1
                    idx_v = (i * _U + u) * _L + lane
                    plsc.store_scatter(
                        row_v, [jnp.minimum(pos_a, k - 1)], idx_v, mask=lt1)
                    plsc.store_scatter(ci_v, [pos_c], idx_v, mask=e1)
                    plsc.store_scatter(cb_v, [pos_c], b, mask=e1)
                    a_base = a_base + plsc.all_reduce_population_count(lt1)
                    c_base = c_base + plsc.all_reduce_population_count(e1)
                return (a_base, c_base)

            return lax.cond(sel > 0, emit, lambda s: s, st)
        zv = jnp.zeros((_L,), jnp.int32)
        _, cnt = lax.fori_loop(0, nvec // _U, p2, (zv, zv))
        cnt_s = jnp.max(cnt)

        hist_pass_compact(14, 0xFF, 22, b1, cnt, cnt_s)
        b2, below2 = scan_hist(hist_v, k1, 256)
        k2 = k1 - below2
        pre2 = (b1 << 8) | b2

        hist_pass_compact(6, 0xFF, 14, pre2, cnt, cnt_s)
        b3, below3 = scan_hist(hist_v, k2, 256)
        k3 = k2 - below3
        pre3 = (pre2 << 8) | b3

        hist_pass_compact(0, 0x3F, 6, pre3, cnt, cnt_s)
        b4, below4 = scan_hist(hist_v, k3, 64)

        t = (pre3 << 6) | b4
        count_lt = below1 + below2 + below3 + below4

        def fp(i, st):
            lt_base, eq_base = st
            b = cb_v[pl.ds(i * _L, _L)]
            valid = (i * _L + lane) < cnt
            lt = valid & (b < t)
            eq = valid & (b == t)
            pos_lt = lt_base + plsc.cumsum(lt.astype(jnp.int32)) - 1
            pos_eq = eq_base + plsc.cumsum(eq.astype(jnp.int32)) - 1
            idx_v = ci_v[pl.ds(i * _L, _L)]
            plsc.store_scatter(
                row_v, [jnp.minimum(pos_lt, k - 1)], idx_v, mask=lt)
            eqm = eq & (pos_eq < k)
            plsc.store_scatter(
                row_v, [jnp.minimum(pos_eq, k - 1)], idx_v, mask=eqm)
            return (lt_base + plsc.all_reduce_population_count(lt),
                    eq_base + plsc.all_reduce_population_count(eq))
        lax.fori_loop(0, (cnt_s + _L - 1) // _L, fp, (below1, count_lt))

        pltpu.sync_copy(row_v, out_h.at[q])

    def per_pair(lp, carry):
        lq0 = 2 * lp
        q0 = wid * qpt + lq0
        loc = lq0 if ysliced else q0
        base = (loc // _L) * _L
        off0 = jnp.full((_L,), loc - base, jnp.int32)
        off1 = off0 + 1
        yxc = yx_v[pl.ds(base, _L)]
        yyc = yy_v[pl.ds(base, _L)]
        yzc = yz_v[pl.ds(base, _L)]
        yx0 = _lane_gather(yxc, off0)
        yy0 = _lane_gather(yyc, off0)
        yz0 = _lane_gather(yzc, off0)
        yx1 = _lane_gather(yxc, off1)
        yy1 = _lane_gather(yyc, off1)
        yz1 = _lane_gather(yzc, off1)

        clear_hist(hist_v)
        clear_hist(hist2_v)

        def p1(i, c):
            for u in range(_U):
                sl = pl.ds((i * _U + u) * _L, _L)
                px = px_v[sl]
                py = py_v[sl]
                pz = pz_v[sl]
                dx0 = px - yx0
                dy0 = py - yy0
                dz0 = pz - yz0
                dx1 = px - yx1
                dy1 = py - yy1
                dz1 = pz - yz1
                d20 = dx0 * dx0 + dy0 * dy0 + dz0 * dz0
                d21 = dx1 * dx1 + dy1 * dy1 + dz1 * dz1
                b0 = lax.bitcast_convert_type(d20, jnp.int32)
                b1 = lax.bitcast_convert_type(d21, jnp.int32)
                bits_v[sl] = b0
                bits2_v[sl] = b1
                plsc.addupdate_scatter(hist_v, [lane * _ST + (b0 >> 22)], ones)
                plsc.addupdate_scatter(hist2_v, [lane * _ST + (b1 >> 22)], ones)
            return c
        lax.fori_loop(0, nvec // _U, p1, 0)

        tail(q0, bits_v, hist_v)
        tail(q0 + 1, bits2_v, hist2_v)
        return carry

    lax.fori_loop(0, qpt // 2, per_pair, 0)


def _ballq_sc(y_pos, pos, k):
    m = y_pos.shape[0]
    n = pos.shape[0]
    qpt = m // 32
    kout = max(k, _L)
    ysliced = qpt >= 32
    ylen = qpt if ysliced else m
    mesh = plsc.VectorSubcoreMesh(core_axis_name="c", subcore_axis_name="s")
    fn = functools.partial(
        pl.kernel,
        mesh=mesh,
        compiler_params=pltpu.CompilerParams(needs_layout_passes=False),
        out_type=jax.ShapeDtypeStruct((m, kout), jnp.int32),
        scratch_types=[
            pltpu.VMEM((n,), jnp.float32),
            pltpu.VMEM((n,), jnp.float32),
            pltpu.VMEM((n,), jnp.float32),
            pltpu.VMEM((ylen,), jnp.float32),
            pltpu.VMEM((ylen,), jnp.float32),
            pltpu.VMEM((ylen,), jnp.float32),
            pltpu.VMEM((n,), jnp.int32),
            pltpu.VMEM((n,), jnp.int32),
            pltpu.VMEM((n,), jnp.int32),
            pltpu.VMEM((n,), jnp.int32),
            pltpu.VMEM((_HW,), jnp.int32),
            pltpu.VMEM((_HW,), jnp.int32),
            pltpu.VMEM((kout,), jnp.int32),
        ],
    )(functools.partial(_ballq_tec, n, k, qpt, ysliced))
    out = fn(pos[:, 0], pos[:, 1], pos[:, 2],
             y_pos[:, 0], y_pos[:, 1], y_pos[:, 2])
    return out[:, :k] if kout != k else out


def _gather_tec(bpw, csz, *refs):
    tab_h, idx_h, out_h, idxc_v, rows_v, sem = refs
    wid = lax.axis_index("s") * 2 + lax.axis_index("c")
    base = wid * bpw

    def chunk(c, carry):
        start = base + c * csz
        pltpu.sync_copy(idx_h.at[pl.ds(start, csz)], idxc_v)
        pltpu.async_copy(tab_h.at[idxc_v], rows_v, sem).wait()
        pltpu.sync_copy(rows_v, out_h.at[pl.ds(start, csz)])
        return carry

    lax.fori_loop(0, bpw // csz, chunk, 0)


def _gather_sc(tab, idx):
    rows = idx.shape[0]
    d = tab.shape[1]
    bpw = rows // 32
    csz = min(bpw, 512)
    mesh = plsc.VectorSubcoreMesh(core_axis_name="c", subcore_axis_name="s")
    fn = functools.partial(
        pl.kernel,
        mesh=mesh,
        compiler_params=pltpu.CompilerParams(
            needs_layout_passes=False, use_tc_tiling_on_sc=False),
        out_type=jax.ShapeDtypeStruct((rows, d), jnp.float32),
        scratch_types=[
            pltpu.VMEM((csz,), jnp.int32),
            pltpu.VMEM((csz, d), jnp.float32),
            pltpu.SemaphoreType.DMA,
        ],
    )(functools.partial(_gather_tec, bpw, csz))
    return fn(tab, idx)


def _fps_body(m, px_ref, py_ref, pz_ref, out_ref):
    px = px_ref[...]
    py = py_ref[...]
    pz = pz_ref[...]
    r = px.shape[0]
    row = jax.lax.broadcasted_iota(jnp.int32, (r, 128), 0)
    colv = jax.lax.broadcasted_iota(jnp.int32, (r, 128), 1)
    flat = row * 128 + colv
    out_ref[...] = jnp.zeros(out_ref.shape, jnp.int32)
    dists0 = jnp.full((r, 128), jnp.inf, jnp.float32)

    def body(i, carry):
        dists, last = carry
        sel = flat == last
        lx = jnp.sum(jnp.where(sel, px, 0.0))
        ly = jnp.sum(jnp.where(sel, py, 0.0))
        lz = jnp.sum(jnp.where(sel, pz, 0.0))
        dxx = px - lx
        dyy = py - ly
        dzz = pz - lz
        d = dxx * dxx + dyy * dyy + dzz * dzz
        dists = jnp.minimum(dists, d)
        mx = jnp.max(dists)
        idx = jnp.min(jnp.where(dists == mx, flat, jnp.int32(2 ** 30)))
        out_ref[pl.ds(i, 1), :] = jnp.reshape(idx, (1, 1))
        return (dists, idx)

    jax.lax.fori_loop(1, m, body, (dists0, jnp.int32(0)))


def _fps_idx(pos, num_samples):
    n = pos.shape[0]
    r = n // 128
    px = pos[:, 0].reshape(r, 128)
    py = pos[:, 1].reshape(r, 128)
    pz = pos[:, 2].reshape(r, 128)
    out = pl.pallas_call(
        functools.partial(_fps_body, num_samples),
        out_shape=jax.ShapeDtypeStruct((num_samples, 1), jnp.int32),
    )(px, py, pz)
    return out.reshape(num_samples)


def _sa_body(nl, kk, r2s, fouts, *refs):
    h_ref, d2_ref = refs[0], refs[1]
    wrefs = refs[2:-1]
    out_ref = refs[-1]
    h0 = h_ref[...]
    d2col = d2_ref[...]
    bm = d2col.shape[0] // kk
    col = 0
    for bi, r2 in enumerate(r2s):
        h = h0
        base = bi * nl * 4
        for li in range(nl):
            w = wrefs[base + li * 4][...]
            b = wrefs[base + li * 4 + 1][...]
            g = wrefs[base + li * 4 + 2][...]
            be = wrefs[base + li * 4 + 3][...]
            h = jnp.maximum(
                jnp.dot(h, w, preferred_element_type=jnp.float32) + b, 0.0)
            h = g * (h * _INV) + be
        fo = fouts[bi]
        penalty = jnp.where(d2col <= r2, 0.0, -jnp.inf)
        h = h + penalty
        o = jnp.max(h.reshape(bm, kk, fo), axis=1)
        o = jnp.where(jnp.isfinite(o), o, 0.0)
        out_ref[:, col:col + fo] = o
        col += fo


def _sa_conv(h_in, d2k, r_list, conv_params, bm):
    m, kk, f = h_in.shape
    h_flat = h_in.reshape(m * kk, f)
    d2col = d2k.reshape(m * kk, 1)
    nl = len(conv_params[0])
    fouts = tuple(int(layers[-1]["W"].shape[1]) for layers in conv_params)
    r2s = tuple(np.float32(r * r) for r in r_list)
    wargs, wspecs = [], []
    for layers in conv_params:
        for lyr in layers:
            for nm in ("W", "b", "gamma", "beta"):
                a = lyr[nm]
                if a.ndim == 1:
                    a = a.reshape(1, -1)
                wargs.append(a)
                wspecs.append(pl.BlockSpec(a.shape, lambda i: (0, 0)))
    out_f = sum(fouts)
    grid = (m // bm,)
    fn = pl.pallas_call(
        functools.partial(_sa_body, nl, kk, r2s, fouts),
        grid=grid,
        in_specs=[
            pl.BlockSpec((bm * kk, f), lambda i: (i, 0)),
            pl.BlockSpec((bm * kk, 1), lambda i: (i, 0)),
        ] + wspecs,
        out_specs=pl.BlockSpec((bm, out_f), lambda i: (i, 0)),
        out_shape=jax.ShapeDtypeStruct((m, out_f), jnp.float32),
    )
    return fn(h_flat, d2col, *wargs)


def _sa_module(x, pos, ratio, r_list, conv_params, bm, max_nbrs=128):
    n = pos.shape[0]
    m = int(round(ratio * n))
    idx = _fps_idx(pos, m)
    y_pos = pos[idx]
    nbr = _ballq_sc(y_pos, pos, max_nbrs)
    c = x.shape[1]
    dpad = ((c + 3 + 15) // 16) * 16
    xp = jnp.concatenate(
        [x, pos, jnp.zeros((n, dpad - c - 3), jnp.float32)], axis=1)
    g = _gather_sc(xp, nbr.reshape(-1)).reshape(m, max_nbrs, dpad)
    x_j = g[:, :, :c]
    rel = g[:, :, c:c + 3] - y_pos[:, None, :]
    d2k = jnp.sum(rel ** 2, axis=-1)
    h_in = jnp.concatenate([x_j, rel], axis=-1)
    return _sa_conv(h_in, d2k, r_list, conv_params, bm), y_pos


def _mlp_body(nl, with_head, *refs):
    h_ref = refs[0]
    wrefs = refs[1:-1]
    out_ref = refs[-1]
    h = h_ref[...]
    for li in range(nl):
        w = wrefs[li * 4][...]
        b = wrefs[li * 4 + 1][...]
        g = wrefs[li * 4 + 2][...]
        be = wrefs[li * 4 + 3][...]
        h = jnp.maximum(
            jnp.dot(h, w, preferred_element_type=jnp.float32) + b, 0.0)
        h = g * (h * _INV) + be
    if with_head:
        base = nl * 4
        w1, b1 = wrefs[base][...], wrefs[base + 1][...]
        w2, b2 = wrefs[base + 2][...], wrefs[base + 3][...]
        w3, b3 = wrefs[base + 4][...], wrefs[base + 5][...]
        h = jnp.maximum(jnp.dot(h, w1, preferred_element_type=jnp.float32) + b1, 0.0)
        h = jnp.maximum(jnp.dot(h, w2, preferred_element_type=jnp.float32) + b2, 0.0)
        h = jnp.dot(h, w3, preferred_element_type=jnp.float32) + b3
        mx = jnp.max(h, axis=-1, keepdims=True)
        sh = h - jax.lax.stop_gradient(mx)
        h = sh - jnp.log(jnp.sum(jnp.exp(sh), axis=-1, keepdims=True))
    out_ref[...] = h


def _mlp_rows(h, layers, br, head=None):
    rows, f = h.shape
    nl = len(layers)
    wargs, wspecs = [], []
    for lyr in layers:
        for nm in ("W", "b", "gamma", "beta"):
            a = lyr[nm]
            if a.ndim == 1:
                a = a.reshape(1, -1)
            wargs.append(a)
            wspecs.append(pl.BlockSpec(a.shape, lambda i: (0, 0)))
    if head is not None:
        for nm in ("W1", "b1", "W2", "b2", "W3", "b3"):
            a = head[nm]
            if a.ndim == 1:
                a = a.reshape(1, -1)
            wargs.append(a)
            wspecs.append(pl.BlockSpec(a.shape, lambda i: (0, 0)))
        out_f = head["W3"].shape[1]
    else:
        out_f = layers[-1]["W"].shape[1]
    fn = pl.pallas_call(
        functools.partial(_mlp_body, nl, head is not None),
        grid=(rows // br,),
        in_specs=[pl.BlockSpec((br, f), lambda i: (i, 0))] + wspecs,
        out_specs=pl.BlockSpec((br, out_f), lambda i: (i, 0)),
        out_shape=jax.ShapeDtypeStruct((rows, out_f), jnp.float32),
    )
    return fn(h, *wargs)


def _knn_interpolate(xf, posc, pos_skip, k=3):
    d2_sg = jax.lax.stop_gradient(
        jnp.sum((pos_skip[:, None, :] - posc[None, :, :]) ** 2, axis=-1))
    colid = jnp.arange(posc.shape[0], dtype=jnp.int32)[None, :]
    picks = []
    dcur = d2_sg
    for _ in range(k):
        i = jnp.argmin(dcur, axis=1).astype(jnp.int32)
        picks.append(i)
        dcur = jnp.where(colid == i[:, None], jnp.inf, dcur)
    idx = jnp.stack(picks, axis=1)
    diff = pos_skip[:, None, :] - posc[idx]
    d2 = jnp.sum(diff ** 2, axis=-1)
    w = 1.0 / jnp.maximum(d2, 1e-16)
    w = w / jnp.sum(w, axis=1, keepdims=True)
    return jnp.sum(xf[idx] * w[:, :, None], axis=1)


def kernel(x, pos, batch, params):
    x1, pos1 = _sa_module(x, pos, 0.25, [0.05, 0.4], params["sa1"], bm=32)
    x2, pos2 = _sa_module(x1, pos1, 0.25, [0.2, 0.8], params["sa2"], bm=32)
    x3, pos3 = _sa_module(x2, pos2, 0.25, [0.4, 1.6], params["sa3"], bm=32)

    xi3 = _knn_interpolate(x3, pos3, pos2)
    f3 = _mlp_rows(jnp.concatenate([xi3, x2], axis=1), params["fp3"], br=256)
    xi2 = _knn_interpolate(f3, pos2, pos1)
    f2 = _mlp_rows(jnp.concatenate([xi2, x1], axis=1), params["fp2"], br=512)
    xi1 = _knn_interpolate(f2, pos1, pos)
    f1 = _mlp_rows(jnp.concatenate([xi1, x], axis=1), params["fp1"], br=1024)

    return _mlp_rows(f1, [], br=1024, head=params["cls"])


# speedup vs baseline: 2.8330x; 1.0039x over previous
import functools

import jax
import jax.numpy as jnp
import numpy as np
from jax import lax
from jax.experimental import pallas as pl
from jax.experimental.pallas import tpu as pltpu
from jax.experimental.pallas import tpu_sc as plsc

_EPS_BN = 1e-5
_INV = np.float32(1.0) / np.sqrt(np.float32(1.0 + _EPS_BN))

_L = 16
_NB = 272
_ST = 273
_HW = 4416


def _lane_gather(vec, idx):
    return lax.gather(
        vec, idx[:, None],
        dimension_numbers=lax.GatherDimensionNumbers(
            offset_dims=(), collapsed_slice_dims=(0,), start_index_map=(0,)),
        slice_sizes=(1,),
        mode=lax.GatherScatterMode.PROMISE_IN_BOUNDS)


def _ballq_tec(n, k, qpt, ysliced, *refs):
    (px_h, py_h, pz_h, yx_h, yy_h, yz_h, out_h,
     px_v, py_v, pz_v, yx_v, yy_v, yz_v, bits_v, bits2_v, cb_v, ci_v,
     hist_v, hist2_v, row_v) = refs
    nvec = n // _L
    wid = lax.axis_index("s") * 2 + lax.axis_index("c")

    pltpu.sync_copy(px_h, px_v)
    pltpu.sync_copy(py_h, py_v)
    pltpu.sync_copy(pz_h, pz_v)
    if ysliced:
        pltpu.sync_copy(yx_h.at[pl.ds(wid * qpt, qpt)], yx_v)
        pltpu.sync_copy(yy_h.at[pl.ds(wid * qpt, qpt)], yy_v)
        pltpu.sync_copy(yz_h.at[pl.ds(wid * qpt, qpt)], yz_v)
    else:
        pltpu.sync_copy(yx_h, yx_v)
        pltpu.sync_copy(yy_h, yy_v)
        pltpu.sync_copy(yz_h, yz_v)

    lane = lax.iota(jnp.int32, _L)
    ones = jnp.full((_L,), 1, jnp.int32)
    _U = 4
    zeros = jnp.zeros((_L,), jnp.int32)
    last = jnp.full((_L,), _L - 1, jnp.int32)

    def clear_hist(href):
        def cj(j, c):
            for u in range(_U):
                href[pl.ds((j * _U + u) * _L, _L)] = zeros
            return c
        lax.fori_loop(0, _HW // (_U * _L), cj, 0)

    def scan_hist(href, k_rem, nbins):
        def sj(j, st):
            found, bsel, below, run = st
            acc = jnp.zeros((_L,), jnp.int32)
            for l in range(_L):
                acc = acc + href[pl.ds(l * _ST + j * _L, _L)]
            cum = plsc.cumsum(acc) + run
            run2 = _lane_gather(cum, last)
            hit = cum > k_rem
            nhit = plsc.all_reduce_population_count(hit)
            ffs = jnp.minimum(plsc.all_reduce_ffs(hit), _L - 1)
            excl = cum - acc
            below_here = _lane_gather(excl, ffs)
            bin_here = j * _L + ffs
            take = (found == 0) & (nhit > 0)
            bsel = jnp.where(take, bin_here, bsel)
            below = jnp.where(take, below_here, below)
            found = jnp.where(nhit > 0, 1, found)
            return (found, bsel, below, run2)
        z = jnp.zeros((_L,), jnp.int32)
        _, bsel, below, _ = lax.fori_loop(0, nbins // _L, sj, (z, z, z, z))
        return bsel, below

    def hist_pass_compact(shift, mask, pshift, prefix, cnt, cnt_s):
        clear_hist(hist_v)

        def pi(i, c):
            b = cb_v[pl.ds(i * _L, _L)]
            valid = (i * _L + lane) < cnt
            binv = (b >> shift) & mask
            m = valid & ((b >> pshift) == prefix)
            plsc.addupdate_scatter(hist_v, [lane * _ST + binv], ones, mask=m)
            return c
        lax.fori_loop(0, (cnt_s + _L - 1) // _L, pi, 0)

    def tail(q, bref, href):
        k0 = jnp.full((_L,), k - 1, jnp.int32)
        b1, below1 = scan_hist(href, k0, _NB)
        k1 = k0 - below1

        def p2(i, st):
            bs = [bref[pl.ds((i * _U + u) * _L, _L)] for u in range(_U)]
            rel = (bs[0] >> 22) <= b1
            for u in range(1, _U):
                rel = rel | ((bs[u] >> 22) <= b1)
            sel = jnp.max(rel.astype(jnp.int32))

            def emit(st):
                a_base, c_base = st
                for u in range(_U):
                    b = bs[u]
                    binv = b >> 22
                    lt1 = binv < b1
                    e1 = binv == b1
                    pos_a = a_base + plsc.cumsum(lt1.astype(jnp.int32)) - 1
                    pos_c = c_base + plsc.cumsum(e1.astype(jnp.int32)) - 1
                    idx_v = (i * _U + u) * _L + lane
                    plsc.store_scatter(
                        row_v, [jnp.minimum(pos_a, k - 1)], idx_v, mask=lt1)
                    plsc.store_scatter(ci_v, [pos_c], idx_v, mask=e1)
                    plsc.store_scatter(cb_v, [pos_c], b, mask=e1)
                    a_base = a_base + plsc.all_reduce_population_count(lt1)
                    c_base = c_base + plsc.all_reduce_population_count(e1)
                return (a_base, c_base)

            return lax.cond(sel > 0, emit, lambda s: s, st)
        zv = jnp.zeros((_L,), jnp.int32)
        _, cnt = lax.fori_loop(0, nvec // _U, p2, (zv, zv))
        cnt_s = jnp.max(cnt)

        hist_pass_compact(14, 0xFF, 22, b1, cnt, cnt_s)
        b2, below2 = scan_hist(hist_v, k1, 256)
        k2 = k1 - below2
        pre2 = (b1 << 8) | b2

        hist_pass_compact(6, 0xFF, 14, pre2, cnt, cnt_s)
        b3, below3 = scan_hist(hist_v, k2, 256)
        k3 = k2 - below3
        pre3 = (pre2 << 8) | b3

        hist_pass_compact(0, 0x3F, 6, pre3, cnt, cnt_s)
        b4, below4 = scan_hist(hist_v, k3, 64)

        t = (pre3 << 6) | b4
        count_lt = below1 + below2 + below3 + below4

        def fp(i, st):
            lt_base, eq_base = st
            b = cb_v[pl.ds(i * _L, _L)]
            valid = (i * _L + lane) < cnt
            lt = valid & (b < t)
            eq = valid & (b == t)
            pos_lt = lt_base + plsc.cumsum(lt.astype(jnp.int32)) - 1
            pos_eq = eq_base + plsc.cumsum(eq.astype(jnp.int32)) - 1
            idx_v = ci_v[pl.ds(i * _L, _L)]
            plsc.store_scatter(
                row_v, [jnp.minimum(pos_lt, k - 1)], idx_v, mask=lt)
            eqm = eq & (pos_eq < k)
            plsc.store_scatter(
                row_v, [jnp.minimum(pos_eq, k - 1)], idx_v, mask=eqm)
            return (lt_base + plsc.all_reduce_population_count(lt),
                    eq_base + plsc.all_reduce_population_count(eq))
        lax.fori_loop(0, (cnt_s + _L - 1) // _L, fp, (below1, count_lt))

        pltpu.sync_copy(row_v, out_h.at[q])

    def per_pair(lp, carry):
        lq0 = 2 * lp
        q0 = wid * qpt + lq0
        loc = lq0 if ysliced else q0
        base = (loc // _L) * _L
        off0 = jnp.full((_L,), loc - base, jnp.int32)
        off1 = off0 + 1
        yxc = yx_v[pl.ds(base, _L)]
        yyc = yy_v[pl.ds(base, _L)]
        yzc = yz_v[pl.ds(base, _L)]
        yx0 = _lane_gather(yxc, off0)
        yy0 = _lane_gather(yyc, off0)
        yz0 = _lane_gather(yzc, off0)
        yx1 = _lane_gather(yxc, off1)
        yy1 = _lane_gather(yyc, off1)
        yz1 = _lane_gather(yzc, off1)

        clear_hist(hist_v)
        clear_hist(hist2_v)

        def p1(i, c):
            for u in range(_U):
                sl = pl.ds((i * _U + u) * _L, _L)
                px = px_v[sl]
                py = py_v[sl]
                pz = pz_v[sl]
                dx0 = px - yx0
                dy0 = py - yy0
                dz0 = pz - yz0
                dx1 = px - yx1
                dy1 = py - yy1
                dz1 = pz - yz1
                d20 = dx0 * dx0 + dy0 * dy0 + dz0 * dz0
                d21 = dx1 * dx1 + dy1 * dy1 + dz1 * dz1
                b0 = lax.bitcast_convert_type(d20, jnp.int32)
                b1 = lax.bitcast_convert_type(d21, jnp.int32)
                bits_v[sl] = b0
                bits2_v[sl] = b1
                plsc.addupdate_scatter(hist_v, [lane * _ST + (b0 >> 22)], ones)
                plsc.addupdate_scatter(hist2_v, [lane * _ST + (b1 >> 22)], ones)
            return c
        lax.fori_loop(0, nvec // _U, p1, 0)

        tail(q0, bits_v, hist_v)
        tail(q0 + 1, bits2_v, hist2_v)
        return carry

    lax.fori_loop(0, qpt // 2, per_pair, 0)


def _ballq_sc(y_pos, pos, k):
    m = y_pos.shape[0]
    n = pos.shape[0]
    qpt = m // 32
    kout = max(k, _L)
    ysliced = qpt >= 32
    ylen = qpt if ysliced else m
    mesh = plsc.VectorSubcoreMesh(core_axis_name="c", subcore_axis_name="s")
    fn = functools.partial(
        pl.kernel,
        mesh=mesh,
        compiler_params=pltpu.CompilerParams(needs_layout_passes=False),
        out_type=jax.ShapeDtypeStruct((m, kout), jnp.int32),
        scratch_types=[
            pltpu.VMEM((n,), jnp.float32),
            pltpu.VMEM((n,), jnp.float32),
            pltpu.VMEM((n,), jnp.float32),
            pltpu.VMEM((ylen,), jnp.float32),
            pltpu.VMEM((ylen,), jnp.float32),
            pltpu.VMEM((ylen,), jnp.float32),
            pltpu.VMEM((n,), jnp.int32),
            pltpu.VMEM((n,), jnp.int32),
            pltpu.VMEM((n,), jnp.int32),
            pltpu.VMEM((n,), jnp.int32),
            pltpu.VMEM((_HW,), jnp.int32),
            pltpu.VMEM((_HW,), jnp.int32),
            pltpu.VMEM((kout,), jnp.int32),
        ],
    )(functools.partial(_ballq_tec, n, k, qpt, ysliced))
    out = fn(pos[:, 0], pos[:, 1], pos[:, 2],
             y_pos[:, 0], y_pos[:, 1], y_pos[:, 2])
    return out[:, :k] if kout != k else out


def _gather_tec(bpw, csz, *refs):
    tab_h, idx_h, out_h, idxc_v, rows_v, sem = refs
    wid = lax.axis_index("s") * 2 + lax.axis_index("c")
    base = wid * bpw

    def chunk(c, carry):
        start = base + c * csz
        pltpu.sync_copy(idx_h.at[pl.ds(start, csz)], idxc_v)
        pltpu.async_copy(tab_h.at[idxc_v], rows_v, sem).wait()
        pltpu.sync_copy(rows_v, out_h.at[pl.ds(start, csz)])
        return carry

    lax.fori_loop(0, bpw // csz, chunk, 0)


def _gather_sc(tab, idx):
    rows = idx.shape[0]
    d = tab.shape[1]
    bpw = rows // 32
    csz = 256
    while csz * 2 <= bpw and (csz * 2) * d <= 98304:
        csz *= 2
    mesh = plsc.VectorSubcoreMesh(core_axis_name="c", subcore_axis_name="s")
    fn = functools.partial(
        pl.kernel,
        mesh=mesh,
        compiler_params=pltpu.CompilerParams(
            needs_layout_passes=False, use_tc_tiling_on_sc=False),
        out_type=jax.ShapeDtypeStruct((rows, d), jnp.float32),
        scratch_types=[
            pltpu.VMEM((csz,), jnp.int32),
            pltpu.VMEM((csz, d), jnp.float32),
            pltpu.SemaphoreType.DMA,
        ],
    )(functools.partial(_gather_tec, bpw, csz))
    return fn(tab, idx)


def _fps_body(m, px_ref, py_ref, pz_ref, out_ref):
    px = px_ref[...]
    py = py_ref[...]
    pz = pz_ref[...]
    r = px.shape[0]
    row = jax.lax.broadcasted_iota(jnp.int32, (r, 128), 0)
    colv = jax.lax.broadcasted_iota(jnp.int32, (r, 128), 1)
    flat = row * 128 + colv
    out_ref[...] = jnp.zeros(out_ref.shape, jnp.int32)
    dists0 = jnp.full((r, 128), jnp.inf, jnp.float32)

    def body(i, carry):
        dists, last = carry
        sel = flat == last
        lx = jnp.sum(jnp.where(sel, px, 0.0))
        ly = jnp.sum(jnp.where(sel, py, 0.0))
        lz = jnp.sum(jnp.where(sel, pz, 0.0))
        dxx = px - lx
        dyy = py - ly
        dzz = pz - lz
        d = dxx * dxx + dyy * dyy + dzz * dzz
        dists = jnp.minimum(dists, d)
        mx = jnp.max(dists)
        idx = jnp.min(jnp.where(dists == mx, flat, jnp.int32(2 ** 30)))
        out_ref[pl.ds(i, 1), :] = jnp.reshape(idx, (1, 1))
        return (dists, idx)

    jax.lax.fori_loop(1, m, body, (dists0, jnp.int32(0)))


def _fps_idx(pos, num_samples):
    n = pos.shape[0]
    r = n // 128
    px = pos[:, 0].reshape(r, 128)
    py = pos[:, 1].reshape(r, 128)
    pz = pos[:, 2].reshape(r, 128)
    out = pl.pallas_call(
        functools.partial(_fps_body, num_samples),
        out_shape=jax.ShapeDtypeStruct((num_samples, 1), jnp.int32),
    )(px, py, pz)
    return out.reshape(num_samples)


def _sa_body(nl, kk, r2s, fouts, *refs):
    h_ref, d2_ref = refs[0], refs[1]
    wrefs = refs[2:-1]
    out_ref = refs[-1]
    h0 = h_ref[...]
    d2col = d2_ref[...]
    bm = d2col.shape[0] // kk
    col = 0
    for bi, r2 in enumerate(r2s):
        h = h0
        base = bi * nl * 4
        for li in range(nl):
            w = wrefs[base + li * 4][...]
            b = wrefs[base + li * 4 + 1][...]
            g = wrefs[base + li * 4 + 2][...]
            be = wrefs[base + li * 4 + 3][...]
            h = jnp.maximum(
                jnp.dot(h, w, preferred_element_type=jnp.float32) + b, 0.0)
            h = g * (h * _INV) + be
        fo = fouts[bi]
        penalty = jnp.where(d2col <= r2, 0.0, -jnp.inf)
        h = h + penalty
        o = jnp.max(h.reshape(bm, kk, fo), axis=1)
        o = jnp.where(jnp.isfinite(o), o, 0.0)
        out_ref[:, col:col + fo] = o
        col += fo


def _sa_conv(h_in, d2k, r_list, conv_params, bm):
    m, kk, f = h_in.shape
    h_flat = h_in.reshape(m * kk, f)
    d2col = d2k.reshape(m * kk, 1)
    nl = len(conv_params[0])
    fouts = tuple(int(layers[-1]["W"].shape[1]) for layers in conv_params)
    r2s = tuple(np.float32(r * r) for r in r_list)
    wargs, wspecs = [], []
    for layers in conv_params:
        for lyr in layers:
            for nm in ("W", "b", "gamma", "beta"):
                a = lyr[nm]
                if a.ndim == 1:
                    a = a.reshape(1, -1)
                wargs.append(a)
                wspecs.append(pl.BlockSpec(a.shape, lambda i: (0, 0)))
    out_f = sum(fouts)
    grid = (m // bm,)
    fn = pl.pallas_call(
        functools.partial(_sa_body, nl, kk, r2s, fouts),
        grid=grid,
        in_specs=[
            pl.BlockSpec((bm * kk, f), lambda i: (i, 0)),
            pl.BlockSpec((bm * kk, 1), lambda i: (i, 0)),
        ] + wspecs,
        out_specs=pl.BlockSpec((bm, out_f), lambda i: (i, 0)),
        out_shape=jax.ShapeDtypeStruct((m, out_f), jnp.float32),
    )
    return fn(h_flat, d2col, *wargs)


def _sa_module(x, pos, ratio, r_list, conv_params, bm, max_nbrs=128):
    n = pos.shape[0]
    m = int(round(ratio * n))
    idx = _fps_idx(pos, m)
    y_pos = pos[idx]
    nbr = _ballq_sc(y_pos, pos, max_nbrs)
    c = x.shape[1]
    dpad = ((c + 3 + 15) // 16) * 16
    xp = jnp.concatenate(
        [x, pos, jnp.zeros((n, dpad - c - 3), jnp.float32)], axis=1)
    g = _gather_sc(xp, nbr.reshape(-1)).reshape(m, max_nbrs, dpad)
    x_j = g[:, :, :c]
    rel = g[:, :, c:c + 3] - y_pos[:, None, :]
    d2k = jnp.sum(rel ** 2, axis=-1)
    h_in = jnp.concatenate([x_j, rel], axis=-1)
    return _sa_conv(h_in, d2k, r_list, conv_params, bm), y_pos


def _mlp_body(nl, with_head, *refs):
    h_ref = refs[0]
    wrefs = refs[1:-1]
    out_ref = refs[-1]
    h = h_ref[...]
    for li in range(nl):
        w = wrefs[li * 4][...]
        b = wrefs[li * 4 + 1][...]
        g = wrefs[li * 4 + 2][...]
        be = wrefs[li * 4 + 3][...]
        h = jnp.maximum(
            jnp.dot(h, w, preferred_element_type=jnp.float32) + b, 0.0)
        h = g * (h * _INV) + be
    if with_head:
        base = nl * 4
        w1, b1 = wrefs[base][...], wrefs[base + 1][...]
        w2, b2 = wrefs[base + 2][...], wrefs[base + 3][...]
        w3, b3 = wrefs[base + 4][...], wrefs[base + 5][...]
        h = jnp.maximum(jnp.dot(h, w1, preferred_element_type=jnp.float32) + b1, 0.0)
        h = jnp.maximum(jnp.dot(h, w2, preferred_element_type=jnp.float32) + b2, 0.0)
        h = jnp.dot(h, w3, preferred_element_type=jnp.float32) + b3
        mx = jnp.max(h, axis=-1, keepdims=True)
        sh = h - jax.lax.stop_gradient(mx)
        h = sh - jnp.log(jnp.sum(jnp.exp(sh), axis=-1, keepdims=True))
    out_ref[...] = h


def _mlp_rows(h, layers, br, head=None):
    rows, f = h.shape
    nl = len(layers)
    wargs, wspecs = [], []
    for lyr in layers:
        for nm in ("W", "b", "gamma", "beta"):
            a = lyr[nm]
            if a.ndim == 1:
                a = a.reshape(1, -1)
            wargs.append(a)
            wspecs.append(pl.BlockSpec(a.shape, lambda i: (0, 0)))
    if head is not None:
        for nm in ("W1", "b1", "W2", "b2", "W3", "b3"):
            a = head[nm]
            if a.ndim == 1:
                a = a.reshape(1, -1)
            wargs.append(a)
            wspecs.append(pl.BlockSpec(a.shape, lambda i: (0, 0)))
        out_f = head["W3"].shape[1]
    else:
        out_f = layers[-1]["W"].shape[1]
    fn = pl.pallas_call(
        functools.partial(_mlp_body, nl, head is not None),
        grid=(rows // br,),
        in_specs=[pl.BlockSpec((br, f), lambda i: (i, 0))] + wspecs,
        out_specs=pl.BlockSpec((br, out_f), lambda i: (i, 0)),
        out_shape=jax.ShapeDtypeStruct((rows, out_f), jnp.float32),
    )
    return fn(h, *wargs)


def _knn_interpolate(xf, posc, pos_skip, k=3):
    d2_sg = jax.lax.stop_gradient(
        jnp.sum((pos_skip[:, None, :] - posc[None, :, :]) ** 2, axis=-1))
    colid = jnp.arange(posc.shape[0], dtype=jnp.int32)[None, :]
    picks = []
    dcur = d2_sg
    for _ in range(k):
        i = jnp.argmin(dcur, axis=1).astype(jnp.int32)
        picks.append(i)
        dcur = jnp.where(colid == i[:, None], jnp.inf, dcur)
    idx = jnp.stack(picks, axis=1)
    diff = pos_skip[:, None, :] - posc[idx]
    d2 = jnp.sum(diff ** 2, axis=-1)
    w = 1.0 / jnp.maximum(d2, 1e-16)
    w = w / jnp.sum(w, axis=1, keepdims=True)
    return jnp.sum(xf[idx] * w[:, :, None], axis=1)


def kernel(x, pos, batch, params):
    x1, pos1 = _sa_module(x, pos, 0.25, [0.05, 0.4], params["sa1"], bm=32)
    x2, pos2 = _sa_module(x1, pos1, 0.25, [0.2, 0.8], params["sa2"], bm=32)
    x3, pos3 = _sa_module(x2, pos2, 0.25, [0.4, 1.6], params["sa3"], bm=32)

    xi3 = _knn_interpolate(x3, pos3, pos2)
    f3 = _mlp_rows(jnp.concatenate([xi3, x2], axis=1), params["fp3"], br=256)
    xi2 = _knn_interpolate(f3, pos2, pos1)
    f2 = _mlp_rows(jnp.concatenate([xi2, x1], axis=1), params["fp2"], br=512)
    xi1 = _knn_interpolate(f2, pos1, pos)
    f1 = _mlp_rows(jnp.concatenate([xi1, x], axis=1), params["fp1"], br=1024)

    return _mlp_rows(f1, [], br=1024, head=params["cls"])


# speedup vs baseline: 2.9865x; 1.0542x over previous
import functools

import jax
import jax.numpy as jnp
import numpy as np
from jax import lax
from jax.experimental import pallas as pl
from jax.experimental.pallas import tpu as pltpu
from jax.experimental.pallas import tpu_sc as plsc

_EPS_BN = 1e-5
_INV = np.float32(1.0) / np.sqrt(np.float32(1.0 + _EPS_BN))

_L = 16
_NB = 272
_ST = 273
_HW = 4416


def _lane_gather(vec, idx):
    return lax.gather(
        vec, idx[:, None],
        dimension_numbers=lax.GatherDimensionNumbers(
            offset_dims=(), collapsed_slice_dims=(0,), start_index_map=(0,)),
        slice_sizes=(1,),
        mode=lax.GatherScatterMode.PROMISE_IN_BOUNDS)


def _ballq_tec(n, k, qpt, ysliced, *refs):
    (px_h, py_h, pz_h, yx_h, yy_h, yz_h, out_h,
     px_v, py_v, pz_v, yx_v, yy_v, yz_v, bits_v, bits2_v, cb_v, ci_v,
     hist_v, hist2_v, row_v) = refs
    nvec = n // _L
    wid = lax.axis_index("s") * 2 + lax.axis_index("c")

    pltpu.sync_copy(px_h, px_v)
    pltpu.sync_copy(py_h, py_v)
    pltpu.sync_copy(pz_h, pz_v)
    if ysliced:
        pltpu.sync_copy(yx_h.at[pl.ds(wid * qpt, qpt)], yx_v)
        pltpu.sync_copy(yy_h.at[pl.ds(wid * qpt, qpt)], yy_v)
        pltpu.sync_copy(yz_h.at[pl.ds(wid * qpt, qpt)], yz_v)
    else:
        pltpu.sync_copy(yx_h, yx_v)
        pltpu.sync_copy(yy_h, yy_v)
        pltpu.sync_copy(yz_h, yz_v)

    lane = lax.iota(jnp.int32, _L)
    ones = jnp.full((_L,), 1, jnp.int32)
    _U = 8
    zeros = jnp.zeros((_L,), jnp.int32)
    last = jnp.full((_L,), _L - 1, jnp.int32)

    def clear_hist(href):
        def cj(j, c):
            for u in range(_U):
                href[pl.ds((j * _U + u) * _L, _L)] = zeros
            return c
        lax.fori_loop(0, _HW // (_U * _L), cj, 0)

    def scan_hist(href, k_rem, nbins):
        def sj(j, st):
            found, bsel, below, run = st
            acc = jnp.zeros((_L,), jnp.int32)
            for l in range(_L):
                acc = acc + href[pl.ds(l * _ST + j * _L, _L)]
            cum = plsc.cumsum(acc) + run
            run2 = _lane_gather(cum, last)
            hit = cum > k_rem
            nhit = plsc.all_reduce_population_count(hit)
            ffs = jnp.minimum(plsc.all_reduce_ffs(hit), _L - 1)
            excl = cum - acc
            below_here = _lane_gather(excl, ffs)
            bin_here = j * _L + ffs
            take = (found == 0) & (nhit > 0)
            bsel = jnp.where(take, bin_here, bsel)
            below = jnp.where(take, below_here, below)
            found = jnp.where(nhit > 0, 1, found)
            return (found, bsel, below, run2)
        z = jnp.zeros((_L,), jnp.int32)
        _, bsel, below, _ = lax.fori_loop(0, nbins // _L, sj, (z, z, z, z))
        return bsel, below

    def hist_pass_compact(shift, mask, pshift, prefix, cnt, cnt_s):
        clear_hist(hist_v)

        def pi(i, c):
            b = cb_v[pl.ds(i * _L, _L)]
            valid = (i * _L + lane) < cnt
            binv = (b >> shift) & mask
            m = valid & ((b >> pshift) == prefix)
            plsc.addupdate_scatter(hist_v, [lane * _ST + binv], ones, mask=m)
            return c
        lax.fori_loop(0, (cnt_s + _L - 1) // _L, pi, 0)

    def tail(q, bref, href):
        k0 = jnp.full((_L,), k - 1, jnp.int32)
        b1, below1 = scan_hist(href, k0, _NB)
        k1 = k0 - below1

        def p2(i, st):
            bs = [bref[pl.ds((i * _U + u) * _L, _L)] for u in range(_U)]
            rel = (bs[0] >> 22) <= b1
            for u in range(1, _U):
                rel = rel | ((bs[u] >> 22) <= b1)
            sel = jnp.max(rel.astype(jnp.int32))

            def emit(st):
                a_base, c_base = st
                for u in range(_U):
                    b = bs[u]
                    binv = b >> 22
                    lt1 = binv < b1
                    e1 = binv == b1
                    pos_a = a_base + plsc.cumsum(lt1.astype(jnp.int32)) - 1
                    pos_c = c_base + plsc.cumsum(e1.astype(jnp.int32)) - 1
                    idx_v = (i * _U + u) * _L + lane
                    plsc.store_scatter(
                        row_v, [jnp.minimum(pos_a, k - 1)], idx_v, mask=lt1)
                    plsc.store_scatter(ci_v, [pos_c], idx_v, mask=e1)
                    plsc.store_scatter(cb_v, [pos_c], b, mask=e1)
                    a_base = a_base + plsc.all_reduce_population_count(lt1)
                    c_base = c_base + plsc.all_reduce_population_count(e1)
                return (a_base, c_base)

            return lax.cond(sel > 0, emit, lambda s: s, st)
        zv = jnp.zeros((_L,), jnp.int32)
        _, cnt = lax.fori_loop(0, nvec // _U, p2, (zv, zv))
        cnt_s = jnp.max(cnt)

        hist_pass_compact(14, 0xFF, 22, b1, cnt, cnt_s)
        b2, below2 = scan_hist(hist_v, k1, 256)
        k2 = k1 - below2
        pre2 = (b1 << 8) | b2

        hist_pass_compact(6, 0xFF, 14, pre2, cnt, cnt_s)
        b3, below3 = scan_hist(hist_v, k2, 256)
        k3 = k2 - below3
        pre3 = (pre2 << 8) | b3

        hist_pass_compact(0, 0x3F, 6, pre3, cnt, cnt_s)
        b4, below4 = scan_hist(hist_v, k3, 64)

        t = (pre3 << 6) | b4
        count_lt = below1 + below2 + below3 + below4

        def fp(i, st):
            lt_base, eq_base = st
            b = cb_v[pl.ds(i * _L, _L)]
            valid = (i * _L + lane) < cnt
            lt = valid & (b < t)
            eq = valid & (b == t)
            pos_lt = lt_base + plsc.cumsum(lt.astype(jnp.int32)) - 1
            pos_eq = eq_base + plsc.cumsum(eq.astype(jnp.int32)) - 1
            idx_v = ci_v[pl.ds(i * _L, _L)]
            plsc.store_scatter(
                row_v, [jnp.minimum(pos_lt, k - 1)], idx_v, mask=lt)
            eqm = eq & (pos_eq < k)
            plsc.store_scatter(
                row_v, [jnp.minimum(pos_eq, k - 1)], idx_v, mask=eqm)
            return (lt_base + plsc.all_reduce_population_count(lt),
                    eq_base + plsc.all_reduce_population_count(eq))
        lax.fori_loop(0, (cnt_s + _L - 1) // _L, fp, (below1, count_lt))

        pltpu.sync_copy(row_v, out_h.at[q])

    def per_pair(lp, carry):
        lq0 = 2 * lp
        q0 = wid * qpt + lq0
        loc = lq0 if ysliced else q0
        base = (loc // _L) * _L
        off0 = jnp.full((_L,), loc - base, jnp.int32)
        off1 = off0 + 1
        yxc = yx_v[pl.ds(base, _L)]
        yyc = yy_v[pl.ds(base, _L)]
        yzc = yz_v[pl.ds(base, _L)]
        yx0 = _lane_gather(yxc, off0)
        yy0 = _lane_gather(yyc, off0)
        yz0 = _lane_gather(yzc, off0)
        yx1 = _lane_gather(yxc, off1)
        yy1 = _lane_gather(yyc, off1)
        yz1 = _lane_gather(yzc, off1)

        clear_hist(hist_v)
        clear_hist(hist2_v)

        def p1(i, c):
            for u in range(_U):
                sl = pl.ds((i * _U + u) * _L, _L)
                px = px_v[sl]
                py = py_v[sl]
                pz = pz_v[sl]
                dx0 = px - yx0
                dy0 = py - yy0
                dz0 = pz - yz0
                dx1 = px - yx1
                dy1 = py - yy1
                dz1 = pz - yz1
                d20 = dx0 * dx0 + dy0 * dy0 + dz0 * dz0
                d21 = dx1 * dx1 + dy1 * dy1 + dz1 * dz1
                b0 = lax.bitcast_convert_type(d20, jnp.int32)
                b1 = lax.bitcast_convert_type(d21, jnp.int32)
                bits_v[sl] = b0
                bits2_v[sl] = b1
                plsc.addupdate_scatter(hist_v, [lane * _ST + (b0 >> 22)], ones)
                plsc.addupdate_scatter(hist2_v, [lane * _ST + (b1 >> 22)], ones)
            return c
        lax.fori_loop(0, nvec // _U, p1, 0)

        tail(q0, bits_v, hist_v)
        tail(q0 + 1, bits2_v, hist2_v)
        return carry

    lax.fori_loop(0, qpt // 2, per_pair, 0)


def _ballq_sc(y_pos, pos, k):
    m = y_pos.shape[0]
    n = pos.shape[0]
    qpt = m // 32
    kout = max(k, _L)
    ysliced = qpt >= 32
    ylen = qpt if ysliced else m
    mesh = plsc.VectorSubcoreMesh(core_axis_name="c", subcore_axis_name="s")
    fn = functools.partial(
        pl.kernel,
        mesh=mesh,
        compiler_params=pltpu.CompilerParams(needs_layout_passes=False),
        out_type=jax.ShapeDtypeStruct((m, kout), jnp.int32),
        scratch_types=[
            pltpu.VMEM((n,), jnp.float32),
            pltpu.VMEM((n,), jnp.float32),
            pltpu.VMEM((n,), jnp.float32),
            pltpu.VMEM((ylen,), jnp.float32),
            pltpu.VMEM((ylen,), jnp.float32),
            pltpu.VMEM((ylen,), jnp.float32),
            pltpu.VMEM((n,), jnp.int32),
            pltpu.VMEM((n,), jnp.int32),
            pltpu.VMEM((n,), jnp.int32),
            pltpu.VMEM((n,), jnp.int32),
            pltpu.VMEM((_HW,), jnp.int32),
            pltpu.VMEM((_HW,), jnp.int32),
            pltpu.VMEM((kout,), jnp.int32),
        ],
    )(functools.partial(_ballq_tec, n, k, qpt, ysliced))
    out = fn(pos[:, 0], pos[:, 1], pos[:, 2],
             y_pos[:, 0], y_pos[:, 1], y_pos[:, 2])
    return out[:, :k] if kout != k else out


def _gather_tec(bpw, csz, *refs):
    tab_h, idx_h, out_h, idxc_v, rows_v, sem = refs
    wid = lax.axis_index("s") * 2 + lax.axis_index("c")
    base = wid * bpw

    def chunk(c, carry):
        start = base + c * csz
        pltpu.sync_copy(idx_h.at[pl.ds(start, csz)], idxc_v)
        pltpu.async_copy(tab_h.at[idxc_v], rows_v, sem).wait()
        pltpu.sync_copy(rows_v, out_h.at[pl.ds(start, csz)])
        return carry

    lax.fori_loop(0, bpw // csz, chunk, 0)


def _gather_sc(tab, idx):
    rows = idx.shape[0]
    d = tab.shape[1]
    bpw = rows // 32
    csz = 256
    while csz * 2 <= bpw and (csz * 2) * d <= 98304:
        csz *= 2
    mesh = plsc.VectorSubcoreMesh(core_axis_name="c", subcore_axis_name="s")
    fn = functools.partial(
        pl.kernel,
        mesh=mesh,
        compiler_params=pltpu.CompilerParams(
            needs_layout_passes=False, use_tc_tiling_on_sc=False),
        out_type=jax.ShapeDtypeStruct((rows, d), jnp.float32),
        scratch_types=[
            pltpu.VMEM((csz,), jnp.int32),
            pltpu.VMEM((csz, d), jnp.float32),
            pltpu.SemaphoreType.DMA,
        ],
    )(functools.partial(_gather_tec, bpw, csz))
    return fn(tab, idx)


def _fps_body(m, px_ref, py_ref, pz_ref, out_ref):
    px = px_ref[...]
    py = py_ref[...]
    pz = pz_ref[...]
    r = px.shape[0]
    row = jax.lax.broadcasted_iota(jnp.int32, (r, 128), 0)
    colv = jax.lax.broadcasted_iota(jnp.int32, (r, 128), 1)
    flat = row * 128 + colv
    out_ref[...] = jnp.zeros(out_ref.shape, jnp.int32)
    dists0 = jnp.full((r, 128), jnp.inf, jnp.float32)

    def body(i, carry):
        dists, last = carry
        sel = flat == last
        lx = jnp.sum(jnp.where(sel, px, 0.0))
        ly = jnp.sum(jnp.where(sel, py, 0.0))
        lz = jnp.sum(jnp.where(sel, pz, 0.0))
        dxx = px - lx
        dyy = py - ly
        dzz = pz - lz
        d = dxx * dxx + dyy * dyy + dzz * dzz
        dists = jnp.minimum(dists, d)
        mx = jnp.max(dists)
        idx = jnp.min(jnp.where(dists == mx, flat, jnp.int32(2 ** 30)))
        out_ref[pl.ds(i, 1), :] = jnp.reshape(idx, (1, 1))
        return (dists, idx)

    jax.lax.fori_loop(1, m, body, (dists0, jnp.int32(0)))


def _fps_idx(pos, num_samples):
    n = pos.shape[0]
    r = n // 128
    px = pos[:, 0].reshape(r, 128)
    py = pos[:, 1].reshape(r, 128)
    pz = pos[:, 2].reshape(r, 128)
    out = pl.pallas_call(
        functools.partial(_fps_body, num_samples),
        out_shape=jax.ShapeDtypeStruct((num_samples, 1), jnp.int32),
    )(px, py, pz)
    return out.reshape(num_samples)


def _sa_body(nl, kk, r2s, fouts, *refs):
    h_ref, d2_ref = refs[0], refs[1]
    wrefs = refs[2:-1]
    out_ref = refs[-1]
    h0 = h_ref[...]
    d2col = d2_ref[...]
    bm = d2col.shape[0] // kk
    col = 0
    for bi, r2 in enumerate(r2s):
        h = h0
        base = bi * nl * 4
        for li in range(nl):
            w = wrefs[base + li * 4][...]
            b = wrefs[base + li * 4 + 1][...]
            g = wrefs[base + li * 4 + 2][...]
            be = wrefs[base + li * 4 + 3][...]
            h = jnp.maximum(
                jnp.dot(h, w, preferred_element_type=jnp.float32) + b, 0.0)
            h = g * (h * _INV) + be
        fo = fouts[bi]
        penalty = jnp.where(d2col <= r2, 0.0, -jnp.inf)
        h = h + penalty
        o = jnp.max(h.reshape(bm, kk, fo), axis=1)
        o = jnp.where(jnp.isfinite(o), o, 0.0)
        out_ref[:, col:col + fo] = o
        col += fo


def _sa_conv(h_in, d2k, r_list, conv_params, bm):
    m, kk, f = h_in.shape
    h_flat = h_in.reshape(m * kk, f)
    d2col = d2k.reshape(m * kk, 1)
    nl = len(conv_params[0])
    fouts = tuple(int(layers[-1]["W"].shape[1]) for layers in conv_params)
    r2s = tuple(np.float32(r * r) for r in r_list)
    wargs, wspecs = [], []
    for layers in conv_params:
        for lyr in layers:
            for nm in ("W", "b", "gamma", "beta"):
                a = lyr[nm]
                if a.ndim == 1:
                    a = a.reshape(1, -1)
                wargs.append(a)
                wspecs.append(pl.BlockSpec(a.shape, lambda i: (0, 0)))
    out_f = sum(fouts)
    grid = (m // bm,)
    fn = pl.pallas_call(
        functools.partial(_sa_body, nl, kk, r2s, fouts),
        grid=grid,
        in_specs=[
            pl.BlockSpec((bm * kk, f), lambda i: (i, 0)),
            pl.BlockSpec((bm * kk, 1), lambda i: (i, 0)),
        ] + wspecs,
        out_specs=pl.BlockSpec((bm, out_f), lambda i: (i, 0)),
        out_shape=jax.ShapeDtypeStruct((m, out_f), jnp.float32),
    )
    return fn(h_flat, d2col, *wargs)


def _sa_module(x, pos, ratio, r_list, conv_params, bm, max_nbrs=128):
    n = pos.shape[0]
    m = int(round(ratio * n))
    idx = _fps_idx(pos, m)
    y_pos = pos[idx]
    nbr = _ballq_sc(y_pos, pos, max_nbrs)
    c = x.shape[1]
    dpad = ((c + 3 + 15) // 16) * 16
    xp = jnp.concatenate(
        [x, pos, jnp.zeros((n, dpad - c - 3), jnp.float32)], axis=1)
    g = _gather_sc(xp, nbr.reshape(-1)).reshape(m, max_nbrs, dpad)
    x_j = g[:, :, :c]
    rel = g[:, :, c:c + 3] - y_pos[:, None, :]
    d2k = jnp.sum(rel ** 2, axis=-1)
    h_in = jnp.concatenate([x_j, rel], axis=-1)
    return _sa_conv(h_in, d2k, r_list, conv_params, bm), y_pos


def _mlp_body(nl, with_head, *refs):
    h_ref = refs[0]
    wrefs = refs[1:-1]
    out_ref = refs[-1]
    h = h_ref[...]
    for li in range(nl):
        w = wrefs[li * 4][...]
        b = wrefs[li * 4 + 1][...]
        g = wrefs[li * 4 + 2][...]
        be = wrefs[li * 4 + 3][...]
        h = jnp.maximum(
            jnp.dot(h, w, preferred_element_type=jnp.float32) + b, 0.0)
        h = g * (h * _INV) + be
    if with_head:
        base = nl * 4
        w1, b1 = wrefs[base][...], wrefs[base + 1][...]
        w2, b2 = wrefs[base + 2][...], wrefs[base + 3][...]
        w3, b3 = wrefs[base + 4][...], wrefs[base + 5][...]
        h = jnp.maximum(jnp.dot(h, w1, preferred_element_type=jnp.float32) + b1, 0.0)
        h = jnp.maximum(jnp.dot(h, w2, preferred_element_type=jnp.float32) + b2, 0.0)
        h = jnp.dot(h, w3, preferred_element_type=jnp.float32) + b3
        mx = jnp.max(h, axis=-1, keepdims=True)
        sh = h - jax.lax.stop_gradient(mx)
        h = sh - jnp.log(jnp.sum(jnp.exp(sh), axis=-1, keepdims=True))
    out_ref[...] = h


def _mlp_rows(h, layers, br, head=None):
    rows, f = h.shape
    nl = len(layers)
    wargs, wspecs = [], []
    for lyr in layers:
        for nm in ("W", "b", "gamma", "beta"):
            a = lyr[nm]
            if a.ndim == 1:
                a = a.reshape(1, -1)
            wargs.append(a)
            wspecs.append(pl.BlockSpec(a.shape, lambda i: (0, 0)))
    if head is not None:
        for nm in ("W1", "b1", "W2", "b2", "W3", "b3"):
            a = head[nm]
            if a.ndim == 1:
                a = a.reshape(1, -1)
            wargs.append(a)
            wspecs.append(pl.BlockSpec(a.shape, lambda i: (0, 0)))
        out_f = head["W3"].shape[1]
    else:
        out_f = layers[-1]["W"].shape[1]
    fn = pl.pallas_call(
        functools.partial(_mlp_body, nl, head is not None),
        grid=(rows // br,),
        in_specs=[pl.BlockSpec((br, f), lambda i: (i, 0))] + wspecs,
        out_specs=pl.BlockSpec((br, out_f), lambda i: (i, 0)),
        out_shape=jax.ShapeDtypeStruct((rows, out_f), jnp.float32),
    )
    return fn(h, *wargs)


def _knn_interpolate(xf, posc, pos_skip, k=3):
    d2_sg = jax.lax.stop_gradient(
        jnp.sum((pos_skip[:, None, :] - posc[None, :, :]) ** 2, axis=-1))
    colid = jnp.arange(posc.shape[0], dtype=jnp.int32)[None, :]
    picks = []
    dcur = d2_sg
    for _ in range(k):
        i = jnp.argmin(dcur, axis=1).astype(jnp.int32)
        picks.append(i)
        dcur = jnp.where(colid == i[:, None], jnp.inf, dcur)
    idx = jnp.stack(picks, axis=1)
    diff = pos_skip[:, None, :] - posc[idx]
    d2 = jnp.sum(diff ** 2, axis=-1)
    w = 1.0 / jnp.maximum(d2, 1e-16)
    w = w / jnp.sum(w, axis=1, keepdims=True)
    return jnp.sum(xf[idx] * w[:, :, None], axis=1)


def kernel(x, pos, batch, params):
    x1, pos1 = _sa_module(x, pos, 0.25, [0.05, 0.4], params["sa1"], bm=32)
    x2, pos2 = _sa_module(x1, pos1, 0.25, [0.2, 0.8], params["sa2"], bm=32)
    x3, pos3 = _sa_module(x2, pos2, 0.25, [0.4, 1.6], params["sa3"], bm=32)

    xi3 = _knn_interpolate(x3, pos3, pos2)
    f3 = _mlp_rows(jnp.concatenate([xi3, x2], axis=1), params["fp3"], br=256)
    xi2 = _knn_interpolate(f3, pos2, pos1)
    f2 = _mlp_rows(jnp.concatenate([xi2, x1], axis=1), params["fp2"], br=512)
    xi1 = _knn_interpolate(f2, pos1, pos)
    f1 = _mlp_rows(jnp.concatenate([xi1, x], axis=1), params["fp1"], br=1024)

    return _mlp_rows(f1, [], br=1024, head=params["cls"])


# speedup vs baseline: 3.0537x; 1.0225x over previous
import functools

import jax
import jax.numpy as jnp
import numpy as np
from jax import lax
from jax.experimental import pallas as pl
from jax.experimental.pallas import tpu as pltpu
from jax.experimental.pallas import tpu_sc as plsc

_EPS_BN = 1e-5
_INV = np.float32(1.0) / np.sqrt(np.float32(1.0 + _EPS_BN))

_L = 16
_NB = 272
_ST = 273
_HW = 4416


def _lane_gather(vec, idx):
    return lax.gather(
        vec, idx[:, None],
        dimension_numbers=lax.GatherDimensionNumbers(
            offset_dims=(), collapsed_slice_dims=(0,), start_index_map=(0,)),
        slice_sizes=(1,),
        mode=lax.GatherScatterMode.PROMISE_IN_BOUNDS)


def _ballq_tec(n, k, qpt, ysliced, *refs):
    (px_h, py_h, pz_h, yx_h, yy_h, yz_h, out_h,
     px_v, py_v, pz_v, yx_v, yy_v, yz_v, bits_v, bits2_v, cb_v, ci_v,
     hist_v, hist2_v, row_v) = refs
    nvec = n // _L
    wid = lax.axis_index("s") * 2 + lax.axis_index("c")

    pltpu.sync_copy(px_h, px_v)
    pltpu.sync_copy(py_h, py_v)
    pltpu.sync_copy(pz_h, pz_v)
    if ysliced:
        pltpu.sync_copy(yx_h.at[pl.ds(wid * qpt, qpt)], yx_v)
        pltpu.sync_copy(yy_h.at[pl.ds(wid * qpt, qpt)], yy_v)
        pltpu.sync_copy(yz_h.at[pl.ds(wid * qpt, qpt)], yz_v)
    else:
        pltpu.sync_copy(yx_h, yx_v)
        pltpu.sync_copy(yy_h, yy_v)
        pltpu.sync_copy(yz_h, yz_v)

    lane = lax.iota(jnp.int32, _L)
    ones = jnp.full((_L,), 1, jnp.int32)
    _U = 16
    zeros = jnp.zeros((_L,), jnp.int32)
    last = jnp.full((_L,), _L - 1, jnp.int32)

    def clear_hist(href):
        def cj(j, c):
            for u in range(_U):
                href[pl.ds((j * _U + u) * _L, _L)] = zeros
            return c
        lax.fori_loop(0, _HW // (_U * _L), cj, 0)

    def scan_hist(href, k_rem, nbins):
        def sj(j, st):
            found, bsel, below, run = st
            acc = jnp.zeros((_L,), jnp.int32)
            for l in range(_L):
                acc = acc + href[pl.ds(l * _ST + j * _L, _L)]
            cum = plsc.cumsum(acc) + run
            run2 = _lane_gather(cum, last)
            hit = cum > k_rem
            nhit = plsc.all_reduce_population_count(hit)
            ffs = jnp.minimum(plsc.all_reduce_ffs(hit), _L - 1)
            excl = cum - acc
            below_here = _lane_gather(excl, ffs)
            bin_here = j * _L + ffs
            take = (found == 0) & (nhit > 0)
            bsel = jnp.where(take, bin_here, bsel)
            below = jnp.where(take, below_here, below)
            found = jnp.where(nhit > 0, 1, found)
            return (found, bsel, below, run2)
        z = jnp.zeros((_L,), jnp.int32)
        _, bsel, below, _ = lax.fori_loop(0, nbins // _L, sj, (z, z, z, z))
        return bsel, below

    def hist_pass_compact(shift, mask, pshift, prefix, cnt, cnt_s):
        clear_hist(hist_v)

        def pi(i, c):
            b = cb_v[pl.ds(i * _L, _L)]
            valid = (i * _L + lane) < cnt
            binv = (b >> shift) & mask
            m = valid & ((b >> pshift) == prefix)
            plsc.addupdate_scatter(hist_v, [lane * _ST + binv], ones, mask=m)
            return c
        lax.fori_loop(0, (cnt_s + _L - 1) // _L, pi, 0)

    def tail(q, bref, href):
        k0 = jnp.full((_L,), k - 1, jnp.int32)
        b1, below1 = scan_hist(href, k0, _NB)
        k1 = k0 - below1

        def p2(i, st):
            bs = [bref[pl.ds((i * _U + u) * _L, _L)] for u in range(_U)]
            rel = (bs[0] >> 22) <= b1
            for u in range(1, _U):
                rel = rel | ((bs[u] >> 22) <= b1)
            sel = jnp.max(rel.astype(jnp.int32))

            def emit(st):
                a_base, c_base = st
                for u in range(_U):
                    b = bs[u]
                    binv = b >> 22
                    lt1 = binv < b1
                    e1 = binv == b1
                    pos_a = a_base + plsc.cumsum(lt1.astype(jnp.int32)) - 1
                    pos_c = c_base + plsc.cumsum(e1.astype(jnp.int32)) - 1
                    idx_v = (i * _U + u) * _L + lane
                    plsc.store_scatter(
                        row_v, [jnp.minimum(pos_a, k - 1)], idx_v, mask=lt1)
                    plsc.store_scatter(ci_v, [pos_c], idx_v, mask=e1)
                    plsc.store_scatter(cb_v, [pos_c], b, mask=e1)
                    a_base = a_base + plsc.all_reduce_population_count(lt1)
                    c_base = c_base + plsc.all_reduce_population_count(e1)
                return (a_base, c_base)

            return lax.cond(sel > 0, emit, lambda s: s, st)
        zv = jnp.zeros((_L,), jnp.int32)
        _, cnt = lax.fori_loop(0, nvec // _U, p2, (zv, zv))
        cnt_s = jnp.max(cnt)

        hist_pass_compact(14, 0xFF, 22, b1, cnt, cnt_s)
        b2, below2 = scan_hist(hist_v, k1, 256)
        k2 = k1 - below2
        pre2 = (b1 << 8) | b2

        hist_pass_compact(6, 0xFF, 14, pre2, cnt, cnt_s)
        b3, below3 = scan_hist(hist_v, k2, 256)
        k3 = k2 - below3
        pre3 = (pre2 << 8) | b3

        hist_pass_compact(0, 0x3F, 6, pre3, cnt, cnt_s)
        b4, below4 = scan_hist(hist_v, k3, 64)

        t = (pre3 << 6) | b4
        count_lt = below1 + below2 + below3 + below4

        def fp(i, st):
            lt_base, eq_base = st
            b = cb_v[pl.ds(i * _L, _L)]
            valid = (i * _L + lane) < cnt
            lt = valid & (b < t)
            eq = valid & (b == t)
            pos_lt = lt_base + plsc.cumsum(lt.astype(jnp.int32)) - 1
            pos_eq = eq_base + plsc.cumsum(eq.astype(jnp.int32)) - 1
            idx_v = ci_v[pl.ds(i * _L, _L)]
            plsc.store_scatter(
                row_v, [jnp.minimum(pos_lt, k - 1)], idx_v, mask=lt)
            eqm = eq & (pos_eq < k)
            plsc.store_scatter(
                row_v, [jnp.minimum(pos_eq, k - 1)], idx_v, mask=eqm)
            return (lt_base + plsc.all_reduce_population_count(lt),
                    eq_base + plsc.all_reduce_population_count(eq))
        lax.fori_loop(0, (cnt_s + _L - 1) // _L, fp, (below1, count_lt))

        pltpu.sync_copy(row_v, out_h.at[q])

    def per_pair(lp, carry):
        lq0 = 2 * lp
        q0 = wid * qpt + lq0
        loc = lq0 if ysliced else q0
        base = (loc // _L) * _L
        off0 = jnp.full((_L,), loc - base, jnp.int32)
        off1 = off0 + 1
        yxc = yx_v[pl.ds(base, _L)]
        yyc = yy_v[pl.ds(base, _L)]
        yzc = yz_v[pl.ds(base, _L)]
        yx0 = _lane_gather(yxc, off0)
        yy0 = _lane_gather(yyc, off0)
        yz0 = _lane_gather(yzc, off0)
        yx1 = _lane_gather(yxc, off1)
        yy1 = _lane_gather(yyc, off1)
        yz1 = _lane_gather(yzc, off1)

        clear_hist(hist_v)
        clear_hist(hist2_v)

        def p1(i, c):
            for u in range(_U):
                sl = pl.ds((i * _U + u) * _L, _L)
                px = px_v[sl]
                py = py_v[sl]
                pz = pz_v[sl]
                dx0 = px - yx0
                dy0 = py - yy0
                dz0 = pz - yz0
                dx1 = px - yx1
                dy1 = py - yy1
                dz1 = pz - yz1
                d20 = dx0 * dx0 + dy0 * dy0 + dz0 * dz0
                d21 = dx1 * dx1 + dy1 * dy1 + dz1 * dz1
                b0 = lax.bitcast_convert_type(d20, jnp.int32)
                b1 = lax.bitcast_convert_type(d21, jnp.int32)
                bits_v[sl] = b0
                bits2_v[sl] = b1
                plsc.addupdate_scatter(hist_v, [lane * _ST + (b0 >> 22)], ones)
                plsc.addupdate_scatter(hist2_v, [lane * _ST + (b1 >> 22)], ones)
            return c
        lax.fori_loop(0, nvec // _U, p1, 0)

        tail(q0, bits_v, hist_v)
        tail(q0 + 1, bits2_v, hist2_v)
        return carry

    lax.fori_loop(0, qpt // 2, per_pair, 0)


def _ballq_sc(y_pos, pos, k):
    m = y_pos.shape[0]
    n = pos.shape[0]
    qpt = m // 32
    kout = max(k, _L)
    ysliced = qpt >= 32
    ylen = qpt if ysliced else m
    mesh = plsc.VectorSubcoreMesh(core_axis_name="c", subcore_axis_name="s")
    fn = functools.partial(
        pl.kernel,
        mesh=mesh,
        compiler_params=pltpu.CompilerParams(needs_layout_passes=False),
        out_type=jax.ShapeDtypeStruct((m, kout), jnp.int32),
        scratch_types=[
            pltpu.VMEM((n,), jnp.float32),
            pltpu.VMEM((n,), jnp.float32),
            pltpu.VMEM((n,), jnp.float32),
            pltpu.VMEM((ylen,), jnp.float32),
            pltpu.VMEM((ylen,), jnp.float32),
            pltpu.VMEM((ylen,), jnp.float32),
            pltpu.VMEM((n,), jnp.int32),
            pltpu.VMEM((n,), jnp.int32),
            pltpu.VMEM((n,), jnp.int32),
            pltpu.VMEM((n,), jnp.int32),
            pltpu.VMEM((_HW,), jnp.int32),
            pltpu.VMEM((_HW,), jnp.int32),
            pltpu.VMEM((kout,), jnp.int32),
        ],
    )(functools.partial(_ballq_tec, n, k, qpt, ysliced))
    out = fn(pos[:, 0], pos[:, 1], pos[:, 2],
             y_pos[:, 0], y_pos[:, 1], y_pos[:, 2])
    return out[:, :k] if kout != k else out


def _gather_tec(bpw, csz, *refs):
    tab_h, idx_h, out_h, idxc_v, rows_v, sem = refs
    wid = lax.axis_index("s") * 2 + lax.axis_index("c")
    base = wid * bpw

    def chunk(c, carry):
        start = base + c * csz
        pltpu.sync_copy(idx_h.at[pl.ds(start, csz)], idxc_v)
        pltpu.async_copy(tab_h.at[idxc_v], rows_v, sem).wait()
        pltpu.sync_copy(rows_v, out_h.at[pl.ds(start, csz)])
        return carry

    lax.fori_loop(0, bpw // csz, chunk, 0)


def _gather_sc(tab, idx):
    rows = idx.shape[0]
    d = tab.shape[1]
    bpw = rows // 32
    csz = 256
    while csz * 2 <= bpw and (csz * 2) * d <= 98304:
        csz *= 2
    mesh = plsc.VectorSubcoreMesh(core_axis_name="c", subcore_axis_name="s")
    fn = functools.partial(
        pl.kernel,
        mesh=mesh,
        compiler_params=pltpu.CompilerParams(
            needs_layout_passes=False, use_tc_tiling_on_sc=False),
        out_type=jax.ShapeDtypeStruct((rows, d), jnp.float32),
        scratch_types=[
            pltpu.VMEM((csz,), jnp.int32),
            pltpu.VMEM((csz, d), jnp.float32),
            pltpu.SemaphoreType.DMA,
        ],
    )(functools.partial(_gather_tec, bpw, csz))
    return fn(tab, idx)


def _fps_body(m, px_ref, py_ref, pz_ref, out_ref):
    px = px_ref[...]
    py = py_ref[...]
    pz = pz_ref[...]
    r = px.shape[0]
    row = jax.lax.broadcasted_iota(jnp.int32, (r, 128), 0)
    colv = jax.lax.broadcasted_iota(jnp.int32, (r, 128), 1)
    flat = row * 128 + colv
    out_ref[...] = jnp.zeros(out_ref.shape, jnp.int32)
    dists0 = jnp.full((r, 128), jnp.inf, jnp.float32)

    def body(i, carry):
        dists, last = carry
        sel = flat == last
        lx = jnp.sum(jnp.where(sel, px, 0.0))
        ly = jnp.sum(jnp.where(sel, py, 0.0))
        lz = jnp.sum(jnp.where(sel, pz, 0.0))
        dxx = px - lx
        dyy = py - ly
        dzz = pz - lz
        d = dxx * dxx + dyy * dyy + dzz * dzz
        dists = jnp.minimum(dists, d)
        mx = jnp.max(dists)
        idx = jnp.min(jnp.where(dists == mx, flat, jnp.int32(2 ** 30)))
        out_ref[pl.ds(i, 1), :] = jnp.reshape(idx, (1, 1))
        return (dists, idx)

    jax.lax.fori_loop(1, m, body, (dists0, jnp.int32(0)))


def _fps_idx(pos, num_samples):
    n = pos.shape[0]
    r = n // 128
    px = pos[:, 0].reshape(r, 128)
    py = pos[:, 1].reshape(r, 128)
    pz = pos[:, 2].reshape(r, 128)
    out = pl.pallas_call(
        functools.partial(_fps_body, num_samples),
        out_shape=jax.ShapeDtypeStruct((num_samples, 1), jnp.int32),
    )(px, py, pz)
    return out.reshape(num_samples)


def _sa_body(nl, kk, r2s, fouts, *refs):
    h_ref, d2_ref = refs[0], refs[1]
    wrefs = refs[2:-1]
    out_ref = refs[-1]
    h0 = h_ref[...]
    d2col = d2_ref[...]
    bm = d2col.shape[0] // kk
    col = 0
    for bi, r2 in enumerate(r2s):
        h = h0
        base = bi * nl * 4
        for li in range(nl):
            w = wrefs[base + li * 4][...]
            b = wrefs[base + li * 4 + 1][...]
            g = wrefs[base + li * 4 + 2][...]
            be = wrefs[base + li * 4 + 3][...]
            h = jnp.maximum(
                jnp.dot(h, w, preferred_element_type=jnp.float32) + b, 0.0)
            h = g * (h * _INV) + be
        fo = fouts[bi]
        penalty = jnp.where(d2col <= r2, 0.0, -jnp.inf)
        h = h + penalty
        o = jnp.max(h.reshape(bm, kk, fo), axis=1)
        o = jnp.where(jnp.isfinite(o), o, 0.0)
        out_ref[:, col:col + fo] = o
        col += fo


def _sa_conv(h_in, d2k, r_list, conv_params, bm):
    m, kk, f = h_in.shape
    h_flat = h_in.reshape(m * kk, f)
    d2col = d2k.reshape(m * kk, 1)
    nl = len(conv_params[0])
    fouts = tuple(int(layers[-1]["W"].shape[1]) for layers in conv_params)
    r2s = tuple(np.float32(r * r) for r in r_list)
    wargs, wspecs = [], []
    for layers in conv_params:
        for lyr in layers:
            for nm in ("W", "b", "gamma", "beta"):
                a = lyr[nm]
                if a.ndim == 1:
                    a = a.reshape(1, -1)
                wargs.append(a)
                wspecs.append(pl.BlockSpec(a.shape, lambda i: (0, 0)))
    out_f = sum(fouts)
    grid = (m // bm,)
    fn = pl.pallas_call(
        functools.partial(_sa_body, nl, kk, r2s, fouts),
        grid=grid,
        in_specs=[
            pl.BlockSpec((bm * kk, f), lambda i: (i, 0)),
            pl.BlockSpec((bm * kk, 1), lambda i: (i, 0)),
        ] + wspecs,
        out_specs=pl.BlockSpec((bm, out_f), lambda i: (i, 0)),
        out_shape=jax.ShapeDtypeStruct((m, out_f), jnp.float32),
    )
    return fn(h_flat, d2col, *wargs)


def _sa_module(x, pos, ratio, r_list, conv_params, bm, max_nbrs=128):
    n = pos.shape[0]
    m = int(round(ratio * n))
    idx = _fps_idx(pos, m)
    y_pos = pos[idx]
    nbr = _ballq_sc(y_pos, pos, max_nbrs)
    c = x.shape[1]
    dpad = ((c + 3 + 15) // 16) * 16
    xp = jnp.concatenate(
        [x, pos, jnp.zeros((n, dpad - c - 3), jnp.float32)], axis=1)
    g = _gather_sc(xp, nbr.reshape(-1)).reshape(m, max_nbrs, dpad)
    x_j = g[:, :, :c]
    rel = g[:, :, c:c + 3] - y_pos[:, None, :]
    d2k = jnp.sum(rel ** 2, axis=-1)
    h_in = jnp.concatenate([x_j, rel], axis=-1)
    return _sa_conv(h_in, d2k, r_list, conv_params, bm), y_pos


def _mlp_body(nl, with_head, *refs):
    h_ref = refs[0]
    wrefs = refs[1:-1]
    out_ref = refs[-1]
    h = h_ref[...]
    for li in range(nl):
        w = wrefs[li * 4][...]
        b = wrefs[li * 4 + 1][...]
        g = wrefs[li * 4 + 2][...]
        be = wrefs[li * 4 + 3][...]
        h = jnp.maximum(
            jnp.dot(h, w, preferred_element_type=jnp.float32) + b, 0.0)
        h = g * (h * _INV) + be
    if with_head:
        base = nl * 4
        w1, b1 = wrefs[base][...], wrefs[base + 1][...]
        w2, b2 = wrefs[base + 2][...], wrefs[base + 3][...]
        w3, b3 = wrefs[base + 4][...], wrefs[base + 5][...]
        h = jnp.maximum(jnp.dot(h, w1, preferred_element_type=jnp.float32) + b1, 0.0)
        h = jnp.maximum(jnp.dot(h, w2, preferred_element_type=jnp.float32) + b2, 0.0)
        h = jnp.dot(h, w3, preferred_element_type=jnp.float32) + b3
        mx = jnp.max(h, axis=-1, keepdims=True)
        sh = h - jax.lax.stop_gradient(mx)
        h = sh - jnp.log(jnp.sum(jnp.exp(sh), axis=-1, keepdims=True))
    out_ref[...] = h


def _mlp_rows(h, layers, br, head=None):
    rows, f = h.shape
    nl = len(layers)
    wargs, wspecs = [], []
    for lyr in layers:
        for nm in ("W", "b", "gamma", "beta"):
            a = lyr[nm]
            if a.ndim == 1:
                a = a.reshape(1, -1)
            wargs.append(a)
            wspecs.append(pl.BlockSpec(a.shape, lambda i: (0, 0)))
    if head is not None:
        for nm in ("W1", "b1", "W2", "b2", "W3", "b3"):
            a = head[nm]
            if a.ndim == 1:
                a = a.reshape(1, -1)
            wargs.append(a)
            wspecs.append(pl.BlockSpec(a.shape, lambda i: (0, 0)))
        out_f = head["W3"].shape[1]
    else:
        out_f = layers[-1]["W"].shape[1]
    fn = pl.pallas_call(
        functools.partial(_mlp_body, nl, head is not None),
        grid=(rows // br,),
        in_specs=[pl.BlockSpec((br, f), lambda i: (i, 0))] + wspecs,
        out_specs=pl.BlockSpec((br, out_f), lambda i: (i, 0)),
        out_shape=jax.ShapeDtypeStruct((rows, out_f), jnp.float32),
    )
    return fn(h, *wargs)


def _knn_interpolate(xf, posc, pos_skip, k=3):
    d2_sg = jax.lax.stop_gradient(
        jnp.sum((pos_skip[:, None, :] - posc[None, :, :]) ** 2, axis=-1))
    colid = jnp.arange(posc.shape[0], dtype=jnp.int32)[None, :]
    picks = []
    dcur = d2_sg
    for _ in range(k):
        i = jnp.argmin(dcur, axis=1).astype(jnp.int32)
        picks.append(i)
        dcur = jnp.where(colid == i[:, None], jnp.inf, dcur)
    idx = jnp.stack(picks, axis=1)
    diff = pos_skip[:, None, :] - posc[idx]
    d2 = jnp.sum(diff ** 2, axis=-1)
    w = 1.0 / jnp.maximum(d2, 1e-16)
    w = w / jnp.sum(w, axis=1, keepdims=True)
    return jnp.sum(xf[idx] * w[:, :, None], axis=1)


def kernel(x, pos, batch, params):
    x1, pos1 = _sa_module(x, pos, 0.25, [0.05, 0.4], params["sa1"], bm=32)
    x2, pos2 = _sa_module(x1, pos1, 0.25, [0.2, 0.8], params["sa2"], bm=32)
    x3, pos3 = _sa_module(x2, pos2, 0.25, [0.4, 1.6], params["sa3"], bm=32)

    xi3 = _knn_interpolate(x3, pos3, pos2)
    f3 = _mlp_rows(jnp.concatenate([xi3, x2], axis=1), params["fp3"], br=256)
    xi2 = _knn_interpolate(f3, pos2, pos1)
    f2 = _mlp_rows(jnp.concatenate([xi2, x1], axis=1), params["fp2"], br=512)
    xi1 = _knn_interpolate(f2, pos1, pos)
    f1 = _mlp_rows(jnp.concatenate([xi1, x], axis=1), params["fp1"], br=1024)

    return _mlp_rows(f1, [], br=1024, head=params["cls"])
